# baseline (speedup 1.0000x reference)
import functools
import jax
import jax.numpy as jnp
from jax import lax
from jax.experimental import pallas as pl
from jax.experimental.pallas import tpu as pltpu
from jax.experimental.pallas import tpu_sc as plsc

F32 = jnp.float32
I32 = jnp.int32

NN = 10000
EE = 160000
NT = 32
RNG = 320
NROW = 328
CH = 128
SCAN_B = 1280
NSCAN = EE // SCAN_B
CLCAP = 10368
SPILL = 10240
CLW = EE + 128

_MESH = dict(core_axis_name="c", subcore_axis_name="s",
             num_cores=2, num_subcores=16)
_PARAMS = pltpu.CompilerParams(needs_layout_passes=False)


def _worker(c, s):
    return c * 16 + s


@functools.partial(
    pl.kernel,
    out_type=(
        jax.ShapeDtypeStruct((NT, 1, CLW), I32),
        jax.ShapeDtypeStruct((NT, 1, 16), I32),
        jax.ShapeDtypeStruct((NT * RNG,), F32),
    ),
    mesh=plsc.VectorSubcoreMesh(**_MESH),
    compiler_params=_PARAMS,
    scratch_types=[
        pltpu.VMEM((2, SCAN_B), I32),
        pltpu.VMEM((CLCAP,), I32),
        pltpu.VMEM((NROW,), F32),
        pltpu.VMEM((16,), I32),
    ],
)
def _sc_scan(ei, clist, counts, deg, ebuf, clbuf, dacc, cstg):
    c = lax.axis_index("c")
    s = lax.axis_index("s")
    t = _worker(c, s)
    base = t * RNG
    iota = lax.broadcasted_iota(I32, (16,), 0)
    ones16 = jnp.ones((16,), F32)
    zero16 = jnp.zeros((16,), F32)

    def dz(r, _):
        dacc[pl.ds(r * 16, 16)] = zero16
        return 0

    lax.fori_loop(0, NROW // 16, dz, 0)

    def chunk(g, carry):
        pltpu.sync_copy(ei.at[:, pl.ds(g * SCAN_B, SCAN_B)], ebuf)

        def group(j, carry):
            off, hoff = carry
            s16 = ebuf[0, pl.ds(j * 16, 16)]
            d16 = ebuf[1, pl.ds(j * 16, 16)]
            ok = (d16 >= base) & (d16 < base + RNG)
            key = jnp.where(ok, iota, 16)
            _, sv = plsc.sort_key_val(key, s16 * 16384 + d16)
            clbuf[pl.ds(off, 16)] = sv
            cntv = plsc.all_reduce_population_count(ok)
            off = off + cntv[0]
            loc = jnp.clip(d16 - base, 0, RNG)
            plsc.addupdate_scatter(dacc, [loc], jnp.where(ok, ones16, zero16))

            @pl.when(off >= SPILL)
            def _():
                pltpu.sync_copy(
                    clbuf.at[pl.ds(0, SPILL)],
                    clist.at[t, 0, pl.ds(pl.multiple_of(hoff, 128), SPILL)])
                tail = clbuf[pl.ds(SPILL, 16)]
                clbuf[pl.ds(0, 16)] = tail

            spilled = off >= SPILL
            off = jnp.where(spilled, off - SPILL, off)
            hoff = jnp.where(spilled, hoff + SPILL, hoff)
            return (off, hoff)

        return lax.fori_loop(0, SCAN_B // 16, group, carry)

    off, hoff = lax.fori_loop(0, NSCAN, chunk, (0, 0))

    padv = jnp.zeros((16,), I32) + (base + RNG)
    for i in range(8):
        clbuf[pl.ds(off + i * 16, 16)] = padv
    nsp = (off + 127) // 128

    def spill_fin(g, _):
        o = pl.multiple_of(g * 128, 128)
        pltpu.sync_copy(
            clbuf.at[pl.ds(o, 128)],
            clist.at[t, 0, pl.ds(pl.multiple_of(hoff, 128) + o, 128)])
        return 0

    lax.fori_loop(0, nsp, spill_fin, 0)
    cstg[pl.ds(0, 16)] = jnp.zeros((16,), I32) + (hoff + nsp * 128)
    pltpu.sync_copy(cstg, counts.at[t, 0])
    pltpu.sync_copy(dacc.at[pl.ds(0, RNG)], deg.at[pl.ds(base, RNG)])


def _read_count(counts, cntb, t):
    pltpu.sync_copy(counts.at[t, 0], cntb)
    return cntb[pl.ds(0, 16)][0]


def _unpack_chunk(gibuf, dlbuf, base):
    for j in range(CH // 16):
        v16 = gibuf[pl.ds(j * 16, 16)]
        d16 = v16 & 16383
        dlbuf[pl.ds(j * 16, 16)] = jnp.clip(d16 - base, 0, RNG)
        gibuf[pl.ds(j * 16, 16)] = lax.shift_right_logical(v16, 14)


def _drain(acc, out_slice, base):
    for k in range(RNG // 64):
        pltpu.sync_copy(acc.at[pl.ds(k * 64, 64)],
                        out_slice.at[pl.ds(base + k * 64, 64)])


def _zero_acc(acc, w):
    z = jnp.zeros((16,), F32)

    def za(r, _):
        for j in range(w // 16):
            acc[r, pl.ds(j * 16, 16)] = z
        return 0

    lax.fori_loop(0, NROW, za, 0)


NQ = 4
QB = CH // NQ


def _make_rowsum(w):
    @functools.partial(
        pl.kernel,
        out_type=jax.ShapeDtypeStruct((NT * RNG, w), F32),
        mesh=plsc.VectorSubcoreMesh(**_MESH),
        compiler_params=_PARAMS,
        scratch_types=[
            pltpu.VMEM((NROW, w), F32),
            pltpu.VMEM((NQ, QB, w), F32),
            pltpu.VMEM((CH,), I32),
            pltpu.VMEM((CH,), I32),
            pltpu.VMEM((16,), I32),
        ] + [pltpu.SemaphoreType.DMA] * NQ,
    )
    def rowsum(table, clist, counts, out, acc, rowbuf, gibuf, dlbuf, cntb,
               *sems):
        c = lax.axis_index("c")
        s = lax.axis_index("s")
        t = _worker(c, s)
        base = t * RNG
        _zero_acc(acc, w)
        nch = _read_count(counts, cntb, t) // CH

        def chunk(g, _):
            pltpu.sync_copy(clist.at[t, 0, pl.ds(g * CH, CH)], gibuf)
            _unpack_chunk(gibuf, dlbuf, base)
            descs = [
                pltpu.async_copy(table.at[gibuf.at[pl.ds(q * QB, QB)]],
                                 rowbuf.at[q], sems[q])
                for q in range(NQ)]
            for q in range(NQ):
                descs[q].wait()

                def rmw(j, _, q=q):
                    dv = dlbuf[pl.ds(q * QB + j * 16, 16)]
                    for l in range(16):
                        d = dv[l]
                        r = j * 16 + l
                        for col in range(w // 16):
                            sl = pl.ds(col * 16, 16)
                            acc[d, sl] = acc[d, sl] + rowbuf[q, r, sl]
                    return 0

                lax.fori_loop(0, QB // 16, rmw, 0)
            return 0

        lax.fori_loop(0, nch, chunk, 0)
        _drain(acc, out, base)

    return rowsum


_sc_rowsum256 = _make_rowsum(256)
_sc_rowsum128 = _make_rowsum(128)


@functools.partial(
    pl.kernel,
    out_type=(
        jax.ShapeDtypeStruct((4, NT * RNG, 128), F32),
        jax.ShapeDtypeStruct((4, NT * RNG, 16), F32),
    ),
    mesh=plsc.VectorSubcoreMesh(**_MESH),
    compiler_params=_PARAMS,
    scratch_types=[
        pltpu.VMEM((NROW, 128), F32),
        pltpu.VMEM((NROW, 16), F32),
        pltpu.VMEM((2 * NN,), F32),
        pltpu.VMEM((NQ, QB, 128), F32),
        pltpu.VMEM((CH,), F32),
        pltpu.VMEM((CH,), I32),
        pltpu.VMEM((CH,), I32),
        pltpu.VMEM((16,), I32),
    ] + [pltpu.SemaphoreType.DMA] * NQ,
)
def _sc_gat(hw2, aa4, clist, counts, num_out, den_out,
            nacc, dacc, aav, rowbuf, wrow, gibuf, dlbuf, cntb, *sems):
    c = lax.axis_index("c")
    s = lax.axis_index("s")
    t = _worker(c, s)
    base = t * RNG
    lane0 = jnp.where(lax.broadcasted_iota(I32, (16,), 0) == 0, 1.0, 0.0)
    nch = _read_count(counts, cntb, t) // CH

    def head(h, _):
        pltpu.sync_copy(aa4.at[h, 0], aav)
        _zero_acc(nacc, 128)
        _zero_acc(dacc, 16)

        def chunk(g, _):
            pltpu.sync_copy(clist.at[t, 0, pl.ds(g * CH, CH)], gibuf)
            for j in range(CH // 16):
                v16 = gibuf[pl.ds(j * 16, 16)]
                d16 = v16 & 16383
                s16 = lax.shift_right_logical(v16, 14)
                a_s = plsc.load_gather(aav, [s16 * 2])
                a_d = plsc.load_gather(aav, [jnp.minimum(d16, NN - 1) * 2 + 1])
                e = a_s + a_d
                e = jnp.maximum(e, 0.2 * e)
                wrow[pl.ds(j * 16, 16)] = jnp.exp(e)
                dlbuf[pl.ds(j * 16, 16)] = jnp.clip(d16 - base, 0, RNG)
                gibuf[pl.ds(j * 16, 16)] = s16 * 4 + h
            descs = [
                pltpu.async_copy(hw2.at[gibuf.at[pl.ds(q * QB, QB)]],
                                 rowbuf.at[q], sems[q])
                for q in range(NQ)]
            for q in range(NQ):
                descs[q].wait()

                def rmw(j, _, q=q):
                    dv = dlbuf[pl.ds(q * QB + j * 16, 16)]
                    wv = wrow[pl.ds(q * QB + j * 16, 16)]
                    for l in range(16):
                        d = dv[l]
                        r = j * 16 + l
                        wl = wv[l]
                        for col in range(8):
                            sl = pl.ds(col * 16, 16)
                            acc_v = nacc[d, sl]
                            nacc[d, sl] = acc_v + rowbuf[q, r, sl] * wl
                        dsl = pl.ds(0, 16)
                        dacc[d, dsl] = dacc[d, dsl] + lane0 * wl
                    return 0

                lax.fori_loop(0, QB // 16, rmw, 0)
            return 0

        lax.fori_loop(0, nch, chunk, 0)
        _drain(nacc, num_out.at[h], base)
        _drain(dacc, den_out.at[h], base)
        return 0

    lax.fori_loop(0, 4, head, 0)


RB = 1000
_GRID = NN // RB


def _ln(h, g, b):
    mu = jnp.mean(h, axis=-1, keepdims=True)
    var = jnp.mean((h - mu) ** 2, axis=-1, keepdims=True)
    return (h - mu) * lax.rsqrt(var + 1e-5) * g + b


def _rows(c):
    return pl.BlockSpec((RB, c), lambda i: (i, 0))


def _full(*shape):
    nd = len(shape)
    return pl.BlockSpec(shape, lambda i: (0,) * nd)


def _tc1_body(x, W, degp, out):
    dinv = lax.rsqrt(degp[...] + 1.0)
    out[...] = jnp.dot(x[...], W[...], preferred_element_type=F32) * dinv


def _tc1(x, W, degp):
    return pl.pallas_call(
        _tc1_body,
        grid=(_GRID,),
        in_specs=[_rows(256), _full(256, 256), _rows(1)],
        out_specs=_rows(256),
        out_shape=jax.ShapeDtypeStruct((NN, 256), F32),
    )(x, W, degp)


def _tc2_body(acc1, hw1p, degp, b1, g1, be1, Wg, asr, ads, Wr2, br2,
              hw2_o, aa_o, res2_o):
    dinv = lax.rsqrt(degp[...] + 1.0)
    h1 = jax.nn.relu(_ln(dinv * (acc1[...] + hw1p[...]) + b1[...],
                         g1[...], be1[...]))
    hw2 = jnp.dot(h1, Wg[...], preferred_element_type=F32)
    hw2_o[...] = hw2
    heads = []
    for h in range(4):
        blk = hw2[:, h * 128:(h + 1) * 128]
        a_s = jnp.sum(blk * asr[...][h][None, :], axis=1, keepdims=True)
        a_d = jnp.sum(blk * ads[...][h][None, :], axis=1, keepdims=True)
        heads.append(jnp.concatenate([a_s, a_d], axis=1)[None])
    aa_o[...] = jnp.concatenate(heads, axis=0)
    res2_o[...] = jnp.dot(h1, Wr2[...], preferred_element_type=F32) + br2[...]


def _tc2(acc1, hw1p, degp, p):
    return pl.pallas_call(
        _tc2_body,
        grid=(_GRID,),
        in_specs=[_rows(256), _rows(256), _rows(1), _full(256), _full(256),
                  _full(256), _full(256, 512), _full(4, 128), _full(4, 128),
                  _full(256, 128), _full(128)],
        out_specs=[_rows(512), pl.BlockSpec((4, RB, 2), lambda i: (0, i, 0)),
                   _rows(128)],
        out_shape=[jax.ShapeDtypeStruct((NN, 512), F32),
                   jax.ShapeDtypeStruct((4, NN, 2), F32),
                   jax.ShapeDtypeStruct((NN, 128), F32)],
    )(acc1, hw1p, degp, p["gcn1_b"], p["ln1_g"], p["ln1_b"], p["gat2_W"],
      p["gat2_att_src"], p["gat2_att_dst"], p["res2_W"], p["res2_b"])


def _tc3_body(num, den, hw2, aa, res2, bg, g2, be2, Wc, bc,
              h2_o, part_o):
    aam = aa[...]
    hw2m = hw2[...]
    gat = jnp.zeros_like(res2[...])
    for h in range(4):
        esl = aam[h, :, 0:1] + aam[h, :, 1:2]
        wsl = jnp.exp(jnp.maximum(esl, 0.2 * esl))
        nh = num[...][h] + wsl * hw2m[:, h * 128:(h + 1) * 128]
        sh = den[...][h][:, 0:1] + wsl
        gat = gat + nh / (sh + 1e-16)
    gat = gat * 0.25 + bg[...]
    h2 = jax.nn.relu(_ln(gat + res2[...], g2[...], be2[...]))
    h2_o[...] = h2
    part_o[...] = jnp.dot(h2, Wc[...], preferred_element_type=F32) + bc[...]


def _tc3(num, den, hw2, aa, res2, p):
    return pl.pallas_call(
        _tc3_body,
        grid=(_GRID,),
        in_specs=[pl.BlockSpec((4, RB, 128), lambda i: (0, i, 0)),
                  pl.BlockSpec((4, RB, 16), lambda i: (0, i, 0)),
                  _rows(512), pl.BlockSpec((4, RB, 2), lambda i: (0, i, 0)),
                  _rows(128), _full(128), _full(128),
                  _full(128), _full(128, 64), _full(64)],
        out_specs=[_rows(128), _rows(64)],
        out_shape=[jax.ShapeDtypeStruct((NN, 128), F32),
                   jax.ShapeDtypeStruct((NN, 64), F32)],
    )(num, den, hw2, aa, res2, p["gat2_b"], p["ln2_g"], p["ln2_b"],
      p["sage3_Wr"] + p["res3_W"],
      p["res3_b"] + p["sage3_bl"])


def _tc4_body(accS, degp, part, Wl, g3, be3, W4, h4p_o):
    deg = degp[...]
    mean = accS[...] / jnp.maximum(deg, 1.0)
    h3 = jax.nn.relu(_ln(jnp.dot(mean, Wl[...], preferred_element_type=F32)
                         + part[...], g3[...], be3[...]))
    h4p = jnp.dot(h3, W4[...], preferred_element_type=F32) * lax.rsqrt(deg + 1.0)
    h4p_o[...] = jnp.concatenate([h4p, jnp.zeros((RB, 64), F32)], axis=1)


def _tc4(accS, degp, part, p):
    return pl.pallas_call(
        _tc4_body,
        grid=(_GRID,),
        in_specs=[_rows(128), _rows(1), _rows(64), _full(128, 64), _full(64),
                  _full(64), _full(64, 64)],
        out_specs=_rows(128),
        out_shape=jax.ShapeDtypeStruct((NN, 128), F32),
    )(accS, degp, part, p["sage3_Wl"], p["ln3_g"], p["ln3_b"], p["gc4_W"])


def _tc5_body(acc4, h4p, degp, batch3, b4, g4, be4, Wro, bro, pool_o):
    i = pl.program_id(0)
    dinv = lax.rsqrt(degp[...] + 1.0)
    h4 = jax.nn.relu(_ln(dinv * (acc4[...] + h4p[...]) + b4[...],
                         g4[...], be4[...]))
    gate = jax.nn.sigmoid(jnp.dot(h4, Wro[...], preferred_element_type=F32) + bro[...])
    gated = h4 * gate
    b = batch3[...][0, 0, :]
    P = (lax.broadcasted_iota(I32, (64, RB), 0) == b[None, :]).astype(F32)
    rhs = jnp.concatenate([gated, gate, jnp.zeros((RB, 63), F32)], axis=1)
    blk = jnp.dot(P, rhs, preferred_element_type=F32)

    @pl.when(i == 0)
    def _():
        pool_o[...] = jnp.zeros_like(pool_o)

    pool_o[...] += blk


def _tc5(acc4, h4p, degp, batch3, p):
    return pl.pallas_call(
        _tc5_body,
        grid=(_GRID,),
        in_specs=[_rows(64), _rows(64), _rows(1),
                  pl.BlockSpec((1, 1, RB), lambda i: (i, 0, 0)),
                  _full(64), _full(64), _full(64), _full(64, 1), _full(1)],
        out_specs=_full(64, 128),
        out_shape=jax.ShapeDtypeStruct((64, 128), F32),
    )(acc4, h4p, degp, batch3, p["gc4_b"], p["ln4_g"], p["ln4_b"],
      p["ro_W"], p["ro_b"])


def _tc6_body(pool, link3, W1, b1, W2, b2, W3, b3, out_o):
    pm = pool[...]
    emb = pm[:, :64] / (pm[:, 64:65] + 1e-8)
    lk = link3[...][:, 0, :]
    g_iota = lax.broadcasted_iota(I32, (4096, 64), 1)
    o1 = (lk[0][:, None] == g_iota).astype(F32)
    o2 = (lk[1][:, None] == g_iota).astype(F32)
    f = jnp.concatenate([
        jnp.dot(o1, emb, preferred_element_type=F32),
        jnp.dot(o2, emb, preferred_element_type=F32)], axis=1)
    f = jax.nn.relu(jnp.dot(f, W1[...], preferred_element_type=F32) + b1[...])
    f = jax.nn.relu(jnp.dot(f, W2[...], preferred_element_type=F32) + b2[...])
    out_o[...] = jax.nn.sigmoid(jnp.dot(f, W3[...], preferred_element_type=F32) + b3[...])


def _tc6(pool, link3, p):
    return pl.pallas_call(
        _tc6_body,
        in_specs=[pl.BlockSpec((64, 128), lambda: (0, 0)),
                  pl.BlockSpec((2, 1, 4096), lambda: (0, 0, 0)),
                  pl.BlockSpec((128, 64), lambda: (0, 0)),
                  pl.BlockSpec((64,), lambda: (0,)),
                  pl.BlockSpec((64, 64), lambda: (0, 0)),
                  pl.BlockSpec((64,), lambda: (0,)),
                  pl.BlockSpec((64, 1), lambda: (0, 0)),
                  pl.BlockSpec((1,), lambda: (0,)),
                  ],
        out_specs=pl.BlockSpec((4096, 1), lambda: (0, 0)),
        out_shape=jax.ShapeDtypeStruct((4096, 1), F32),
    )(pool, link3, p["mlp1_W"], p["mlp1_b"], p["mlp2_W"], p["mlp2_b"],
      p["mlp3_W"], p["mlp3_b"])


def kernel(x, edge_index, batch, link_indices, params):
    p = params

    clist, counts, deg = _sc_scan(edge_index)
    degp = deg[:NN, None]
    hw1p = _tc1(x, p["gcn1_W"], degp)
    acc1 = _sc_rowsum256(hw1p, clist, counts)[:NN]
    hw2, aa, res2 = _tc2(acc1, hw1p, degp, p)
    num, den = _sc_gat(hw2.reshape(NN * 4, 128),
                       aa.reshape(4, 1, 2 * NN), clist, counts)
    h2, part = _tc3(num[:, :NN], den[:, :NN], hw2, aa, res2, p)
    accS = _sc_rowsum128(h2, clist, counts)[:NN]
    h4p = _tc4(accS, degp, part, p)
    acc4 = _sc_rowsum128(h4p, clist, counts)[:NN, :64]
    pool = _tc5(acc4, h4p[:, :64], degp, batch.reshape(_GRID, 1, RB), p)
    out = _tc6(pool, link_indices.reshape(2, 1, 4096), p)
    return out[:, 0]


# speedup vs baseline: 1.1661x; 1.1661x over previous
import functools
import jax
import jax.numpy as jnp
from jax import lax
from jax.experimental import pallas as pl
from jax.experimental.pallas import tpu as pltpu
from jax.experimental.pallas import tpu_sc as plsc

F32 = jnp.float32
I32 = jnp.int32

NN = 10000
EE = 160000
NT = 32
RNG = 320
NROW = 328
CH = 128
SCAN_B = 1280
NSCAN = EE // SCAN_B
CLCAP = 10368
SPILL = 10240
CLW = EE + 128

_MESH = dict(core_axis_name="c", subcore_axis_name="s",
             num_cores=2, num_subcores=16)
_PARAMS = pltpu.CompilerParams(needs_layout_passes=False)


def _coli():
    io = lax.broadcasted_iota(I32, (16,), 0)
    return [io + c * 16 for c in range(16)]


class _ColI:
    def __getitem__(self, c):
        io = lax.broadcasted_iota(I32, (16,), 0)
        return io + c * 16


_COLI = _ColI()


def _worker(c, s):
    return c * 16 + s


@functools.partial(
    pl.kernel,
    out_type=(
        jax.ShapeDtypeStruct((NT, 1, CLW), I32),
        jax.ShapeDtypeStruct((NT, 1, 16), I32),
        jax.ShapeDtypeStruct((NT * RNG,), F32),
    ),
    mesh=plsc.VectorSubcoreMesh(**_MESH),
    compiler_params=_PARAMS,
    scratch_types=[
        pltpu.VMEM((2, SCAN_B), I32),
        pltpu.VMEM((CLCAP,), I32),
        pltpu.VMEM((NROW,), F32),
        pltpu.VMEM((16,), I32),
    ],
)
def _sc_scan(ei, clist, counts, deg, ebuf, clbuf, dacc, cstg):
    c = lax.axis_index("c")
    s = lax.axis_index("s")
    t = _worker(c, s)
    base = t * RNG
    iota = lax.broadcasted_iota(I32, (16,), 0)
    ones16 = jnp.ones((16,), F32)
    zero16 = jnp.zeros((16,), F32)

    def dz(r, _):
        dacc[pl.ds(r * 16, 16)] = zero16
        return 0

    lax.fori_loop(0, NROW // 16, dz, 0)

    def chunk(g, carry):
        pltpu.sync_copy(ei.at[:, pl.ds(g * SCAN_B, SCAN_B)], ebuf)

        def group(j, carry):
            off, hoff = carry
            s16 = ebuf[0, pl.ds(j * 16, 16)]
            d16 = ebuf[1, pl.ds(j * 16, 16)]
            ok = (d16 >= base) & (d16 < base + RNG)
            key = jnp.where(ok, iota, 16)
            _, sv = plsc.sort_key_val(key, s16 * 16384 + d16)
            clbuf[pl.ds(off, 16)] = sv
            cntv = plsc.all_reduce_population_count(ok)
            off = off + cntv[0]
            loc = jnp.clip(d16 - base, 0, RNG)
            plsc.addupdate_scatter(dacc, [loc], jnp.where(ok, ones16, zero16))

            @pl.when(off >= SPILL)
            def _():
                pltpu.sync_copy(
                    clbuf.at[pl.ds(0, SPILL)],
                    clist.at[t, 0, pl.ds(pl.multiple_of(hoff, 128), SPILL)])
                tail = clbuf[pl.ds(SPILL, 16)]
                clbuf[pl.ds(0, 16)] = tail

            spilled = off >= SPILL
            off = jnp.where(spilled, off - SPILL, off)
            hoff = jnp.where(spilled, hoff + SPILL, hoff)
            return (off, hoff)

        return lax.fori_loop(0, SCAN_B // 16, group, carry)

    off, hoff = lax.fori_loop(0, NSCAN, chunk, (0, 0))

    padv = jnp.zeros((16,), I32) + (base + RNG)
    for i in range(8):
        clbuf[pl.ds(off + i * 16, 16)] = padv
    nsp = (off + 127) // 128

    def spill_fin(g, _):
        o = pl.multiple_of(g * 128, 128)
        pltpu.sync_copy(
            clbuf.at[pl.ds(o, 128)],
            clist.at[t, 0, pl.ds(pl.multiple_of(hoff, 128) + o, 128)])
        return 0

    lax.fori_loop(0, nsp, spill_fin, 0)
    cstg[pl.ds(0, 16)] = jnp.zeros((16,), I32) + (hoff + nsp * 128)
    pltpu.sync_copy(cstg, counts.at[t, 0])
    pltpu.sync_copy(dacc.at[pl.ds(0, RNG)], deg.at[pl.ds(base, RNG)])


def _read_count(counts, cntb, t):
    pltpu.sync_copy(counts.at[t, 0], cntb)
    return cntb[pl.ds(0, 16)][0]


def _unpack_chunk(gibuf, dlbuf, base):
    for j in range(CH // 16):
        v16 = gibuf[pl.ds(j * 16, 16)]
        d16 = v16 & 16383
        dlbuf[pl.ds(j * 16, 16)] = jnp.clip(d16 - base, 0, RNG)
        gibuf[pl.ds(j * 16, 16)] = lax.shift_right_logical(v16, 14)


def _drain(acc, out_slice, base):
    for k in range(RNG // 64):
        pltpu.sync_copy(acc.at[pl.ds(k * 64, 64)],
                        out_slice.at[pl.ds(base + k * 64, 64)])


def _zero_acc(acc, w):
    z = jnp.zeros((16,), F32)

    def za(r, _):
        for j in range(w // 16):
            acc[r, pl.ds(j * 16, 16)] = z
        return 0

    lax.fori_loop(0, NROW, za, 0)


NQ = 4
QB = CH // NQ


def _make_rowsum(w):
    @functools.partial(
        pl.kernel,
        out_type=jax.ShapeDtypeStruct((NT * RNG, w), F32),
        mesh=plsc.VectorSubcoreMesh(**_MESH),
        compiler_params=_PARAMS,
        scratch_types=[
            pltpu.VMEM((NROW, w), F32),
            pltpu.VMEM((NQ, QB, w), F32),
            pltpu.VMEM((CH,), I32),
            pltpu.VMEM((CH,), I32),
            pltpu.VMEM((16,), I32),
        ] + [pltpu.SemaphoreType.DMA] * NQ,
    )
    def rowsum(table, clist, counts, out, acc, rowbuf, gibuf, dlbuf, cntb,
               *sems):
        c = lax.axis_index("c")
        s = lax.axis_index("s")
        t = _worker(c, s)
        base = t * RNG
        _zero_acc(acc, w)
        nch = _read_count(counts, cntb, t) // CH

        def chunk(g, _):
            pltpu.sync_copy(clist.at[t, 0, pl.ds(g * CH, CH)], gibuf)
            _unpack_chunk(gibuf, dlbuf, base)
            descs = [
                pltpu.async_copy(table.at[gibuf.at[pl.ds(q * QB, QB)]],
                                 rowbuf.at[q], sems[q])
                for q in range(NQ)]
            for q in range(NQ):
                descs[q].wait()

                def rmw(j, _, q=q):
                    dv = dlbuf[pl.ds(q * QB + j * 16, 16)]
                    for l in range(16):
                        rows = lax.broadcast(dv[l], (16,))
                        r = j * 16 + l
                        for col in range(w // 16):
                            plsc.addupdate_scatter(
                                acc, [rows, _COLI[col]],
                                rowbuf[q, r, pl.ds(col * 16, 16)])
                    return 0

                lax.fori_loop(0, QB // 16, rmw, 0)
            return 0

        lax.fori_loop(0, nch, chunk, 0)
        _drain(acc, out, base)

    return rowsum


_sc_rowsum256 = _make_rowsum(256)
_sc_rowsum128 = _make_rowsum(128)


@functools.partial(
    pl.kernel,
    out_type=(
        jax.ShapeDtypeStruct((4, NT * RNG, 128), F32),
        jax.ShapeDtypeStruct((4, NT * RNG, 16), F32),
    ),
    mesh=plsc.VectorSubcoreMesh(**_MESH),
    compiler_params=_PARAMS,
    scratch_types=[
        pltpu.VMEM((NROW, 128), F32),
        pltpu.VMEM((NROW, 16), F32),
        pltpu.VMEM((2 * NN,), F32),
        pltpu.VMEM((NQ, QB, 128), F32),
        pltpu.VMEM((CH,), F32),
        pltpu.VMEM((CH,), I32),
        pltpu.VMEM((CH,), I32),
        pltpu.VMEM((16,), I32),
    ] + [pltpu.SemaphoreType.DMA] * NQ,
)
def _sc_gat(hw2, aa4, clist, counts, num_out, den_out,
            nacc, dacc, aav, rowbuf, wrow, gibuf, dlbuf, cntb, *sems):
    c = lax.axis_index("c")
    s = lax.axis_index("s")
    t = _worker(c, s)
    base = t * RNG
    lane0 = jnp.where(lax.broadcasted_iota(I32, (16,), 0) == 0, 1.0, 0.0)
    nch = _read_count(counts, cntb, t) // CH

    def head(h, _):
        pltpu.sync_copy(aa4.at[h, 0], aav)
        _zero_acc(nacc, 128)
        _zero_acc(dacc, 16)

        def chunk(g, _):
            pltpu.sync_copy(clist.at[t, 0, pl.ds(g * CH, CH)], gibuf)
            for j in range(CH // 16):
                v16 = gibuf[pl.ds(j * 16, 16)]
                d16 = v16 & 16383
                s16 = lax.shift_right_logical(v16, 14)
                a_s = plsc.load_gather(aav, [s16 * 2])
                a_d = plsc.load_gather(aav, [jnp.minimum(d16, NN - 1) * 2 + 1])
                e = a_s + a_d
                e = jnp.maximum(e, 0.2 * e)
                wrow[pl.ds(j * 16, 16)] = jnp.exp(e)
                dlbuf[pl.ds(j * 16, 16)] = jnp.clip(d16 - base, 0, RNG)
                gibuf[pl.ds(j * 16, 16)] = s16 * 4 + h
            descs = [
                pltpu.async_copy(hw2.at[gibuf.at[pl.ds(q * QB, QB)]],
                                 rowbuf.at[q], sems[q])
                for q in range(NQ)]
            for q in range(NQ):
                descs[q].wait()

                def rmw(j, _, q=q):
                    dv = dlbuf[pl.ds(q * QB + j * 16, 16)]
                    wv = wrow[pl.ds(q * QB + j * 16, 16)]
                    for l in range(16):
                        rows = lax.broadcast(dv[l], (16,))
                        r = j * 16 + l
                        wl = wv[l]
                        for col in range(8):
                            plsc.addupdate_scatter(
                                nacc, [rows, _COLI[col]],
                                rowbuf[q, r, pl.ds(col * 16, 16)] * wl)
                        plsc.addupdate_scatter(dacc, [rows, _COLI[0]],
                                               lane0 * wl)
                    return 0

                lax.fori_loop(0, QB // 16, rmw, 0)
            return 0

        lax.fori_loop(0, nch, chunk, 0)
        _drain(nacc, num_out.at[h], base)
        _drain(dacc, den_out.at[h], base)
        return 0

    lax.fori_loop(0, 4, head, 0)


RB = 1000
_GRID = NN // RB


def _ln(h, g, b):
    mu = jnp.mean(h, axis=-1, keepdims=True)
    var = jnp.mean((h - mu) ** 2, axis=-1, keepdims=True)
    return (h - mu) * lax.rsqrt(var + 1e-5) * g + b


def _rows(c):
    return pl.BlockSpec((RB, c), lambda i: (i, 0))


def _full(*shape):
    nd = len(shape)
    return pl.BlockSpec(shape, lambda i: (0,) * nd)


def _tc1_body(x, W, degp, out):
    dinv = lax.rsqrt(degp[...] + 1.0)
    out[...] = jnp.dot(x[...], W[...], preferred_element_type=F32) * dinv


def _tc1(x, W, degp):
    return pl.pallas_call(
        _tc1_body,
        grid=(_GRID,),
        in_specs=[_rows(256), _full(256, 256), _rows(1)],
        out_specs=_rows(256),
        out_shape=jax.ShapeDtypeStruct((NN, 256), F32),
    )(x, W, degp)


def _tc2_body(acc1, hw1p, degp, b1, g1, be1, Wg, asr, ads, Wr2, br2,
              hw2_o, aa_o, res2_o):
    dinv = lax.rsqrt(degp[...] + 1.0)
    h1 = jax.nn.relu(_ln(dinv * (acc1[...] + hw1p[...]) + b1[...],
                         g1[...], be1[...]))
    hw2 = jnp.dot(h1, Wg[...], preferred_element_type=F32)
    hw2_o[...] = hw2
    heads = []
    for h in range(4):
        blk = hw2[:, h * 128:(h + 1) * 128]
        a_s = jnp.sum(blk * asr[...][h][None, :], axis=1, keepdims=True)
        a_d = jnp.sum(blk * ads[...][h][None, :], axis=1, keepdims=True)
        heads.append(jnp.concatenate([a_s, a_d], axis=1)[None])
    aa_o[...] = jnp.concatenate(heads, axis=0)
    res2_o[...] = jnp.dot(h1, Wr2[...], preferred_element_type=F32) + br2[...]


def _tc2(acc1, hw1p, degp, p):
    return pl.pallas_call(
        _tc2_body,
        grid=(_GRID,),
        in_specs=[_rows(256), _rows(256), _rows(1), _full(256), _full(256),
                  _full(256), _full(256, 512), _full(4, 128), _full(4, 128),
                  _full(256, 128), _full(128)],
        out_specs=[_rows(512), pl.BlockSpec((4, RB, 2), lambda i: (0, i, 0)),
                   _rows(128)],
        out_shape=[jax.ShapeDtypeStruct((NN, 512), F32),
                   jax.ShapeDtypeStruct((4, NN, 2), F32),
                   jax.ShapeDtypeStruct((NN, 128), F32)],
    )(acc1, hw1p, degp, p["gcn1_b"], p["ln1_g"], p["ln1_b"], p["gat2_W"],
      p["gat2_att_src"], p["gat2_att_dst"], p["res2_W"], p["res2_b"])


def _tc3_body(num, den, hw2, aa, res2, bg, g2, be2, Wc, bc,
              h2_o, part_o):
    aam = aa[...]
    hw2m = hw2[...]
    gat = jnp.zeros_like(res2[...])
    for h in range(4):
        esl = aam[h, :, 0:1] + aam[h, :, 1:2]
        wsl = jnp.exp(jnp.maximum(esl, 0.2 * esl))
        nh = num[...][h] + wsl * hw2m[:, h * 128:(h + 1) * 128]
        sh = den[...][h][:, 0:1] + wsl
        gat = gat + nh / (sh + 1e-16)
    gat = gat * 0.25 + bg[...]
    h2 = jax.nn.relu(_ln(gat + res2[...], g2[...], be2[...]))
    h2_o[...] = h2
    part_o[...] = jnp.dot(h2, Wc[...], preferred_element_type=F32) + bc[...]


def _tc3(num, den, hw2, aa, res2, p):
    return pl.pallas_call(
        _tc3_body,
        grid=(_GRID,),
        in_specs=[pl.BlockSpec((4, RB, 128), lambda i: (0, i, 0)),
                  pl.BlockSpec((4, RB, 16), lambda i: (0, i, 0)),
                  _rows(512), pl.BlockSpec((4, RB, 2), lambda i: (0, i, 0)),
                  _rows(128), _full(128), _full(128),
                  _full(128), _full(128, 64), _full(64)],
        out_specs=[_rows(128), _rows(64)],
        out_shape=[jax.ShapeDtypeStruct((NN, 128), F32),
                   jax.ShapeDtypeStruct((NN, 64), F32)],
    )(num, den, hw2, aa, res2, p["gat2_b"], p["ln2_g"], p["ln2_b"],
      p["sage3_Wr"] + p["res3_W"],
      p["res3_b"] + p["sage3_bl"])


def _tc4_body(accS, degp, part, Wl, g3, be3, W4, h4p_o):
    deg = degp[...]
    mean = accS[...] / jnp.maximum(deg, 1.0)
    h3 = jax.nn.relu(_ln(jnp.dot(mean, Wl[...], preferred_element_type=F32)
                         + part[...], g3[...], be3[...]))
    h4p = jnp.dot(h3, W4[...], preferred_element_type=F32) * lax.rsqrt(deg + 1.0)
    h4p_o[...] = jnp.concatenate([h4p, jnp.zeros((RB, 64), F32)], axis=1)


def _tc4(accS, degp, part, p):
    return pl.pallas_call(
        _tc4_body,
        grid=(_GRID,),
        in_specs=[_rows(128), _rows(1), _rows(64), _full(128, 64), _full(64),
                  _full(64), _full(64, 64)],
        out_specs=_rows(128),
        out_shape=jax.ShapeDtypeStruct((NN, 128), F32),
    )(accS, degp, part, p["sage3_Wl"], p["ln3_g"], p["ln3_b"], p["gc4_W"])


def _tc5_body(acc4, h4p, degp, batch3, b4, g4, be4, Wro, bro, pool_o):
    i = pl.program_id(0)
    dinv = lax.rsqrt(degp[...] + 1.0)
    h4 = jax.nn.relu(_ln(dinv * (acc4[...] + h4p[...]) + b4[...],
                         g4[...], be4[...]))
    gate = jax.nn.sigmoid(jnp.dot(h4, Wro[...], preferred_element_type=F32) + bro[...])
    gated = h4 * gate
    b = batch3[...][0, 0, :]
    P = (lax.broadcasted_iota(I32, (64, RB), 0) == b[None, :]).astype(F32)
    rhs = jnp.concatenate([gated, gate, jnp.zeros((RB, 63), F32)], axis=1)
    blk = jnp.dot(P, rhs, preferred_element_type=F32)

    @pl.when(i == 0)
    def _():
        pool_o[...] = jnp.zeros_like(pool_o)

    pool_o[...] += blk


def _tc5(acc4, h4p, degp, batch3, p):
    return pl.pallas_call(
        _tc5_body,
        grid=(_GRID,),
        in_specs=[_rows(64), _rows(64), _rows(1),
                  pl.BlockSpec((1, 1, RB), lambda i: (i, 0, 0)),
                  _full(64), _full(64), _full(64), _full(64, 1), _full(1)],
        out_specs=_full(64, 128),
        out_shape=jax.ShapeDtypeStruct((64, 128), F32),
    )(acc4, h4p, degp, batch3, p["gc4_b"], p["ln4_g"], p["ln4_b"],
      p["ro_W"], p["ro_b"])


def _tc6_body(pool, link3, W1, b1, W2, b2, W3, b3, out_o):
    pm = pool[...]
    emb = pm[:, :64] / (pm[:, 64:65] + 1e-8)
    lk = link3[...][:, 0, :]
    g_iota = lax.broadcasted_iota(I32, (4096, 64), 1)
    o1 = (lk[0][:, None] == g_iota).astype(F32)
    o2 = (lk[1][:, None] == g_iota).astype(F32)
    f = jnp.concatenate([
        jnp.dot(o1, emb, preferred_element_type=F32),
        jnp.dot(o2, emb, preferred_element_type=F32)], axis=1)
    f = jax.nn.relu(jnp.dot(f, W1[...], preferred_element_type=F32) + b1[...])
    f = jax.nn.relu(jnp.dot(f, W2[...], preferred_element_type=F32) + b2[...])
    out_o[...] = jax.nn.sigmoid(jnp.dot(f, W3[...], preferred_element_type=F32) + b3[...])


def _tc6(pool, link3, p):
    return pl.pallas_call(
        _tc6_body,
        in_specs=[pl.BlockSpec((64, 128), lambda: (0, 0)),
                  pl.BlockSpec((2, 1, 4096), lambda: (0, 0, 0)),
                  pl.BlockSpec((128, 64), lambda: (0, 0)),
                  pl.BlockSpec((64,), lambda: (0,)),
                  pl.BlockSpec((64, 64), lambda: (0, 0)),
                  pl.BlockSpec((64,), lambda: (0,)),
                  pl.BlockSpec((64, 1), lambda: (0, 0)),
                  pl.BlockSpec((1,), lambda: (0,)),
                  ],
        out_specs=pl.BlockSpec((4096, 1), lambda: (0, 0)),
        out_shape=jax.ShapeDtypeStruct((4096, 1), F32),
    )(pool, link3, p["mlp1_W"], p["mlp1_b"], p["mlp2_W"], p["mlp2_b"],
      p["mlp3_W"], p["mlp3_b"])


def kernel(x, edge_index, batch, link_indices, params):
    p = params

    clist, counts, deg = _sc_scan(edge_index)
    degp = deg[:NN, None]
    hw1p = _tc1(x, p["gcn1_W"], degp)
    acc1 = _sc_rowsum256(hw1p, clist, counts)[:NN]
    hw2, aa, res2 = _tc2(acc1, hw1p, degp, p)
    num, den = _sc_gat(hw2.reshape(NN * 4, 128),
                       aa.reshape(4, 1, 2 * NN), clist, counts)
    h2, part = _tc3(num[:, :NN], den[:, :NN], hw2, aa, res2, p)
    accS = _sc_rowsum128(h2, clist, counts)[:NN]
    h4p = _tc4(accS, degp, part, p)
    acc4 = _sc_rowsum128(h4p, clist, counts)[:NN, :64]
    pool = _tc5(acc4, h4p[:, :64], degp, batch.reshape(_GRID, 1, RB), p)
    out = _tc6(pool, link_indices.reshape(2, 1, 4096), p)
    return out[:, 0]


# speedup vs baseline: 1.1775x; 1.0097x over previous
import functools
import jax
import jax.numpy as jnp
from jax import lax
from jax.experimental import pallas as pl
from jax.experimental.pallas import tpu as pltpu
from jax.experimental.pallas import tpu_sc as plsc

F32 = jnp.float32
I32 = jnp.int32

NN = 10000
EE = 160000
NT = 32
RNG = 320
NROW = 328
CH = 128
SCAN_B = 2560
NSCAN = EE // SCAN_B
CLCAP = 10368
SPILL = 10240
CLW = EE + 128

_MESH = dict(core_axis_name="c", subcore_axis_name="s",
             num_cores=2, num_subcores=16)
_PARAMS = pltpu.CompilerParams(needs_layout_passes=False)


def _coli():
    io = lax.broadcasted_iota(I32, (16,), 0)
    return [io + c * 16 for c in range(16)]


class _ColI:
    def __getitem__(self, c):
        io = lax.broadcasted_iota(I32, (16,), 0)
        return io + c * 16


_COLI = _ColI()


def _worker(c, s):
    return c * 16 + s


@functools.partial(
    pl.kernel,
    out_type=(
        jax.ShapeDtypeStruct((NT, 1, CLW), I32),
        jax.ShapeDtypeStruct((NT, 1, 16), I32),
        jax.ShapeDtypeStruct((NT * RNG,), F32),
    ),
    mesh=plsc.VectorSubcoreMesh(**_MESH),
    compiler_params=_PARAMS,
    scratch_types=[
        pltpu.VMEM((2, SCAN_B), I32),
        pltpu.VMEM((CLCAP,), I32),
        pltpu.VMEM((NROW,), F32),
        pltpu.VMEM((16,), I32),
    ],
)
def _sc_scan(ei, clist, counts, deg, ebuf, clbuf, dacc, cstg):
    c = lax.axis_index("c")
    s = lax.axis_index("s")
    t = _worker(c, s)
    base = t * RNG
    iota = lax.broadcasted_iota(I32, (16,), 0)
    ones16 = jnp.ones((16,), F32)
    zero16 = jnp.zeros((16,), F32)

    def dz(r, _):
        dacc[pl.ds(r * 16, 16)] = zero16
        return 0

    lax.fori_loop(0, NROW // 16, dz, 0)

    def chunk(g, carry):
        pltpu.sync_copy(ei.at[:, pl.ds(g * SCAN_B, SCAN_B)], ebuf)

        def group(j, carry):
            off, hoff = carry
            s16 = ebuf[0, pl.ds(j * 16, 16)]
            d16 = ebuf[1, pl.ds(j * 16, 16)]
            ok = (d16 >= base) & (d16 < base + RNG)
            key = jnp.where(ok, iota, 16)
            _, sv = plsc.sort_key_val(key, s16 * 16384 + d16)
            clbuf[pl.ds(off, 16)] = sv
            cntv = plsc.all_reduce_population_count(ok)
            off = off + cntv[0]
            loc = jnp.clip(d16 - base, 0, RNG)
            plsc.addupdate_scatter(dacc, [loc], jnp.where(ok, ones16, zero16))

            @pl.when(off >= SPILL)
            def _():
                pltpu.sync_copy(
                    clbuf.at[pl.ds(0, SPILL)],
                    clist.at[t, 0, pl.ds(pl.multiple_of(hoff, 128), SPILL)])
                tail = clbuf[pl.ds(SPILL, 16)]
                clbuf[pl.ds(0, 16)] = tail

            spilled = off >= SPILL
            off = jnp.where(spilled, off - SPILL, off)
            hoff = jnp.where(spilled, hoff + SPILL, hoff)
            return (off, hoff)

        return lax.fori_loop(0, SCAN_B // 16, group, carry)

    off, hoff = lax.fori_loop(0, NSCAN, chunk, (0, 0))

    padv = jnp.zeros((16,), I32) + (base + RNG)
    for i in range(8):
        clbuf[pl.ds(off + i * 16, 16)] = padv
    nsp = (off + 127) // 128

    def spill_fin(g, _):
        o = pl.multiple_of(g * 128, 128)
        pltpu.sync_copy(
            clbuf.at[pl.ds(o, 128)],
            clist.at[t, 0, pl.ds(pl.multiple_of(hoff, 128) + o, 128)])
        return 0

    lax.fori_loop(0, nsp, spill_fin, 0)
    cstg[pl.ds(0, 16)] = jnp.zeros((16,), I32) + (hoff + nsp * 128)
    pltpu.sync_copy(cstg, counts.at[t, 0])
    pltpu.sync_copy(dacc.at[pl.ds(0, RNG)], deg.at[pl.ds(base, RNG)])


def _read_count(counts, cntb, t):
    pltpu.sync_copy(counts.at[t, 0], cntb)
    return cntb[pl.ds(0, 16)][0]


def _unpack_chunk(gibuf, dlbuf, base):
    for j in range(CH // 16):
        v16 = gibuf[pl.ds(j * 16, 16)]
        d16 = v16 & 16383
        dlbuf[pl.ds(j * 16, 16)] = jnp.clip(d16 - base, 0, RNG)
        gibuf[pl.ds(j * 16, 16)] = lax.shift_right_logical(v16, 14)


def _drain(acc, out_slice, base):
    for k in range(RNG // 64):
        pltpu.sync_copy(acc.at[pl.ds(k * 64, 64)],
                        out_slice.at[pl.ds(base + k * 64, 64)])


def _zero_acc(acc, w):
    z = jnp.zeros((16,), F32)

    def za(r, _):
        for j in range(w // 16):
            acc[r, pl.ds(j * 16, 16)] = z
        return 0

    lax.fori_loop(0, NROW, za, 0)


NQ = 4
QB = CH // NQ


def _make_rowsum(w):
    @functools.partial(
        pl.kernel,
        out_type=jax.ShapeDtypeStruct((NT * RNG, w), F32),
        mesh=plsc.VectorSubcoreMesh(**_MESH),
        compiler_params=_PARAMS,
        scratch_types=[
            pltpu.VMEM((NROW, w), F32),
            pltpu.VMEM((NQ, QB, w), F32),
            pltpu.VMEM((CH,), I32),
            pltpu.VMEM((CH,), I32),
            pltpu.VMEM((16,), I32),
        ] + [pltpu.SemaphoreType.DMA] * NQ,
    )
    def rowsum(table, clist, counts, out, acc, rowbuf, gibuf, dlbuf, cntb,
               *sems):
        c = lax.axis_index("c")
        s = lax.axis_index("s")
        t = _worker(c, s)
        base = t * RNG
        _zero_acc(acc, w)
        nch = _read_count(counts, cntb, t) // CH

        def chunk(g, _):
            pltpu.sync_copy(clist.at[t, 0, pl.ds(g * CH, CH)], gibuf)
            _unpack_chunk(gibuf, dlbuf, base)
            descs = [
                pltpu.async_copy(table.at[gibuf.at[pl.ds(q * QB, QB)]],
                                 rowbuf.at[q], sems[q])
                for q in range(NQ)]
            for q in range(NQ):
                descs[q].wait()

                def rmw(j, _, q=q):
                    dv = dlbuf[pl.ds(q * QB + j * 16, 16)]
                    for l in range(16):
                        rows = lax.broadcast(dv[l], (16,))
                        r = j * 16 + l
                        for col in range(w // 16):
                            plsc.addupdate_scatter(
                                acc, [rows, _COLI[col]],
                                rowbuf[q, r, pl.ds(col * 16, 16)])
                    return 0

                lax.fori_loop(0, QB // 16, rmw, 0)
            return 0

        lax.fori_loop(0, nch, chunk, 0)
        _drain(acc, out, base)

    return rowsum


_sc_rowsum256 = _make_rowsum(256)
_sc_rowsum128 = _make_rowsum(128)


@functools.partial(
    pl.kernel,
    out_type=(
        jax.ShapeDtypeStruct((4, NT * RNG, 128), F32),
        jax.ShapeDtypeStruct((4, NT * RNG, 16), F32),
    ),
    mesh=plsc.VectorSubcoreMesh(**_MESH),
    compiler_params=_PARAMS,
    scratch_types=[
        pltpu.VMEM((NROW, 128), F32),
        pltpu.VMEM((NROW, 16), F32),
        pltpu.VMEM((2 * NN,), F32),
        pltpu.VMEM((NQ, QB, 128), F32),
        pltpu.VMEM((CH,), F32),
        pltpu.VMEM((CH,), I32),
        pltpu.VMEM((CH,), I32),
        pltpu.VMEM((16,), I32),
    ] + [pltpu.SemaphoreType.DMA] * NQ,
)
def _sc_gat(hw2, aa4, clist, counts, num_out, den_out,
            nacc, dacc, aav, rowbuf, wrow, gibuf, dlbuf, cntb, *sems):
    c = lax.axis_index("c")
    s = lax.axis_index("s")
    t = _worker(c, s)
    base = t * RNG
    lane0 = jnp.where(lax.broadcasted_iota(I32, (16,), 0) == 0, 1.0, 0.0)
    nch = _read_count(counts, cntb, t) // CH

    def head(h, _):
        pltpu.sync_copy(aa4.at[h, 0], aav)
        _zero_acc(nacc, 128)
        _zero_acc(dacc, 16)

        def chunk(g, _):
            pltpu.sync_copy(clist.at[t, 0, pl.ds(g * CH, CH)], gibuf)
            for j in range(CH // 16):
                v16 = gibuf[pl.ds(j * 16, 16)]
                d16 = v16 & 16383
                s16 = lax.shift_right_logical(v16, 14)
                a_s = plsc.load_gather(aav, [s16 * 2])
                a_d = plsc.load_gather(aav, [jnp.minimum(d16, NN - 1) * 2 + 1])
                e = a_s + a_d
                e = jnp.maximum(e, 0.2 * e)
                wrow[pl.ds(j * 16, 16)] = jnp.exp(e)
                dlbuf[pl.ds(j * 16, 16)] = jnp.clip(d16 - base, 0, RNG)
                gibuf[pl.ds(j * 16, 16)] = s16 * 4 + h
            descs = [
                pltpu.async_copy(hw2.at[gibuf.at[pl.ds(q * QB, QB)]],
                                 rowbuf.at[q], sems[q])
                for q in range(NQ)]
            for q in range(NQ):
                descs[q].wait()

                def rmw(j, _, q=q):
                    dv = dlbuf[pl.ds(q * QB + j * 16, 16)]
                    wv = wrow[pl.ds(q * QB + j * 16, 16)]
                    for l in range(16):
                        rows = lax.broadcast(dv[l], (16,))
                        r = j * 16 + l
                        wl = wv[l]
                        for col in range(8):
                            plsc.addupdate_scatter(
                                nacc, [rows, _COLI[col]],
                                rowbuf[q, r, pl.ds(col * 16, 16)] * wl)
                        plsc.addupdate_scatter(dacc, [rows, _COLI[0]],
                                               lane0 * wl)
                    return 0

                lax.fori_loop(0, QB // 16, rmw, 0)
            return 0

        lax.fori_loop(0, nch, chunk, 0)
        _drain(nacc, num_out.at[h], base)
        _drain(dacc, den_out.at[h], base)
        return 0

    lax.fori_loop(0, 4, head, 0)


RB = 1000
_GRID = NN // RB


def _ln(h, g, b):
    mu = jnp.mean(h, axis=-1, keepdims=True)
    var = jnp.mean((h - mu) ** 2, axis=-1, keepdims=True)
    return (h - mu) * lax.rsqrt(var + 1e-5) * g + b


def _rows(c):
    return pl.BlockSpec((RB, c), lambda i: (i, 0))


def _full(*shape):
    nd = len(shape)
    return pl.BlockSpec(shape, lambda i: (0,) * nd)


def _tc1_body(x, W, degp, out):
    dinv = lax.rsqrt(degp[...] + 1.0)
    out[...] = jnp.dot(x[...], W[...], preferred_element_type=F32) * dinv


def _tc1(x, W, degp):
    return pl.pallas_call(
        _tc1_body,
        grid=(_GRID,),
        in_specs=[_rows(256), _full(256, 256), _rows(1)],
        out_specs=_rows(256),
        out_shape=jax.ShapeDtypeStruct((NN, 256), F32),
    )(x, W, degp)


def _tc2_body(acc1, hw1p, degp, b1, g1, be1, Wg, asr, ads, Wr2, br2,
              hw2_o, aa_o, res2_o):
    dinv = lax.rsqrt(degp[...] + 1.0)
    h1 = jax.nn.relu(_ln(dinv * (acc1[...] + hw1p[...]) + b1[...],
                         g1[...], be1[...]))
    hw2 = jnp.dot(h1, Wg[...], preferred_element_type=F32)
    hw2_o[...] = hw2
    heads = []
    for h in range(4):
        blk = hw2[:, h * 128:(h + 1) * 128]
        a_s = jnp.sum(blk * asr[...][h][None, :], axis=1, keepdims=True)
        a_d = jnp.sum(blk * ads[...][h][None, :], axis=1, keepdims=True)
        heads.append(jnp.concatenate([a_s, a_d], axis=1)[None])
    aa_o[...] = jnp.concatenate(heads, axis=0)
    res2_o[...] = jnp.dot(h1, Wr2[...], preferred_element_type=F32) + br2[...]


def _tc2(acc1, hw1p, degp, p):
    return pl.pallas_call(
        _tc2_body,
        grid=(_GRID,),
        in_specs=[_rows(256), _rows(256), _rows(1), _full(256), _full(256),
                  _full(256), _full(256, 512), _full(4, 128), _full(4, 128),
                  _full(256, 128), _full(128)],
        out_specs=[_rows(512), pl.BlockSpec((4, RB, 2), lambda i: (0, i, 0)),
                   _rows(128)],
        out_shape=[jax.ShapeDtypeStruct((NN, 512), F32),
                   jax.ShapeDtypeStruct((4, NN, 2), F32),
                   jax.ShapeDtypeStruct((NN, 128), F32)],
    )(acc1, hw1p, degp, p["gcn1_b"], p["ln1_g"], p["ln1_b"], p["gat2_W"],
      p["gat2_att_src"], p["gat2_att_dst"], p["res2_W"], p["res2_b"])


def _tc3_body(num, den, hw2, aa, res2, bg, g2, be2, Wc, bc,
              h2_o, part_o):
    aam = aa[...]
    hw2m = hw2[...]
    gat = jnp.zeros_like(res2[...])
    for h in range(4):
        esl = aam[h, :, 0:1] + aam[h, :, 1:2]
        wsl = jnp.exp(jnp.maximum(esl, 0.2 * esl))
        nh = num[...][h] + wsl * hw2m[:, h * 128:(h + 1) * 128]
        sh = den[...][h][:, 0:1] + wsl
        gat = gat + nh / (sh + 1e-16)
    gat = gat * 0.25 + bg[...]
    h2 = jax.nn.relu(_ln(gat + res2[...], g2[...], be2[...]))
    h2_o[...] = h2
    part_o[...] = jnp.dot(h2, Wc[...], preferred_element_type=F32) + bc[...]


def _tc3(num, den, hw2, aa, res2, p):
    return pl.pallas_call(
        _tc3_body,
        grid=(_GRID,),
        in_specs=[pl.BlockSpec((4, RB, 128), lambda i: (0, i, 0)),
                  pl.BlockSpec((4, RB, 16), lambda i: (0, i, 0)),
                  _rows(512), pl.BlockSpec((4, RB, 2), lambda i: (0, i, 0)),
                  _rows(128), _full(128), _full(128),
                  _full(128), _full(128, 64), _full(64)],
        out_specs=[_rows(128), _rows(64)],
        out_shape=[jax.ShapeDtypeStruct((NN, 128), F32),
                   jax.ShapeDtypeStruct((NN, 64), F32)],
    )(num, den, hw2, aa, res2, p["gat2_b"], p["ln2_g"], p["ln2_b"],
      p["sage3_Wr"] + p["res3_W"],
      p["res3_b"] + p["sage3_bl"])


def _tc4_body(accS, degp, part, Wl, g3, be3, W4, h4p_o):
    deg = degp[...]
    mean = accS[...] / jnp.maximum(deg, 1.0)
    h3 = jax.nn.relu(_ln(jnp.dot(mean, Wl[...], preferred_element_type=F32)
                         + part[...], g3[...], be3[...]))
    h4p = jnp.dot(h3, W4[...], preferred_element_type=F32) * lax.rsqrt(deg + 1.0)
    h4p_o[...] = jnp.concatenate([h4p, jnp.zeros((RB, 64), F32)], axis=1)


def _tc4(accS, degp, part, p):
    return pl.pallas_call(
        _tc4_body,
        grid=(_GRID,),
        in_specs=[_rows(128), _rows(1), _rows(64), _full(128, 64), _full(64),
                  _full(64), _full(64, 64)],
        out_specs=_rows(128),
        out_shape=jax.ShapeDtypeStruct((NN, 128), F32),
    )(accS, degp, part, p["sage3_Wl"], p["ln3_g"], p["ln3_b"], p["gc4_W"])


def _tc5_body(acc4, h4p, degp, batch3, b4, g4, be4, Wro, bro, pool_o):
    i = pl.program_id(0)
    dinv = lax.rsqrt(degp[...] + 1.0)
    h4 = jax.nn.relu(_ln(dinv * (acc4[...] + h4p[...]) + b4[...],
                         g4[...], be4[...]))
    gate = jax.nn.sigmoid(jnp.dot(h4, Wro[...], preferred_element_type=F32) + bro[...])
    gated = h4 * gate
    b = batch3[...][0, 0, :]
    P = (lax.broadcasted_iota(I32, (64, RB), 0) == b[None, :]).astype(F32)
    rhs = jnp.concatenate([gated, gate, jnp.zeros((RB, 63), F32)], axis=1)
    blk = jnp.dot(P, rhs, preferred_element_type=F32)

    @pl.when(i == 0)
    def _():
        pool_o[...] = jnp.zeros_like(pool_o)

    pool_o[...] += blk


def _tc5(acc4, h4p, degp, batch3, p):
    return pl.pallas_call(
        _tc5_body,
        grid=(_GRID,),
        in_specs=[_rows(64), _rows(64), _rows(1),
                  pl.BlockSpec((1, 1, RB), lambda i: (i, 0, 0)),
                  _full(64), _full(64), _full(64), _full(64, 1), _full(1)],
        out_specs=_full(64, 128),
        out_shape=jax.ShapeDtypeStruct((64, 128), F32),
    )(acc4, h4p, degp, batch3, p["gc4_b"], p["ln4_g"], p["ln4_b"],
      p["ro_W"], p["ro_b"])


def _tc6_body(pool, link3, W1, b1, W2, b2, W3, b3, out_o):
    pm = pool[...]
    emb = pm[:, :64] / (pm[:, 64:65] + 1e-8)
    lk = link3[...][:, 0, :]
    g_iota = lax.broadcasted_iota(I32, (4096, 64), 1)
    o1 = (lk[0][:, None] == g_iota).astype(F32)
    o2 = (lk[1][:, None] == g_iota).astype(F32)
    f = jnp.concatenate([
        jnp.dot(o1, emb, preferred_element_type=F32),
        jnp.dot(o2, emb, preferred_element_type=F32)], axis=1)
    f = jax.nn.relu(jnp.dot(f, W1[...], preferred_element_type=F32) + b1[...])
    f = jax.nn.relu(jnp.dot(f, W2[...], preferred_element_type=F32) + b2[...])
    out_o[...] = jax.nn.sigmoid(jnp.dot(f, W3[...], preferred_element_type=F32) + b3[...])


def _tc6(pool, link3, p):
    return pl.pallas_call(
        _tc6_body,
        in_specs=[pl.BlockSpec((64, 128), lambda: (0, 0)),
                  pl.BlockSpec((2, 1, 4096), lambda: (0, 0, 0)),
                  pl.BlockSpec((128, 64), lambda: (0, 0)),
                  pl.BlockSpec((64,), lambda: (0,)),
                  pl.BlockSpec((64, 64), lambda: (0, 0)),
                  pl.BlockSpec((64,), lambda: (0,)),
                  pl.BlockSpec((64, 1), lambda: (0, 0)),
                  pl.BlockSpec((1,), lambda: (0,)),
                  ],
        out_specs=pl.BlockSpec((4096, 1), lambda: (0, 0)),
        out_shape=jax.ShapeDtypeStruct((4096, 1), F32),
    )(pool, link3, p["mlp1_W"], p["mlp1_b"], p["mlp2_W"], p["mlp2_b"],
      p["mlp3_W"], p["mlp3_b"])


def kernel(x, edge_index, batch, link_indices, params):
    p = params

    clist, counts, deg = _sc_scan(edge_index)
    degp = deg[:NN, None]
    hw1p = _tc1(x, p["gcn1_W"], degp)
    acc1 = _sc_rowsum256(hw1p, clist, counts)[:NN]
    hw2, aa, res2 = _tc2(acc1, hw1p, degp, p)
    num, den = _sc_gat(hw2.reshape(NN * 4, 128),
                       aa.reshape(4, 1, 2 * NN), clist, counts)
    h2, part = _tc3(num[:, :NN], den[:, :NN], hw2, aa, res2, p)
    accS = _sc_rowsum128(h2, clist, counts)[:NN]
    h4p = _tc4(accS, degp, part, p)
    acc4 = _sc_rowsum128(h4p, clist, counts)[:NN, :64]
    pool = _tc5(acc4, h4p[:, :64], degp, batch.reshape(_GRID, 1, RB), p)
    out = _tc6(pool, link_indices.reshape(2, 1, 4096), p)
    return out[:, 0]


# speedup vs baseline: 1.2186x; 1.0350x over previous
import functools
import jax
import jax.numpy as jnp
from jax import lax
from jax.experimental import pallas as pl
from jax.experimental.pallas import tpu as pltpu
from jax.experimental.pallas import tpu_sc as plsc

F32 = jnp.float32
I32 = jnp.int32

NN = 10000
EE = 160000
NT = 32
RNG = 320
NROW = 328
CH = 128
SCAN_B = 2560
NSCAN = EE // SCAN_B
CLCAP = 10368
SPILL = 10240
CLW = EE + 128

_MESH = dict(core_axis_name="c", subcore_axis_name="s",
             num_cores=2, num_subcores=16)
_PARAMS = pltpu.CompilerParams(needs_layout_passes=False)


def _coli():
    io = lax.broadcasted_iota(I32, (16,), 0)
    return [io + c * 16 for c in range(16)]


class _ColI:
    def __getitem__(self, c):
        io = lax.broadcasted_iota(I32, (16,), 0)
        return io + c * 16


_COLI = _ColI()


def _worker(c, s):
    return c * 16 + s


@functools.partial(
    pl.kernel,
    out_type=(
        jax.ShapeDtypeStruct((NT, 1, CLW), I32),
        jax.ShapeDtypeStruct((NT, 1, 16), I32),
        jax.ShapeDtypeStruct((NT * RNG,), F32),
    ),
    mesh=plsc.VectorSubcoreMesh(**_MESH),
    compiler_params=_PARAMS,
    scratch_types=[
        pltpu.VMEM((2, SCAN_B), I32),
        pltpu.VMEM((CLCAP,), I32),
        pltpu.VMEM((NROW,), F32),
        pltpu.VMEM((16,), I32),
    ],
)
def _sc_scan(ei, clist, counts, deg, ebuf, clbuf, dacc, cstg):
    c = lax.axis_index("c")
    s = lax.axis_index("s")
    t = _worker(c, s)
    base = t * RNG
    iota = lax.broadcasted_iota(I32, (16,), 0)
    ones16 = jnp.ones((16,), F32)
    zero16 = jnp.zeros((16,), F32)

    def dz(r, _):
        dacc[pl.ds(r * 16, 16)] = zero16
        return 0

    lax.fori_loop(0, NROW // 16, dz, 0)

    def chunk(g, carry):
        pltpu.sync_copy(ei.at[:, pl.ds(g * SCAN_B, SCAN_B)], ebuf)

        def group(j, carry):
            off, hoff = carry
            s16 = ebuf[0, pl.ds(j * 16, 16)]
            d16 = ebuf[1, pl.ds(j * 16, 16)]
            ok = (d16 >= base) & (d16 < base + RNG)
            key = jnp.where(ok, iota, 16)
            _, sv = plsc.sort_key_val(key, s16 * 16384 + d16)
            clbuf[pl.ds(off, 16)] = sv
            cntv = plsc.all_reduce_population_count(ok)
            off = off + cntv[0]
            loc = jnp.clip(d16 - base, 0, RNG)
            plsc.addupdate_scatter(dacc, [loc], jnp.where(ok, ones16, zero16))

            @pl.when(off >= SPILL)
            def _():
                pltpu.sync_copy(
                    clbuf.at[pl.ds(0, SPILL)],
                    clist.at[t, 0, pl.ds(pl.multiple_of(hoff, 128), SPILL)])
                tail = clbuf[pl.ds(SPILL, 16)]
                clbuf[pl.ds(0, 16)] = tail

            spilled = off >= SPILL
            off = jnp.where(spilled, off - SPILL, off)
            hoff = jnp.where(spilled, hoff + SPILL, hoff)
            return (off, hoff)

        return lax.fori_loop(0, SCAN_B // 16, group, carry)

    off, hoff = lax.fori_loop(0, NSCAN, chunk, (0, 0))

    padv = jnp.zeros((16,), I32) + (base + RNG)
    for i in range(8):
        clbuf[pl.ds(off + i * 16, 16)] = padv
    nsp = (off + 127) // 128

    def spill_fin(g, _):
        o = pl.multiple_of(g * 128, 128)
        pltpu.sync_copy(
            clbuf.at[pl.ds(o, 128)],
            clist.at[t, 0, pl.ds(pl.multiple_of(hoff, 128) + o, 128)])
        return 0

    lax.fori_loop(0, nsp, spill_fin, 0)
    cstg[pl.ds(0, 16)] = jnp.zeros((16,), I32) + (hoff + nsp * 128)
    pltpu.sync_copy(cstg, counts.at[t, 0])
    pltpu.sync_copy(dacc.at[pl.ds(0, RNG)], deg.at[pl.ds(base, RNG)])


def _read_count(counts, cntb, t):
    pltpu.sync_copy(counts.at[t, 0], cntb)
    return cntb[pl.ds(0, 16)][0]


def _unpack_chunk(gibuf, dlbuf, base):
    for j in range(CH // 16):
        v16 = gibuf[pl.ds(j * 16, 16)]
        d16 = v16 & 16383
        dlbuf[pl.ds(j * 16, 16)] = jnp.clip(d16 - base, 0, RNG)
        gibuf[pl.ds(j * 16, 16)] = lax.shift_right_logical(v16, 14)


def _drain(acc, out_slice, base):
    for k in range(RNG // 64):
        pltpu.sync_copy(acc.at[pl.ds(k * 64, 64)],
                        out_slice.at[pl.ds(base + k * 64, 64)])


def _zero_acc(acc, w):
    z = jnp.zeros((16,), F32)

    def za(r, _):
        for j in range(w // 16):
            acc[r, pl.ds(j * 16, 16)] = z
        return 0

    lax.fori_loop(0, NROW, za, 0)


NQ = 4
QB = CH // NQ


def _make_rowsum(w):
    @functools.partial(
        pl.kernel,
        out_type=jax.ShapeDtypeStruct((NT * RNG, w), F32),
        mesh=plsc.VectorSubcoreMesh(**_MESH),
        compiler_params=_PARAMS,
        scratch_types=[
            pltpu.VMEM((NROW, w), F32),
            pltpu.VMEM((NQ, QB, w), F32),
            pltpu.VMEM((CH,), I32),
            pltpu.VMEM((CH,), I32),
            pltpu.VMEM((16,), I32),
        ] + [pltpu.SemaphoreType.DMA] * NQ,
    )
    def rowsum(table, clist, counts, out, acc, rowbuf, gibuf, dlbuf, cntb,
               *sems):
        c = lax.axis_index("c")
        s = lax.axis_index("s")
        t = _worker(c, s)
        base = t * RNG
        _zero_acc(acc, w)
        nch = _read_count(counts, cntb, t) // CH

        def chunk(g, _):
            pltpu.sync_copy(clist.at[t, 0, pl.ds(g * CH, CH)], gibuf)
            _unpack_chunk(gibuf, dlbuf, base)
            descs = [
                pltpu.async_copy(table.at[gibuf.at[pl.ds(q * QB, QB)]],
                                 rowbuf.at[q], sems[q])
                for q in range(NQ)]
            for q in range(NQ):
                descs[q].wait()

                def rmw(j, _, q=q):
                    dv = dlbuf[pl.ds(q * QB + j * 16, 16)]
                    for l in range(16):
                        rows = lax.broadcast(dv[l], (16,))
                        r = j * 16 + l
                        for col in range(w // 16):
                            plsc.addupdate_scatter(
                                acc, [rows, _COLI[col]],
                                rowbuf[q, r, pl.ds(col * 16, 16)])
                    return 0

                lax.fori_loop(0, QB // 16, rmw, 0)
            return 0

        lax.fori_loop(0, nch, chunk, 0)
        _drain(acc, out, base)

    return rowsum


_sc_rowsum256 = _make_rowsum(256)
_sc_rowsum128 = _make_rowsum(128)


@functools.partial(
    pl.kernel,
    out_type=(
        jax.ShapeDtypeStruct((4, NT * RNG, 128), F32),
        jax.ShapeDtypeStruct((4, NT * RNG, 16), F32),
    ),
    mesh=plsc.VectorSubcoreMesh(**_MESH),
    compiler_params=_PARAMS,
    scratch_types=[
        pltpu.VMEM((NROW, 128), F32),
        pltpu.VMEM((NROW, 16), F32),
        pltpu.VMEM((2 * NN,), F32),
        pltpu.VMEM((CH, 128), F32),
        pltpu.VMEM((CH,), F32),
        pltpu.VMEM((CH,), I32),
        pltpu.VMEM((CH,), I32),
        pltpu.VMEM((16,), I32),
    ] + [pltpu.SemaphoreType.DMA] * NQ,
)
def _sc_gat(hw2, aa4, clist, counts, num_out, den_out,
            nacc, dacc, aav, rowbuf, wrow, gibuf, dlbuf, cntb, *sems):
    c = lax.axis_index("c")
    s = lax.axis_index("s")
    t = _worker(c, s)
    base = t * RNG
    lane0 = jnp.where(lax.broadcasted_iota(I32, (16,), 0) == 0, 1.0, 0.0)
    nch = _read_count(counts, cntb, t) // CH

    def head(h, _):
        pltpu.sync_copy(aa4.at[h, 0], aav)
        _zero_acc(nacc, 128)
        _zero_acc(dacc, 16)

        def chunk(g, _):
            pltpu.sync_copy(clist.at[t, 0, pl.ds(g * CH, CH)], gibuf)
            for j in range(CH // 16):
                v16 = gibuf[pl.ds(j * 16, 16)]
                d16 = v16 & 16383
                s16 = lax.shift_right_logical(v16, 14)
                a_s = plsc.load_gather(aav, [s16 * 2])
                a_d = plsc.load_gather(aav, [jnp.minimum(d16, NN - 1) * 2 + 1])
                e = a_s + a_d
                e = jnp.maximum(e, 0.2 * e)
                wrow[pl.ds(j * 16, 16)] = jnp.exp(e)
                dlbuf[pl.ds(j * 16, 16)] = jnp.clip(d16 - base, 0, RNG)
                gibuf[pl.ds(j * 16, 16)] = s16 * 4 + h
            pltpu.async_copy(hw2.at[gibuf], rowbuf, sems[0]).wait()

            def rmw(j, _):
                dv = dlbuf[pl.ds(j * 16, 16)]
                wv = wrow[pl.ds(j * 16, 16)]
                for l in range(16):
                    rows = lax.broadcast(dv[l], (16,))
                    r = j * 16 + l
                    wl = wv[l]
                    for col in range(8):
                        plsc.addupdate_scatter(
                            nacc, [rows, _COLI[col]],
                            rowbuf[r, pl.ds(col * 16, 16)] * wl)
                    plsc.addupdate_scatter(dacc, [rows, _COLI[0]],
                                           lane0 * wl)
                return 0

            lax.fori_loop(0, CH // 16, rmw, 0)
            return 0

        lax.fori_loop(0, nch, chunk, 0)
        _drain(nacc, num_out.at[h], base)
        _drain(dacc, den_out.at[h], base)
        return 0

    lax.fori_loop(0, 4, head, 0)


RB = 1000
_GRID = NN // RB


def _ln(h, g, b):
    mu = jnp.mean(h, axis=-1, keepdims=True)
    var = jnp.mean((h - mu) ** 2, axis=-1, keepdims=True)
    return (h - mu) * lax.rsqrt(var + 1e-5) * g + b


def _rows(c):
    return pl.BlockSpec((RB, c), lambda i: (i, 0))


def _full(*shape):
    nd = len(shape)
    return pl.BlockSpec(shape, lambda i: (0,) * nd)


def _tc1_body(x, W, degp, out):
    dinv = lax.rsqrt(degp[...] + 1.0)
    out[...] = jnp.dot(x[...], W[...], preferred_element_type=F32) * dinv


def _tc1(x, W, degp):
    return pl.pallas_call(
        _tc1_body,
        grid=(_GRID,),
        in_specs=[_rows(256), _full(256, 256), _rows(1)],
        out_specs=_rows(256),
        out_shape=jax.ShapeDtypeStruct((NN, 256), F32),
    )(x, W, degp)


def _tc2_body(acc1, hw1p, degp, b1, g1, be1, Wg, asr, ads, Wr2, br2,
              hw2_o, aa_o, res2_o):
    dinv = lax.rsqrt(degp[...] + 1.0)
    h1 = jax.nn.relu(_ln(dinv * (acc1[...] + hw1p[...]) + b1[...],
                         g1[...], be1[...]))
    hw2 = jnp.dot(h1, Wg[...], preferred_element_type=F32)
    hw2_o[...] = hw2
    heads = []
    for h in range(4):
        blk = hw2[:, h * 128:(h + 1) * 128]
        a_s = jnp.sum(blk * asr[...][h][None, :], axis=1, keepdims=True)
        a_d = jnp.sum(blk * ads[...][h][None, :], axis=1, keepdims=True)
        heads.append(jnp.concatenate([a_s, a_d], axis=1)[None])
    aa_o[...] = jnp.concatenate(heads, axis=0)
    res2_o[...] = jnp.dot(h1, Wr2[...], preferred_element_type=F32) + br2[...]


def _tc2(acc1, hw1p, degp, p):
    return pl.pallas_call(
        _tc2_body,
        grid=(_GRID,),
        in_specs=[_rows(256), _rows(256), _rows(1), _full(256), _full(256),
                  _full(256), _full(256, 512), _full(4, 128), _full(4, 128),
                  _full(256, 128), _full(128)],
        out_specs=[_rows(512), pl.BlockSpec((4, RB, 2), lambda i: (0, i, 0)),
                   _rows(128)],
        out_shape=[jax.ShapeDtypeStruct((NN, 512), F32),
                   jax.ShapeDtypeStruct((4, NN, 2), F32),
                   jax.ShapeDtypeStruct((NN, 128), F32)],
    )(acc1, hw1p, degp, p["gcn1_b"], p["ln1_g"], p["ln1_b"], p["gat2_W"],
      p["gat2_att_src"], p["gat2_att_dst"], p["res2_W"], p["res2_b"])


def _tc3_body(num, den, hw2, aa, res2, bg, g2, be2, Wc, bc,
              h2_o, part_o):
    aam = aa[...]
    hw2m = hw2[...]
    gat = jnp.zeros_like(res2[...])
    for h in range(4):
        esl = aam[h, :, 0:1] + aam[h, :, 1:2]
        wsl = jnp.exp(jnp.maximum(esl, 0.2 * esl))
        nh = num[...][h] + wsl * hw2m[:, h * 128:(h + 1) * 128]
        sh = den[...][h][:, 0:1] + wsl
        gat = gat + nh / (sh + 1e-16)
    gat = gat * 0.25 + bg[...]
    h2 = jax.nn.relu(_ln(gat + res2[...], g2[...], be2[...]))
    h2_o[...] = h2
    part_o[...] = jnp.dot(h2, Wc[...], preferred_element_type=F32) + bc[...]


def _tc3(num, den, hw2, aa, res2, p):
    return pl.pallas_call(
        _tc3_body,
        grid=(_GRID,),
        in_specs=[pl.BlockSpec((4, RB, 128), lambda i: (0, i, 0)),
                  pl.BlockSpec((4, RB, 16), lambda i: (0, i, 0)),
                  _rows(512), pl.BlockSpec((4, RB, 2), lambda i: (0, i, 0)),
                  _rows(128), _full(128), _full(128),
                  _full(128), _full(128, 64), _full(64)],
        out_specs=[_rows(128), _rows(64)],
        out_shape=[jax.ShapeDtypeStruct((NN, 128), F32),
                   jax.ShapeDtypeStruct((NN, 64), F32)],
    )(num, den, hw2, aa, res2, p["gat2_b"], p["ln2_g"], p["ln2_b"],
      p["sage3_Wr"] + p["res3_W"],
      p["res3_b"] + p["sage3_bl"])


def _tc4_body(accS, degp, part, Wl, g3, be3, W4, h4p_o):
    deg = degp[...]
    mean = accS[...] / jnp.maximum(deg, 1.0)
    h3 = jax.nn.relu(_ln(jnp.dot(mean, Wl[...], preferred_element_type=F32)
                         + part[...], g3[...], be3[...]))
    h4p = jnp.dot(h3, W4[...], preferred_element_type=F32) * lax.rsqrt(deg + 1.0)
    h4p_o[...] = jnp.concatenate([h4p, jnp.zeros((RB, 64), F32)], axis=1)


def _tc4(accS, degp, part, p):
    return pl.pallas_call(
        _tc4_body,
        grid=(_GRID,),
        in_specs=[_rows(128), _rows(1), _rows(64), _full(128, 64), _full(64),
                  _full(64), _full(64, 64)],
        out_specs=_rows(128),
        out_shape=jax.ShapeDtypeStruct((NN, 128), F32),
    )(accS, degp, part, p["sage3_Wl"], p["ln3_g"], p["ln3_b"], p["gc4_W"])


def _tc5_body(acc4, h4p, degp, batch3, b4, g4, be4, Wro, bro, pool_o):
    i = pl.program_id(0)
    dinv = lax.rsqrt(degp[...] + 1.0)
    h4 = jax.nn.relu(_ln(dinv * (acc4[...] + h4p[...]) + b4[...],
                         g4[...], be4[...]))
    gate = jax.nn.sigmoid(jnp.dot(h4, Wro[...], preferred_element_type=F32) + bro[...])
    gated = h4 * gate
    b = batch3[...][0, 0, :]
    P = (lax.broadcasted_iota(I32, (64, RB), 0) == b[None, :]).astype(F32)
    rhs = jnp.concatenate([gated, gate, jnp.zeros((RB, 63), F32)], axis=1)
    blk = jnp.dot(P, rhs, preferred_element_type=F32)

    @pl.when(i == 0)
    def _():
        pool_o[...] = jnp.zeros_like(pool_o)

    pool_o[...] += blk


def _tc5(acc4, h4p, degp, batch3, p):
    return pl.pallas_call(
        _tc5_body,
        grid=(_GRID,),
        in_specs=[_rows(64), _rows(64), _rows(1),
                  pl.BlockSpec((1, 1, RB), lambda i: (i, 0, 0)),
                  _full(64), _full(64), _full(64), _full(64, 1), _full(1)],
        out_specs=_full(64, 128),
        out_shape=jax.ShapeDtypeStruct((64, 128), F32),
    )(acc4, h4p, degp, batch3, p["gc4_b"], p["ln4_g"], p["ln4_b"],
      p["ro_W"], p["ro_b"])


def _tc6_body(pool, link3, W1, b1, W2, b2, W3, b3, out_o):
    pm = pool[...]
    emb = pm[:, :64] / (pm[:, 64:65] + 1e-8)
    lk = link3[...][:, 0, :]
    g_iota = lax.broadcasted_iota(I32, (4096, 64), 1)
    o1 = (lk[0][:, None] == g_iota).astype(F32)
    o2 = (lk[1][:, None] == g_iota).astype(F32)
    f = jnp.concatenate([
        jnp.dot(o1, emb, preferred_element_type=F32),
        jnp.dot(o2, emb, preferred_element_type=F32)], axis=1)
    f = jax.nn.relu(jnp.dot(f, W1[...], preferred_element_type=F32) + b1[...])
    f = jax.nn.relu(jnp.dot(f, W2[...], preferred_element_type=F32) + b2[...])
    out_o[...] = jax.nn.sigmoid(jnp.dot(f, W3[...], preferred_element_type=F32) + b3[...])


def _tc6(pool, link3, p):
    return pl.pallas_call(
        _tc6_body,
        in_specs=[pl.BlockSpec((64, 128), lambda: (0, 0)),
                  pl.BlockSpec((2, 1, 4096), lambda: (0, 0, 0)),
                  pl.BlockSpec((128, 64), lambda: (0, 0)),
                  pl.BlockSpec((64,), lambda: (0,)),
                  pl.BlockSpec((64, 64), lambda: (0, 0)),
                  pl.BlockSpec((64,), lambda: (0,)),
                  pl.BlockSpec((64, 1), lambda: (0, 0)),
                  pl.BlockSpec((1,), lambda: (0,)),
                  ],
        out_specs=pl.BlockSpec((4096, 1), lambda: (0, 0)),
        out_shape=jax.ShapeDtypeStruct((4096, 1), F32),
    )(pool, link3, p["mlp1_W"], p["mlp1_b"], p["mlp2_W"], p["mlp2_b"],
      p["mlp3_W"], p["mlp3_b"])


def kernel(x, edge_index, batch, link_indices, params):
    p = params

    clist, counts, deg = _sc_scan(edge_index)
    degp = deg[:NN, None]
    hw1p = _tc1(x, p["gcn1_W"], degp)
    acc1 = _sc_rowsum256(hw1p, clist, counts)[:NN]
    hw2, aa, res2 = _tc2(acc1, hw1p, degp, p)
    num, den = _sc_gat(hw2.reshape(NN * 4, 128),
                       aa.reshape(4, 1, 2 * NN), clist, counts)
    h2, part = _tc3(num[:, :NN], den[:, :NN], hw2, aa, res2, p)
    accS = _sc_rowsum128(h2, clist, counts)[:NN]
    h4p = _tc4(accS, degp, part, p)
    acc4 = _sc_rowsum128(h4p, clist, counts)[:NN, :64]
    pool = _tc5(acc4, h4p[:, :64], degp, batch.reshape(_GRID, 1, RB), p)
    out = _tc6(pool, link_indices.reshape(2, 1, 4096), p)
    return out[:, 0]


# speedup vs baseline: 1.2191x; 1.0004x over previous
import functools
import jax
import jax.numpy as jnp
from jax import lax
from jax.experimental import pallas as pl
from jax.experimental.pallas import tpu as pltpu
from jax.experimental.pallas import tpu_sc as plsc

F32 = jnp.float32
I32 = jnp.int32

NN = 10000
EE = 160000
NT = 32
RNG = 320
NROW = 328
CH = 128
SCAN_B = 2560
NSCAN = EE // SCAN_B
CLCAP = 10368
SPILL = 10240
CLW = EE + 128

_MESH = dict(core_axis_name="c", subcore_axis_name="s",
             num_cores=2, num_subcores=16)
_PARAMS = pltpu.CompilerParams(needs_layout_passes=False)


def _coli():
    io = lax.broadcasted_iota(I32, (16,), 0)
    return [io + c * 16 for c in range(16)]


class _ColI:
    def __getitem__(self, c):
        io = lax.broadcasted_iota(I32, (16,), 0)
        return io + c * 16


_COLI = _ColI()


def _worker(c, s):
    return c * 16 + s


@functools.partial(
    pl.kernel,
    out_type=(
        jax.ShapeDtypeStruct((NT, 1, CLW), I32),
        jax.ShapeDtypeStruct((NT, 1, 16), I32),
        jax.ShapeDtypeStruct((NT * RNG,), F32),
    ),
    mesh=plsc.VectorSubcoreMesh(**_MESH),
    compiler_params=_PARAMS,
    scratch_types=[
        pltpu.VMEM((2, SCAN_B), I32),
        pltpu.VMEM((CLCAP,), I32),
        pltpu.VMEM((NROW,), F32),
        pltpu.VMEM((16,), I32),
    ],
)
def _sc_scan(ei, clist, counts, deg, ebuf, clbuf, dacc, cstg):
    c = lax.axis_index("c")
    s = lax.axis_index("s")
    t = _worker(c, s)
    base = t * RNG
    iota = lax.broadcasted_iota(I32, (16,), 0)
    ones16 = jnp.ones((16,), F32)
    zero16 = jnp.zeros((16,), F32)

    def dz(r, _):
        dacc[pl.ds(r * 16, 16)] = zero16
        return 0

    lax.fori_loop(0, NROW // 16, dz, 0)

    def chunk(g, carry):
        pltpu.sync_copy(ei.at[:, pl.ds(g * SCAN_B, SCAN_B)], ebuf)

        def group(j, carry):
            off, hoff = carry
            s16 = ebuf[0, pl.ds(j * 16, 16)]
            d16 = ebuf[1, pl.ds(j * 16, 16)]
            ok = (d16 >= base) & (d16 < base + RNG)
            key = jnp.where(ok, iota, 16)
            _, sv = plsc.sort_key_val(key, s16 * 16384 + d16)
            clbuf[pl.ds(off, 16)] = sv
            cntv = plsc.all_reduce_population_count(ok)
            off = off + cntv[0]
            loc = jnp.clip(d16 - base, 0, RNG)
            plsc.addupdate_scatter(dacc, [loc], jnp.where(ok, ones16, zero16))

            @pl.when(off >= SPILL)
            def _():
                pltpu.sync_copy(
                    clbuf.at[pl.ds(0, SPILL)],
                    clist.at[t, 0, pl.ds(pl.multiple_of(hoff, 128), SPILL)])
                tail = clbuf[pl.ds(SPILL, 16)]
                clbuf[pl.ds(0, 16)] = tail

            spilled = off >= SPILL
            off = jnp.where(spilled, off - SPILL, off)
            hoff = jnp.where(spilled, hoff + SPILL, hoff)
            return (off, hoff)

        return lax.fori_loop(0, SCAN_B // 16, group, carry)

    off, hoff = lax.fori_loop(0, NSCAN, chunk, (0, 0))

    padv = jnp.zeros((16,), I32) + (base + RNG)
    for i in range(8):
        clbuf[pl.ds(off + i * 16, 16)] = padv
    nsp = (off + 127) // 128

    def spill_fin(g, _):
        o = pl.multiple_of(g * 128, 128)
        pltpu.sync_copy(
            clbuf.at[pl.ds(o, 128)],
            clist.at[t, 0, pl.ds(pl.multiple_of(hoff, 128) + o, 128)])
        return 0

    lax.fori_loop(0, nsp, spill_fin, 0)
    cstg[pl.ds(0, 16)] = jnp.zeros((16,), I32) + (hoff + nsp * 128)
    pltpu.sync_copy(cstg, counts.at[t, 0])
    pltpu.sync_copy(dacc.at[pl.ds(0, RNG)], deg.at[pl.ds(base, RNG)])


def _read_count(counts, cntb, t):
    pltpu.sync_copy(counts.at[t, 0], cntb)
    return cntb[pl.ds(0, 16)][0]


def _unpack_chunk(gibuf, dlbuf, base):
    for j in range(CH // 16):
        v16 = gibuf[pl.ds(j * 16, 16)]
        d16 = v16 & 16383
        dlbuf[pl.ds(j * 16, 16)] = jnp.clip(d16 - base, 0, RNG)
        gibuf[pl.ds(j * 16, 16)] = lax.shift_right_logical(v16, 14)


def _drain(acc, out_slice, base):
    for k in range(RNG // 64):
        pltpu.sync_copy(acc.at[pl.ds(k * 64, 64)],
                        out_slice.at[pl.ds(base + k * 64, 64)])


def _zero_acc(acc, w):
    z = jnp.zeros((16,), F32)

    def za(r, _):
        for j in range(w // 16):
            acc[r, pl.ds(j * 16, 16)] = z
        return 0

    lax.fori_loop(0, NROW, za, 0)


NQ = 4
QB = CH // NQ


def _make_rowsum(w):
    @functools.partial(
        pl.kernel,
        out_type=jax.ShapeDtypeStruct((NT * RNG, w), F32),
        mesh=plsc.VectorSubcoreMesh(**_MESH),
        compiler_params=_PARAMS,
        scratch_types=[
            pltpu.VMEM((NROW, w), F32),
            pltpu.VMEM((CH, w), F32),
            pltpu.VMEM((CH,), I32),
            pltpu.VMEM((CH,), I32),
            pltpu.VMEM((16,), I32),
        ] + [pltpu.SemaphoreType.DMA] * NQ,
    )
    def rowsum(table, clist, counts, out, acc, rowbuf, gibuf, dlbuf, cntb,
               *sems):
        c = lax.axis_index("c")
        s = lax.axis_index("s")
        t = _worker(c, s)
        base = t * RNG
        _zero_acc(acc, w)
        nch = _read_count(counts, cntb, t) // CH

        def chunk(g, _):
            pltpu.sync_copy(clist.at[t, 0, pl.ds(g * CH, CH)], gibuf)
            _unpack_chunk(gibuf, dlbuf, base)
            pltpu.async_copy(table.at[gibuf], rowbuf, sems[0]).wait()

            def rmw(j, _):
                dv = dlbuf[pl.ds(j * 16, 16)]
                for l in range(16):
                    rows = lax.broadcast(dv[l], (16,))
                    r = j * 16 + l
                    for col in range(w // 16):
                        plsc.addupdate_scatter(
                            acc, [rows, _COLI[col]],
                            rowbuf[r, pl.ds(col * 16, 16)])
                return 0

            lax.fori_loop(0, CH // 16, rmw, 0)
            return 0

        lax.fori_loop(0, nch, chunk, 0)
        _drain(acc, out, base)

    return rowsum


_sc_rowsum256 = _make_rowsum(256)
_sc_rowsum128 = _make_rowsum(128)


@functools.partial(
    pl.kernel,
    out_type=(
        jax.ShapeDtypeStruct((4, NT * RNG, 128), F32),
        jax.ShapeDtypeStruct((4, NT * RNG, 16), F32),
    ),
    mesh=plsc.VectorSubcoreMesh(**_MESH),
    compiler_params=_PARAMS,
    scratch_types=[
        pltpu.VMEM((NROW, 128), F32),
        pltpu.VMEM((NROW, 16), F32),
        pltpu.VMEM((2 * NN,), F32),
        pltpu.VMEM((CH, 128), F32),
        pltpu.VMEM((CH,), F32),
        pltpu.VMEM((CH,), I32),
        pltpu.VMEM((CH,), I32),
        pltpu.VMEM((16,), I32),
    ] + [pltpu.SemaphoreType.DMA] * NQ,
)
def _sc_gat(hw2, aa4, clist, counts, num_out, den_out,
            nacc, dacc, aav, rowbuf, wrow, gibuf, dlbuf, cntb, *sems):
    c = lax.axis_index("c")
    s = lax.axis_index("s")
    t = _worker(c, s)
    base = t * RNG
    lane0 = jnp.where(lax.broadcasted_iota(I32, (16,), 0) == 0, 1.0, 0.0)
    nch = _read_count(counts, cntb, t) // CH

    def head(h, _):
        pltpu.sync_copy(aa4.at[h, 0], aav)
        _zero_acc(nacc, 128)
        _zero_acc(dacc, 16)

        def chunk(g, _):
            pltpu.sync_copy(clist.at[t, 0, pl.ds(g * CH, CH)], gibuf)
            for j in range(CH // 16):
                v16 = gibuf[pl.ds(j * 16, 16)]
                d16 = v16 & 16383
                s16 = lax.shift_right_logical(v16, 14)
                a_s = plsc.load_gather(aav, [s16 * 2])
                a_d = plsc.load_gather(aav, [jnp.minimum(d16, NN - 1) * 2 + 1])
                e = a_s + a_d
                e = jnp.maximum(e, 0.2 * e)
                wrow[pl.ds(j * 16, 16)] = jnp.exp(e)
                dlbuf[pl.ds(j * 16, 16)] = jnp.clip(d16 - base, 0, RNG)
                gibuf[pl.ds(j * 16, 16)] = s16 * 4 + h
            pltpu.async_copy(hw2.at[gibuf], rowbuf, sems[0]).wait()

            def rmw(j, _):
                dv = dlbuf[pl.ds(j * 16, 16)]
                wv = wrow[pl.ds(j * 16, 16)]
                for l in range(16):
                    rows = lax.broadcast(dv[l], (16,))
                    r = j * 16 + l
                    wl = wv[l]
                    for col in range(8):
                        plsc.addupdate_scatter(
                            nacc, [rows, _COLI[col]],
                            rowbuf[r, pl.ds(col * 16, 16)] * wl)
                    plsc.addupdate_scatter(dacc, [rows, _COLI[0]],
                                           lane0 * wl)
                return 0

            lax.fori_loop(0, CH // 16, rmw, 0)
            return 0

        lax.fori_loop(0, nch, chunk, 0)
        _drain(nacc, num_out.at[h], base)
        _drain(dacc, den_out.at[h], base)
        return 0

    lax.fori_loop(0, 4, head, 0)


RB = 1000
_GRID = NN // RB


def _ln(h, g, b):
    mu = jnp.mean(h, axis=-1, keepdims=True)
    var = jnp.mean((h - mu) ** 2, axis=-1, keepdims=True)
    return (h - mu) * lax.rsqrt(var + 1e-5) * g + b


def _rows(c):
    return pl.BlockSpec((RB, c), lambda i: (i, 0))


def _full(*shape):
    nd = len(shape)
    return pl.BlockSpec(shape, lambda i: (0,) * nd)


def _tc1_body(x, W, degp, out):
    dinv = lax.rsqrt(degp[...] + 1.0)
    out[...] = jnp.dot(x[...], W[...], preferred_element_type=F32) * dinv


def _tc1(x, W, degp):
    return pl.pallas_call(
        _tc1_body,
        grid=(_GRID,),
        in_specs=[_rows(256), _full(256, 256), _rows(1)],
        out_specs=_rows(256),
        out_shape=jax.ShapeDtypeStruct((NN, 256), F32),
    )(x, W, degp)


def _tc2_body(acc1, hw1p, degp, b1, g1, be1, Wg, asr, ads, Wr2, br2,
              hw2_o, aa_o, res2_o):
    dinv = lax.rsqrt(degp[...] + 1.0)
    h1 = jax.nn.relu(_ln(dinv * (acc1[...] + hw1p[...]) + b1[...],
                         g1[...], be1[...]))
    hw2 = jnp.dot(h1, Wg[...], preferred_element_type=F32)
    hw2_o[...] = hw2
    heads = []
    for h in range(4):
        blk = hw2[:, h * 128:(h + 1) * 128]
        a_s = jnp.sum(blk * asr[...][h][None, :], axis=1, keepdims=True)
        a_d = jnp.sum(blk * ads[...][h][None, :], axis=1, keepdims=True)
        heads.append(jnp.concatenate([a_s, a_d], axis=1)[None])
    aa_o[...] = jnp.concatenate(heads, axis=0)
    res2_o[...] = jnp.dot(h1, Wr2[...], preferred_element_type=F32) + br2[...]


def _tc2(acc1, hw1p, degp, p):
    return pl.pallas_call(
        _tc2_body,
        grid=(_GRID,),
        in_specs=[_rows(256), _rows(256), _rows(1), _full(256), _full(256),
                  _full(256), _full(256, 512), _full(4, 128), _full(4, 128),
                  _full(256, 128), _full(128)],
        out_specs=[_rows(512), pl.BlockSpec((4, RB, 2), lambda i: (0, i, 0)),
                   _rows(128)],
        out_shape=[jax.ShapeDtypeStruct((NN, 512), F32),
                   jax.ShapeDtypeStruct((4, NN, 2), F32),
                   jax.ShapeDtypeStruct((NN, 128), F32)],
    )(acc1, hw1p, degp, p["gcn1_b"], p["ln1_g"], p["ln1_b"], p["gat2_W"],
      p["gat2_att_src"], p["gat2_att_dst"], p["res2_W"], p["res2_b"])


def _tc3_body(num, den, hw2, aa, res2, bg, g2, be2, Wc, bc,
              h2_o, part_o):
    aam = aa[...]
    hw2m = hw2[...]
    gat = jnp.zeros_like(res2[...])
    for h in range(4):
        esl = aam[h, :, 0:1] + aam[h, :, 1:2]
        wsl = jnp.exp(jnp.maximum(esl, 0.2 * esl))
        nh = num[...][h] + wsl * hw2m[:, h * 128:(h + 1) * 128]
        sh = den[...][h][:, 0:1] + wsl
        gat = gat + nh / (sh + 1e-16)
    gat = gat * 0.25 + bg[...]
    h2 = jax.nn.relu(_ln(gat + res2[...], g2[...], be2[...]))
    h2_o[...] = h2
    part_o[...] = jnp.dot(h2, Wc[...], preferred_element_type=F32) + bc[...]


def _tc3(num, den, hw2, aa, res2, p):
    return pl.pallas_call(
        _tc3_body,
        grid=(_GRID,),
        in_specs=[pl.BlockSpec((4, RB, 128), lambda i: (0, i, 0)),
                  pl.BlockSpec((4, RB, 16), lambda i: (0, i, 0)),
                  _rows(512), pl.BlockSpec((4, RB, 2), lambda i: (0, i, 0)),
                  _rows(128), _full(128), _full(128),
                  _full(128), _full(128, 64), _full(64)],
        out_specs=[_rows(128), _rows(64)],
        out_shape=[jax.ShapeDtypeStruct((NN, 128), F32),
                   jax.ShapeDtypeStruct((NN, 64), F32)],
    )(num, den, hw2, aa, res2, p["gat2_b"], p["ln2_g"], p["ln2_b"],
      p["sage3_Wr"] + p["res3_W"],
      p["res3_b"] + p["sage3_bl"])


def _tc4_body(accS, degp, part, Wl, g3, be3, W4, h4p_o):
    deg = degp[...]
    mean = accS[...] / jnp.maximum(deg, 1.0)
    h3 = jax.nn.relu(_ln(jnp.dot(mean, Wl[...], preferred_element_type=F32)
                         + part[...], g3[...], be3[...]))
    h4p = jnp.dot(h3, W4[...], preferred_element_type=F32) * lax.rsqrt(deg + 1.0)
    h4p_o[...] = jnp.concatenate([h4p, jnp.zeros((RB, 64), F32)], axis=1)


def _tc4(accS, degp, part, p):
    return pl.pallas_call(
        _tc4_body,
        grid=(_GRID,),
        in_specs=[_rows(128), _rows(1), _rows(64), _full(128, 64), _full(64),
                  _full(64), _full(64, 64)],
        out_specs=_rows(128),
        out_shape=jax.ShapeDtypeStruct((NN, 128), F32),
    )(accS, degp, part, p["sage3_Wl"], p["ln3_g"], p["ln3_b"], p["gc4_W"])


def _tc5_body(acc4, h4p, degp, batch3, b4, g4, be4, Wro, bro, pool_o):
    i = pl.program_id(0)
    dinv = lax.rsqrt(degp[...] + 1.0)
    h4 = jax.nn.relu(_ln(dinv * (acc4[...] + h4p[...]) + b4[...],
                         g4[...], be4[...]))
    gate = jax.nn.sigmoid(jnp.dot(h4, Wro[...], preferred_element_type=F32) + bro[...])
    gated = h4 * gate
    b = batch3[...][0, 0, :]
    P = (lax.broadcasted_iota(I32, (64, RB), 0) == b[None, :]).astype(F32)
    rhs = jnp.concatenate([gated, gate, jnp.zeros((RB, 63), F32)], axis=1)
    blk = jnp.dot(P, rhs, preferred_element_type=F32)

    @pl.when(i == 0)
    def _():
        pool_o[...] = jnp.zeros_like(pool_o)

    pool_o[...] += blk


def _tc5(acc4, h4p, degp, batch3, p):
    return pl.pallas_call(
        _tc5_body,
        grid=(_GRID,),
        in_specs=[_rows(64), _rows(64), _rows(1),
                  pl.BlockSpec((1, 1, RB), lambda i: (i, 0, 0)),
                  _full(64), _full(64), _full(64), _full(64, 1), _full(1)],
        out_specs=_full(64, 128),
        out_shape=jax.ShapeDtypeStruct((64, 128), F32),
    )(acc4, h4p, degp, batch3, p["gc4_b"], p["ln4_g"], p["ln4_b"],
      p["ro_W"], p["ro_b"])


def _tc6_body(pool, link3, W1, b1, W2, b2, W3, b3, out_o):
    pm = pool[...]
    emb = pm[:, :64] / (pm[:, 64:65] + 1e-8)
    lk = link3[...][:, 0, :]
    g_iota = lax.broadcasted_iota(I32, (4096, 64), 1)
    o1 = (lk[0][:, None] == g_iota).astype(F32)
    o2 = (lk[1][:, None] == g_iota).astype(F32)
    f = jnp.concatenate([
        jnp.dot(o1, emb, preferred_element_type=F32),
        jnp.dot(o2, emb, preferred_element_type=F32)], axis=1)
    f = jax.nn.relu(jnp.dot(f, W1[...], preferred_element_type=F32) + b1[...])
    f = jax.nn.relu(jnp.dot(f, W2[...], preferred_element_type=F32) + b2[...])
    out_o[...] = jax.nn.sigmoid(jnp.dot(f, W3[...], preferred_element_type=F32) + b3[...])


def _tc6(pool, link3, p):
    return pl.pallas_call(
        _tc6_body,
        in_specs=[pl.BlockSpec((64, 128), lambda: (0, 0)),
                  pl.BlockSpec((2, 1, 4096), lambda: (0, 0, 0)),
                  pl.BlockSpec((128, 64), lambda: (0, 0)),
                  pl.BlockSpec((64,), lambda: (0,)),
                  pl.BlockSpec((64, 64), lambda: (0, 0)),
                  pl.BlockSpec((64,), lambda: (0,)),
                  pl.BlockSpec((64, 1), lambda: (0, 0)),
                  pl.BlockSpec((1,), lambda: (0,)),
                  ],
        out_specs=pl.BlockSpec((4096, 1), lambda: (0, 0)),
        out_shape=jax.ShapeDtypeStruct((4096, 1), F32),
    )(pool, link3, p["mlp1_W"], p["mlp1_b"], p["mlp2_W"], p["mlp2_b"],
      p["mlp3_W"], p["mlp3_b"])


def kernel(x, edge_index, batch, link_indices, params):
    p = params

    clist, counts, deg = _sc_scan(edge_index)
    degp = deg[:NN, None]
    hw1p = _tc1(x, p["gcn1_W"], degp)
    acc1 = _sc_rowsum256(hw1p, clist, counts)[:NN]
    hw2, aa, res2 = _tc2(acc1, hw1p, degp, p)
    num, den = _sc_gat(hw2.reshape(NN * 4, 128),
                       aa.reshape(4, 1, 2 * NN), clist, counts)
    h2, part = _tc3(num[:, :NN], den[:, :NN], hw2, aa, res2, p)
    accS = _sc_rowsum128(h2, clist, counts)[:NN]
    h4p = _tc4(accS, degp, part, p)
    acc4 = _sc_rowsum128(h4p, clist, counts)[:NN, :64]
    pool = _tc5(acc4, h4p[:, :64], degp, batch.reshape(_GRID, 1, RB), p)
    out = _tc6(pool, link_indices.reshape(2, 1, 4096), p)
    return out[:, 0]


# speedup vs baseline: 1.3023x; 1.0682x over previous
import functools
import jax
import jax.numpy as jnp
from jax import lax
from jax.experimental import pallas as pl
from jax.experimental.pallas import tpu as pltpu
from jax.experimental.pallas import tpu_sc as plsc

F32 = jnp.float32
I32 = jnp.int32

NN = 10000
EE = 160000
NT = 32
RNG = 320
NROW = 328
CH = 128
SCAN_B = 2560
NSCAN = EE // SCAN_B
CLCAP = 10368
SPILL = 10240
CLW = EE + 128

_MESH = dict(core_axis_name="c", subcore_axis_name="s",
             num_cores=2, num_subcores=16)
_PARAMS = pltpu.CompilerParams(needs_layout_passes=False)


def _coli():
    io = lax.broadcasted_iota(I32, (16,), 0)
    return [io + c * 16 for c in range(16)]


class _ColI:
    def __getitem__(self, c):
        io = lax.broadcasted_iota(I32, (16,), 0)
        return io + c * 16


_COLI = _ColI()


def _worker(c, s):
    return c * 16 + s


@functools.partial(
    pl.kernel,
    out_type=(
        jax.ShapeDtypeStruct((NT, 1, CLW), I32),
        jax.ShapeDtypeStruct((NT, 1, 16), I32),
        jax.ShapeDtypeStruct((NT * RNG,), F32),
    ),
    mesh=plsc.VectorSubcoreMesh(**_MESH),
    compiler_params=_PARAMS,
    scratch_types=[
        pltpu.VMEM((2, SCAN_B), I32),
        pltpu.VMEM((CLCAP,), I32),
        pltpu.VMEM((NROW,), F32),
        pltpu.VMEM((16,), I32),
    ],
)
def _sc_scan(ei, clist, counts, deg, ebuf, clbuf, dacc, cstg):
    c = lax.axis_index("c")
    s = lax.axis_index("s")
    t = _worker(c, s)
    base = t * RNG
    iota = lax.broadcasted_iota(I32, (16,), 0)
    ones16 = jnp.ones((16,), F32)
    zero16 = jnp.zeros((16,), F32)

    def dz(r, _):
        dacc[pl.ds(r * 16, 16)] = zero16
        return 0

    lax.fori_loop(0, NROW // 16, dz, 0)

    def chunk(g, carry):
        pltpu.sync_copy(ei.at[:, pl.ds(g * SCAN_B, SCAN_B)], ebuf)

        def group(j, carry):
            off, hoff = carry
            s16 = ebuf[0, pl.ds(j * 16, 16)]
            d16 = ebuf[1, pl.ds(j * 16, 16)]
            ok = (d16 >= base) & (d16 < base + RNG)
            key = jnp.where(ok, iota, 16)
            _, sv = plsc.sort_key_val(key, s16 * 16384 + d16)
            clbuf[pl.ds(off, 16)] = sv
            cntv = plsc.all_reduce_population_count(ok)
            off = off + cntv[0]
            loc = jnp.clip(d16 - base, 0, RNG)
            plsc.addupdate_scatter(dacc, [loc], jnp.where(ok, ones16, zero16))

            @pl.when(off >= SPILL)
            def _():
                pltpu.sync_copy(
                    clbuf.at[pl.ds(0, SPILL)],
                    clist.at[t, 0, pl.ds(pl.multiple_of(hoff, 128), SPILL)])
                tail = clbuf[pl.ds(SPILL, 16)]
                clbuf[pl.ds(0, 16)] = tail

            spilled = off >= SPILL
            off = jnp.where(spilled, off - SPILL, off)
            hoff = jnp.where(spilled, hoff + SPILL, hoff)
            return (off, hoff)

        return lax.fori_loop(0, SCAN_B // 16, group, carry)

    off, hoff = lax.fori_loop(0, NSCAN, chunk, (0, 0))

    padv = jnp.zeros((16,), I32) + (base + RNG)
    for i in range(8):
        clbuf[pl.ds(off + i * 16, 16)] = padv
    nsp = (off + 127) // 128

    def spill_fin(g, _):
        o = pl.multiple_of(g * 128, 128)
        pltpu.sync_copy(
            clbuf.at[pl.ds(o, 128)],
            clist.at[t, 0, pl.ds(pl.multiple_of(hoff, 128) + o, 128)])
        return 0

    lax.fori_loop(0, nsp, spill_fin, 0)
    cstg[pl.ds(0, 16)] = jnp.zeros((16,), I32) + (hoff + nsp * 128)
    pltpu.sync_copy(cstg, counts.at[t, 0])
    pltpu.sync_copy(dacc.at[pl.ds(0, RNG)], deg.at[pl.ds(base, RNG)])


def _read_count(counts, cntb, t):
    pltpu.sync_copy(counts.at[t, 0], cntb)
    return cntb[pl.ds(0, 16)][0]


def _unpack_chunk(gibuf, dlbuf, base):
    for j in range(CH // 16):
        v16 = gibuf[pl.ds(j * 16, 16)]
        d16 = v16 & 16383
        dlbuf[pl.ds(j * 16, 16)] = jnp.clip(d16 - base, 0, RNG)
        gibuf[pl.ds(j * 16, 16)] = lax.shift_right_logical(v16, 14)


def _drain(acc, out_slice, base):
    for k in range(RNG // 64):
        pltpu.sync_copy(acc.at[pl.ds(k * 64, 64)],
                        out_slice.at[pl.ds(base + k * 64, 64)])


def _zero_acc(acc, w):
    z = jnp.zeros((16,), F32)

    def za(r, _):
        for j in range(w // 16):
            acc[r, pl.ds(j * 16, 16)] = z
        return 0

    lax.fori_loop(0, NROW, za, 0)


NQ = 4
QB = CH // NQ


def _make_rowsum(w):
    @functools.partial(
        pl.kernel,
        out_type=jax.ShapeDtypeStruct((NT * RNG, w), F32),
        mesh=plsc.VectorSubcoreMesh(**_MESH),
        compiler_params=_PARAMS,
        scratch_types=[
            pltpu.VMEM((NROW, w), F32),
            pltpu.VMEM((CH, w), F32),
            pltpu.VMEM((CH,), I32),
            pltpu.VMEM((CH,), I32),
            pltpu.VMEM((16,), I32),
        ] + [pltpu.SemaphoreType.DMA] * NQ,
    )
    def rowsum(table, clist, counts, out, acc, rowbuf, gibuf, dlbuf, cntb,
               *sems):
        c = lax.axis_index("c")
        s = lax.axis_index("s")
        t = _worker(c, s)
        base = t * RNG
        _zero_acc(acc, w)
        nch = _read_count(counts, cntb, t) // CH

        def chunk(g, _):
            pltpu.sync_copy(clist.at[t, 0, pl.ds(g * CH, CH)], gibuf)
            _unpack_chunk(gibuf, dlbuf, base)
            pltpu.async_copy(table.at[gibuf], rowbuf, sems[0]).wait()

            def rmw(j, _):
                dv = dlbuf[pl.ds(j * 16, 16)]
                for l in range(16):
                    rows = lax.broadcast(dv[l], (16,))
                    r = j * 16 + l
                    for col in range(w // 16):
                        plsc.addupdate_scatter(
                            acc, [rows, _COLI[col]],
                            rowbuf[r, pl.ds(col * 16, 16)])
                return 0

            lax.fori_loop(0, CH // 16, rmw, 0)
            return 0

        lax.fori_loop(0, nch, chunk, 0)
        _drain(acc, out, base)

    return rowsum


_sc_rowsum256 = _make_rowsum(256)
_sc_rowsum128 = _make_rowsum(128)


@functools.partial(
    pl.kernel,
    out_type=(
        jax.ShapeDtypeStruct((4, NT * RNG, 128), F32),
        jax.ShapeDtypeStruct((4, NT * RNG * 16), F32),
    ),
    mesh=plsc.VectorSubcoreMesh(**_MESH),
    compiler_params=_PARAMS,
    scratch_types=[
        pltpu.VMEM((NROW, 128), F32),
        pltpu.VMEM((NROW * 16,), F32),
        pltpu.VMEM((2 * NN,), F32),
        pltpu.VMEM((2, CH, 128), F32),
        pltpu.VMEM((2, CH), F32),
        pltpu.VMEM((2, CH), I32),
        pltpu.VMEM((2, CH), I32),
        pltpu.VMEM((16,), I32),
    ] + [pltpu.SemaphoreType.DMA] * 2,
)
def _sc_gat(hw2, aa4, clist, counts, num_out, den_out,
            nacc, dacc, aav, rowbuf, wrow, gibuf, dlbuf, cntb, *sems):
    c = lax.axis_index("c")
    s = lax.axis_index("s")
    t = _worker(c, s)
    base = t * RNG
    lane0 = jnp.where(lax.broadcasted_iota(I32, (16,), 0) == 0, 1.0, 0.0)
    nch = _read_count(counts, cntb, t) // CH

    def head(h, _):
        pltpu.sync_copy(aa4.at[h, 0], aav)
        _zero_acc(nacc, 128)

        def dz(r, _):
            dacc[pl.ds(r * 16, 16)] = jnp.zeros((16,), F32)
            return 0

        lax.fori_loop(0, NROW, dz, 0)

        def prep_fire(g, b):
            pltpu.sync_copy(clist.at[t, 0, pl.ds(g * CH, CH)], gibuf.at[b])
            for j in range(CH // 16):
                v16 = gibuf[b, pl.ds(j * 16, 16)]
                d16 = v16 & 16383
                s16 = lax.shift_right_logical(v16, 14)
                a_s = plsc.load_gather(aav, [s16 * 2])
                a_d = plsc.load_gather(aav, [jnp.minimum(d16, NN - 1) * 2 + 1])
                e = a_s + a_d
                e = jnp.maximum(e, 0.2 * e)
                wrow[b, pl.ds(j * 16, 16)] = jnp.exp(e)
                dlbuf[b, pl.ds(j * 16, 16)] = jnp.clip(d16 - base, 0, RNG)
                gibuf[b, pl.ds(j * 16, 16)] = s16 * 4 + h
            pltpu.async_copy(hw2.at[gibuf.at[b]], rowbuf.at[b], sems[b])

        def wait_rmw(b):
            pltpu.make_async_copy(hw2.at[gibuf.at[b]], rowbuf.at[b],
                                  sems[b]).wait()

            def rmw(j, _):
                dv = dlbuf[b, pl.ds(j * 16, 16)]
                wv = wrow[b, pl.ds(j * 16, 16)]
                for l in range(16):
                    rows = lax.broadcast(dv[l], (16,))
                    r = j * 16 + l
                    wl = wv[l]
                    for col in range(8):
                        plsc.addupdate_scatter(
                            nacc, [rows, _COLI[col]],
                            rowbuf[b, r, pl.ds(col * 16, 16)] * wl)
                    plsc.addupdate_scatter(dacc, [rows * 16 + _COLI[0]],
                                           lane0 * wl)
                return 0

            lax.fori_loop(0, CH // 16, rmw, 0)

        @pl.when(nch > 0)
        def _():
            prep_fire(0, 0)

        def pairbody(i, _):
            k1 = 2 * i + 1

            @pl.when(k1 < nch)
            def _():
                prep_fire(k1, 1)

            wait_rmw(0)

            @pl.when(k1 < nch)
            def _():
                @pl.when(k1 + 1 < nch)
                def _():
                    prep_fire(k1 + 1, 0)

                wait_rmw(1)

            return 0

        lax.fori_loop(0, (nch + 1) // 2, pairbody, 0)
        _drain(nacc, num_out.at[h], base)
        pltpu.sync_copy(dacc.at[pl.ds(0, RNG * 16)],
                        den_out.at[h, pl.ds(base * 16, RNG * 16)])
        return 0

    lax.fori_loop(0, 4, head, 0)


RB = 1000
_GRID = NN // RB


def _ln(h, g, b):
    mu = jnp.mean(h, axis=-1, keepdims=True)
    var = jnp.mean((h - mu) ** 2, axis=-1, keepdims=True)
    return (h - mu) * lax.rsqrt(var + 1e-5) * g + b


def _rows(c):
    return pl.BlockSpec((RB, c), lambda i: (i, 0))


def _full(*shape):
    nd = len(shape)
    return pl.BlockSpec(shape, lambda i: (0,) * nd)


def _tc1_body(x, W, degp, out):
    dinv = lax.rsqrt(degp[...] + 1.0)
    out[...] = jnp.dot(x[...], W[...], preferred_element_type=F32) * dinv


def _tc1(x, W, degp):
    return pl.pallas_call(
        _tc1_body,
        grid=(_GRID,),
        in_specs=[_rows(256), _full(256, 256), _rows(1)],
        out_specs=_rows(256),
        out_shape=jax.ShapeDtypeStruct((NN, 256), F32),
    )(x, W, degp)


def _tc2_body(acc1, hw1p, degp, b1, g1, be1, Wg, asr, ads, Wr2, br2,
              hw2_o, aa_o, res2_o):
    dinv = lax.rsqrt(degp[...] + 1.0)
    h1 = jax.nn.relu(_ln(dinv * (acc1[...] + hw1p[...]) + b1[...],
                         g1[...], be1[...]))
    hw2 = jnp.dot(h1, Wg[...], preferred_element_type=F32)
    hw2_o[...] = hw2
    heads = []
    for h in range(4):
        blk = hw2[:, h * 128:(h + 1) * 128]
        a_s = jnp.sum(blk * asr[...][h][None, :], axis=1, keepdims=True)
        a_d = jnp.sum(blk * ads[...][h][None, :], axis=1, keepdims=True)
        heads.append(jnp.concatenate([a_s, a_d], axis=1)[None])
    aa_o[...] = jnp.concatenate(heads, axis=0)
    res2_o[...] = jnp.dot(h1, Wr2[...], preferred_element_type=F32) + br2[...]


def _tc2(acc1, hw1p, degp, p):
    return pl.pallas_call(
        _tc2_body,
        grid=(_GRID,),
        in_specs=[_rows(256), _rows(256), _rows(1), _full(256), _full(256),
                  _full(256), _full(256, 512), _full(4, 128), _full(4, 128),
                  _full(256, 128), _full(128)],
        out_specs=[_rows(512), pl.BlockSpec((4, RB, 2), lambda i: (0, i, 0)),
                   _rows(128)],
        out_shape=[jax.ShapeDtypeStruct((NN, 512), F32),
                   jax.ShapeDtypeStruct((4, NN, 2), F32),
                   jax.ShapeDtypeStruct((NN, 128), F32)],
    )(acc1, hw1p, degp, p["gcn1_b"], p["ln1_g"], p["ln1_b"], p["gat2_W"],
      p["gat2_att_src"], p["gat2_att_dst"], p["res2_W"], p["res2_b"])


def _tc3_body(num, den, hw2, aa, res2, bg, g2, be2, Wc, bc,
              h2_o, part_o):
    aam = aa[...]
    hw2m = hw2[...]
    gat = jnp.zeros_like(res2[...])
    for h in range(4):
        esl = aam[h, :, 0:1] + aam[h, :, 1:2]
        wsl = jnp.exp(jnp.maximum(esl, 0.2 * esl))
        nh = num[...][h] + wsl * hw2m[:, h * 128:(h + 1) * 128]
        sh = den[...][h][:, 0:1] + wsl
        gat = gat + nh / (sh + 1e-16)
    gat = gat * 0.25 + bg[...]
    h2 = jax.nn.relu(_ln(gat + res2[...], g2[...], be2[...]))
    h2_o[...] = h2
    part_o[...] = jnp.dot(h2, Wc[...], preferred_element_type=F32) + bc[...]


def _tc3(num, den, hw2, aa, res2, p):
    return pl.pallas_call(
        _tc3_body,
        grid=(_GRID,),
        in_specs=[pl.BlockSpec((4, RB, 128), lambda i: (0, i, 0)),
                  pl.BlockSpec((4, RB, 16), lambda i: (0, i, 0)),
                  _rows(512), pl.BlockSpec((4, RB, 2), lambda i: (0, i, 0)),
                  _rows(128), _full(128), _full(128),
                  _full(128), _full(128, 64), _full(64)],
        out_specs=[_rows(128), _rows(64)],
        out_shape=[jax.ShapeDtypeStruct((NN, 128), F32),
                   jax.ShapeDtypeStruct((NN, 64), F32)],
    )(num, den, hw2, aa, res2, p["gat2_b"], p["ln2_g"], p["ln2_b"],
      p["sage3_Wr"] + p["res3_W"],
      p["res3_b"] + p["sage3_bl"])


def _tc4_body(accS, degp, part, Wl, g3, be3, W4, h4p_o):
    deg = degp[...]
    mean = accS[...] / jnp.maximum(deg, 1.0)
    h3 = jax.nn.relu(_ln(jnp.dot(mean, Wl[...], preferred_element_type=F32)
                         + part[...], g3[...], be3[...]))
    h4p = jnp.dot(h3, W4[...], preferred_element_type=F32) * lax.rsqrt(deg + 1.0)
    h4p_o[...] = jnp.concatenate([h4p, jnp.zeros((RB, 64), F32)], axis=1)


def _tc4(accS, degp, part, p):
    return pl.pallas_call(
        _tc4_body,
        grid=(_GRID,),
        in_specs=[_rows(128), _rows(1), _rows(64), _full(128, 64), _full(64),
                  _full(64), _full(64, 64)],
        out_specs=_rows(128),
        out_shape=jax.ShapeDtypeStruct((NN, 128), F32),
    )(accS, degp, part, p["sage3_Wl"], p["ln3_g"], p["ln3_b"], p["gc4_W"])


def _tc5_body(acc4, h4p, degp, batch3, b4, g4, be4, Wro, bro, pool_o):
    i = pl.program_id(0)
    dinv = lax.rsqrt(degp[...] + 1.0)
    h4 = jax.nn.relu(_ln(dinv * (acc4[...] + h4p[...]) + b4[...],
                         g4[...], be4[...]))
    gate = jax.nn.sigmoid(jnp.dot(h4, Wro[...], preferred_element_type=F32) + bro[...])
    gated = h4 * gate
    b = batch3[...][0, 0, :]
    P = (lax.broadcasted_iota(I32, (64, RB), 0) == b[None, :]).astype(F32)
    rhs = jnp.concatenate([gated, gate, jnp.zeros((RB, 63), F32)], axis=1)
    blk = jnp.dot(P, rhs, preferred_element_type=F32)

    @pl.when(i == 0)
    def _():
        pool_o[...] = jnp.zeros_like(pool_o)

    pool_o[...] += blk


def _tc5(acc4, h4p, degp, batch3, p):
    return pl.pallas_call(
        _tc5_body,
        grid=(_GRID,),
        in_specs=[_rows(64), _rows(64), _rows(1),
                  pl.BlockSpec((1, 1, RB), lambda i: (i, 0, 0)),
                  _full(64), _full(64), _full(64), _full(64, 1), _full(1)],
        out_specs=_full(64, 128),
        out_shape=jax.ShapeDtypeStruct((64, 128), F32),
    )(acc4, h4p, degp, batch3, p["gc4_b"], p["ln4_g"], p["ln4_b"],
      p["ro_W"], p["ro_b"])


def _tc6_body(pool, link3, W1, b1, W2, b2, W3, b3, out_o):
    pm = pool[...]
    emb = pm[:, :64] / (pm[:, 64:65] + 1e-8)
    lk = link3[...][:, 0, :]
    g_iota = lax.broadcasted_iota(I32, (4096, 64), 1)
    o1 = (lk[0][:, None] == g_iota).astype(F32)
    o2 = (lk[1][:, None] == g_iota).astype(F32)
    f = jnp.concatenate([
        jnp.dot(o1, emb, preferred_element_type=F32),
        jnp.dot(o2, emb, preferred_element_type=F32)], axis=1)
    f = jax.nn.relu(jnp.dot(f, W1[...], preferred_element_type=F32) + b1[...])
    f = jax.nn.relu(jnp.dot(f, W2[...], preferred_element_type=F32) + b2[...])
    out_o[...] = jax.nn.sigmoid(jnp.dot(f, W3[...], preferred_element_type=F32) + b3[...])


def _tc6(pool, link3, p):
    return pl.pallas_call(
        _tc6_body,
        in_specs=[pl.BlockSpec((64, 128), lambda: (0, 0)),
                  pl.BlockSpec((2, 1, 4096), lambda: (0, 0, 0)),
                  pl.BlockSpec((128, 64), lambda: (0, 0)),
                  pl.BlockSpec((64,), lambda: (0,)),
                  pl.BlockSpec((64, 64), lambda: (0, 0)),
                  pl.BlockSpec((64,), lambda: (0,)),
                  pl.BlockSpec((64, 1), lambda: (0, 0)),
                  pl.BlockSpec((1,), lambda: (0,)),
                  ],
        out_specs=pl.BlockSpec((4096, 1), lambda: (0, 0)),
        out_shape=jax.ShapeDtypeStruct((4096, 1), F32),
    )(pool, link3, p["mlp1_W"], p["mlp1_b"], p["mlp2_W"], p["mlp2_b"],
      p["mlp3_W"], p["mlp3_b"])


def kernel(x, edge_index, batch, link_indices, params):
    p = params

    clist, counts, deg = _sc_scan(edge_index)
    degp = deg[:NN, None]
    hw1p = _tc1(x, p["gcn1_W"], degp)
    acc1 = _sc_rowsum256(hw1p, clist, counts)[:NN]
    hw2, aa, res2 = _tc2(acc1, hw1p, degp, p)
    num, den = _sc_gat(hw2.reshape(NN * 4, 128),
                       aa.reshape(4, 1, 2 * NN), clist, counts)
    den = den.reshape(4, NT * RNG, 16)
    h2, part = _tc3(num[:, :NN], den[:, :NN], hw2, aa, res2, p)
    accS = _sc_rowsum128(h2, clist, counts)[:NN]
    h4p = _tc4(accS, degp, part, p)
    acc4 = _sc_rowsum128(h4p, clist, counts)[:NN, :64]
    pool = _tc5(acc4, h4p[:, :64], degp, batch.reshape(_GRID, 1, RB), p)
    out = _tc6(pool, link_indices.reshape(2, 1, 4096), p)
    return out[:, 0]


# speedup vs baseline: 1.3490x; 1.0358x over previous
import functools
import jax
import jax.numpy as jnp
from jax import lax
from jax.experimental import pallas as pl
from jax.experimental.pallas import tpu as pltpu
from jax.experimental.pallas import tpu_sc as plsc

F32 = jnp.float32
I32 = jnp.int32

NN = 10000
EE = 160000
NT = 32
RNG = 320
NROW = 328
CH = 128
SCAN_B = 2560
NSCAN = EE // SCAN_B
CLCAP = 10368
SPILL = 10240
CLW = EE + 128

_MESH = dict(core_axis_name="c", subcore_axis_name="s",
             num_cores=2, num_subcores=16)
_PARAMS = pltpu.CompilerParams(needs_layout_passes=False)


def _coli():
    io = lax.broadcasted_iota(I32, (16,), 0)
    return [io + c * 16 for c in range(16)]


class _ColI:
    def __getitem__(self, c):
        io = lax.broadcasted_iota(I32, (16,), 0)
        return io + c * 16


_COLI = _ColI()


def _worker(c, s):
    return c * 16 + s


@functools.partial(
    pl.kernel,
    out_type=(
        jax.ShapeDtypeStruct((NT, 1, CLW), I32),
        jax.ShapeDtypeStruct((NT, 1, 16), I32),
        jax.ShapeDtypeStruct((NT * RNG,), F32),
    ),
    mesh=plsc.VectorSubcoreMesh(**_MESH),
    compiler_params=_PARAMS,
    scratch_types=[
        pltpu.VMEM((2, SCAN_B), I32),
        pltpu.VMEM((CLCAP,), I32),
        pltpu.VMEM((NROW,), F32),
        pltpu.VMEM((16,), I32),
    ],
)
def _sc_scan(ei, clist, counts, deg, ebuf, clbuf, dacc, cstg):
    c = lax.axis_index("c")
    s = lax.axis_index("s")
    t = _worker(c, s)
    base = t * RNG
    iota = lax.broadcasted_iota(I32, (16,), 0)
    ones16 = jnp.ones((16,), F32)
    zero16 = jnp.zeros((16,), F32)

    def dz(r, _):
        dacc[pl.ds(r * 16, 16)] = zero16
        return 0

    lax.fori_loop(0, NROW // 16, dz, 0)

    def chunk(g, carry):
        pltpu.sync_copy(ei.at[:, pl.ds(g * SCAN_B, SCAN_B)], ebuf)

        def group(j, carry):
            off, hoff = carry
            s16 = ebuf[0, pl.ds(j * 16, 16)]
            d16 = ebuf[1, pl.ds(j * 16, 16)]
            ok = (d16 >= base) & (d16 < base + RNG)
            key = jnp.where(ok, iota, 16)
            _, sv = plsc.sort_key_val(key, s16 * 16384 + d16)
            clbuf[pl.ds(off, 16)] = sv
            cntv = plsc.all_reduce_population_count(ok)
            off = off + cntv[0]
            loc = jnp.clip(d16 - base, 0, RNG)
            plsc.addupdate_scatter(dacc, [loc], jnp.where(ok, ones16, zero16))

            @pl.when(off >= SPILL)
            def _():
                pltpu.sync_copy(
                    clbuf.at[pl.ds(0, SPILL)],
                    clist.at[t, 0, pl.ds(pl.multiple_of(hoff, 128), SPILL)])
                tail = clbuf[pl.ds(SPILL, 16)]
                clbuf[pl.ds(0, 16)] = tail

            spilled = off >= SPILL
            off = jnp.where(spilled, off - SPILL, off)
            hoff = jnp.where(spilled, hoff + SPILL, hoff)
            return (off, hoff)

        return lax.fori_loop(0, SCAN_B // 16, group, carry)

    off, hoff = lax.fori_loop(0, NSCAN, chunk, (0, 0))

    padv = jnp.zeros((16,), I32) + (base + RNG)
    for i in range(8):
        clbuf[pl.ds(off + i * 16, 16)] = padv
    nsp = (off + 127) // 128

    def spill_fin(g, _):
        o = pl.multiple_of(g * 128, 128)
        pltpu.sync_copy(
            clbuf.at[pl.ds(o, 128)],
            clist.at[t, 0, pl.ds(pl.multiple_of(hoff, 128) + o, 128)])
        return 0

    lax.fori_loop(0, nsp, spill_fin, 0)
    cstg[pl.ds(0, 16)] = jnp.zeros((16,), I32) + (hoff + nsp * 128)
    pltpu.sync_copy(cstg, counts.at[t, 0])
    pltpu.sync_copy(dacc.at[pl.ds(0, RNG)], deg.at[pl.ds(base, RNG)])


def _read_count(counts, cntb, t):
    pltpu.sync_copy(counts.at[t, 0], cntb)
    return cntb[pl.ds(0, 16)][0]


def _unpack_chunk(gibuf, dlbuf, base):
    for j in range(CH // 16):
        v16 = gibuf[pl.ds(j * 16, 16)]
        d16 = v16 & 16383
        dlbuf[pl.ds(j * 16, 16)] = jnp.clip(d16 - base, 0, RNG)
        gibuf[pl.ds(j * 16, 16)] = lax.shift_right_logical(v16, 14)


def _drain(acc, out_slice, base):
    for k in range(RNG // 64):
        pltpu.sync_copy(acc.at[pl.ds(k * 64, 64)],
                        out_slice.at[pl.ds(base + k * 64, 64)])


def _zero_acc(acc, w):
    z = jnp.zeros((16,), F32)

    def za(r, _):
        for j in range(w // 16):
            acc[r, pl.ds(j * 16, 16)] = z
        return 0

    lax.fori_loop(0, NROW, za, 0)


NQ = 4
QB = CH // NQ


def _make_rowsum(w, nbuf):
    @functools.partial(
        pl.kernel,
        out_type=jax.ShapeDtypeStruct((NT * RNG, w), F32),
        mesh=plsc.VectorSubcoreMesh(**_MESH),
        compiler_params=_PARAMS,
        scratch_types=[
            pltpu.VMEM((NROW, w), F32),
            pltpu.VMEM((nbuf, CH, w), F32),
            pltpu.VMEM((nbuf, CH), I32),
            pltpu.VMEM((nbuf, CH), I32),
            pltpu.VMEM((16,), I32),
        ] + [pltpu.SemaphoreType.DMA] * nbuf,
    )
    def rowsum(table, clist, counts, out, acc, rowbuf, gibuf, dlbuf, cntb,
               *sems):
        c = lax.axis_index("c")
        s = lax.axis_index("s")
        t = _worker(c, s)
        base = t * RNG
        _zero_acc(acc, w)
        nch = _read_count(counts, cntb, t) // CH

        def prep_fire(g, b):
            pltpu.sync_copy(clist.at[t, 0, pl.ds(g * CH, CH)], gibuf.at[b])
            for j in range(CH // 16):
                v16 = gibuf[b, pl.ds(j * 16, 16)]
                d16 = v16 & 16383
                dlbuf[b, pl.ds(j * 16, 16)] = jnp.clip(d16 - base, 0, RNG)
                gibuf[b, pl.ds(j * 16, 16)] = lax.shift_right_logical(v16, 14)
            pltpu.async_copy(table.at[gibuf.at[b]], rowbuf.at[b], sems[b])

        def wait_rmw(b):
            pltpu.make_async_copy(table.at[gibuf.at[b]], rowbuf.at[b],
                                  sems[b]).wait()

            def rmw(j, _):
                dv = dlbuf[b, pl.ds(j * 16, 16)]
                for l in range(16):
                    rows = lax.broadcast(dv[l], (16,))
                    r = j * 16 + l
                    for col in range(w // 16):
                        plsc.addupdate_scatter(
                            acc, [rows, _COLI[col]],
                            rowbuf[b, r, pl.ds(col * 16, 16)])
                return 0

            lax.fori_loop(0, CH // 16, rmw, 0)

        if nbuf == 1:
            def chunk(g, _):
                prep_fire(g, 0)
                wait_rmw(0)
                return 0

            lax.fori_loop(0, nch, chunk, 0)
        else:
            @pl.when(nch > 0)
            def _():
                prep_fire(0, 0)

            def pairbody(i, _):
                k1 = 2 * i + 1

                @pl.when(k1 < nch)
                def _():
                    prep_fire(k1, 1)

                wait_rmw(0)

                @pl.when(k1 < nch)
                def _():
                    @pl.when(k1 + 1 < nch)
                    def _():
                        prep_fire(k1 + 1, 0)

                    wait_rmw(1)

                return 0

            lax.fori_loop(0, (nch + 1) // 2, pairbody, 0)
        _drain(acc, out, base)

    return rowsum


_sc_rowsum256 = _make_rowsum(256, 1)
_sc_rowsum128 = _make_rowsum(128, 2)


@functools.partial(
    pl.kernel,
    out_type=(
        jax.ShapeDtypeStruct((4, NT * RNG, 128), F32),
        jax.ShapeDtypeStruct((4, NT * RNG * 16), F32),
    ),
    mesh=plsc.VectorSubcoreMesh(**_MESH),
    compiler_params=_PARAMS,
    scratch_types=[
        pltpu.VMEM((NROW, 128), F32),
        pltpu.VMEM((NROW * 16,), F32),
        pltpu.VMEM((2 * NN,), F32),
        pltpu.VMEM((2, CH, 128), F32),
        pltpu.VMEM((2, CH), F32),
        pltpu.VMEM((2, CH), I32),
        pltpu.VMEM((2, CH), I32),
        pltpu.VMEM((16,), I32),
    ] + [pltpu.SemaphoreType.DMA] * 2,
)
def _sc_gat(hw2, aa4, clist, counts, num_out, den_out,
            nacc, dacc, aav, rowbuf, wrow, gibuf, dlbuf, cntb, *sems):
    c = lax.axis_index("c")
    s = lax.axis_index("s")
    t = _worker(c, s)
    base = t * RNG
    lane0 = jnp.where(lax.broadcasted_iota(I32, (16,), 0) == 0, 1.0, 0.0)
    nch = _read_count(counts, cntb, t) // CH

    def head(h, _):
        pltpu.sync_copy(aa4.at[h, 0], aav)
        _zero_acc(nacc, 128)

        def dz(r, _):
            dacc[pl.ds(r * 16, 16)] = jnp.zeros((16,), F32)
            return 0

        lax.fori_loop(0, NROW, dz, 0)

        def prep_fire(g, b):
            pltpu.sync_copy(clist.at[t, 0, pl.ds(g * CH, CH)], gibuf.at[b])
            for j in range(CH // 16):
                v16 = gibuf[b, pl.ds(j * 16, 16)]
                d16 = v16 & 16383
                s16 = lax.shift_right_logical(v16, 14)
                a_s = plsc.load_gather(aav, [s16 * 2])
                a_d = plsc.load_gather(aav, [jnp.minimum(d16, NN - 1) * 2 + 1])
                e = a_s + a_d
                e = jnp.maximum(e, 0.2 * e)
                wrow[b, pl.ds(j * 16, 16)] = jnp.exp(e)
                dlbuf[b, pl.ds(j * 16, 16)] = jnp.clip(d16 - base, 0, RNG)
                gibuf[b, pl.ds(j * 16, 16)] = s16 * 4 + h
            pltpu.async_copy(hw2.at[gibuf.at[b]], rowbuf.at[b], sems[b])

        def wait_rmw(b):
            pltpu.make_async_copy(hw2.at[gibuf.at[b]], rowbuf.at[b],
                                  sems[b]).wait()

            def rmw(j, _):
                dv = dlbuf[b, pl.ds(j * 16, 16)]
                wv = wrow[b, pl.ds(j * 16, 16)]
                for l in range(16):
                    rows = lax.broadcast(dv[l], (16,))
                    r = j * 16 + l
                    wl = wv[l]
                    for col in range(8):
                        plsc.addupdate_scatter(
                            nacc, [rows, _COLI[col]],
                            rowbuf[b, r, pl.ds(col * 16, 16)] * wl)
                    plsc.addupdate_scatter(dacc, [rows * 16 + _COLI[0]],
                                           lane0 * wl)
                return 0

            lax.fori_loop(0, CH // 16, rmw, 0)

        @pl.when(nch > 0)
        def _():
            prep_fire(0, 0)

        def pairbody(i, _):
            k1 = 2 * i + 1

            @pl.when(k1 < nch)
            def _():
                prep_fire(k1, 1)

            wait_rmw(0)

            @pl.when(k1 < nch)
            def _():
                @pl.when(k1 + 1 < nch)
                def _():
                    prep_fire(k1 + 1, 0)

                wait_rmw(1)

            return 0

        lax.fori_loop(0, (nch + 1) // 2, pairbody, 0)
        _drain(nacc, num_out.at[h], base)
        pltpu.sync_copy(dacc.at[pl.ds(0, RNG * 16)],
                        den_out.at[h, pl.ds(base * 16, RNG * 16)])
        return 0

    lax.fori_loop(0, 4, head, 0)


RB = 1000
_GRID = NN // RB


def _ln(h, g, b):
    mu = jnp.mean(h, axis=-1, keepdims=True)
    var = jnp.mean((h - mu) ** 2, axis=-1, keepdims=True)
    return (h - mu) * lax.rsqrt(var + 1e-5) * g + b


def _rows(c):
    return pl.BlockSpec((RB, c), lambda i: (i, 0))


def _full(*shape):
    nd = len(shape)
    return pl.BlockSpec(shape, lambda i: (0,) * nd)


def _tc1_body(x, W, degp, out):
    dinv = lax.rsqrt(degp[...] + 1.0)
    out[...] = jnp.dot(x[...], W[...], preferred_element_type=F32) * dinv


def _tc1(x, W, degp):
    return pl.pallas_call(
        _tc1_body,
        grid=(_GRID,),
        in_specs=[_rows(256), _full(256, 256), _rows(1)],
        out_specs=_rows(256),
        out_shape=jax.ShapeDtypeStruct((NN, 256), F32),
    )(x, W, degp)


def _tc2_body(acc1, hw1p, degp, b1, g1, be1, Wg, asr, ads, Wr2, br2,
              hw2_o, aa_o, res2_o):
    dinv = lax.rsqrt(degp[...] + 1.0)
    h1 = jax.nn.relu(_ln(dinv * (acc1[...] + hw1p[...]) + b1[...],
                         g1[...], be1[...]))
    hw2 = jnp.dot(h1, Wg[...], preferred_element_type=F32)
    hw2_o[...] = hw2
    heads = []
    for h in range(4):
        blk = hw2[:, h * 128:(h + 1) * 128]
        a_s = jnp.sum(blk * asr[...][h][None, :], axis=1, keepdims=True)
        a_d = jnp.sum(blk * ads[...][h][None, :], axis=1, keepdims=True)
        heads.append(jnp.concatenate([a_s, a_d], axis=1)[None])
    aa_o[...] = jnp.concatenate(heads, axis=0)
    res2_o[...] = jnp.dot(h1, Wr2[...], preferred_element_type=F32) + br2[...]


def _tc2(acc1, hw1p, degp, p):
    return pl.pallas_call(
        _tc2_body,
        grid=(_GRID,),
        in_specs=[_rows(256), _rows(256), _rows(1), _full(256), _full(256),
                  _full(256), _full(256, 512), _full(4, 128), _full(4, 128),
                  _full(256, 128), _full(128)],
        out_specs=[_rows(512), pl.BlockSpec((4, RB, 2), lambda i: (0, i, 0)),
                   _rows(128)],
        out_shape=[jax.ShapeDtypeStruct((NN, 512), F32),
                   jax.ShapeDtypeStruct((4, NN, 2), F32),
                   jax.ShapeDtypeStruct((NN, 128), F32)],
    )(acc1, hw1p, degp, p["gcn1_b"], p["ln1_g"], p["ln1_b"], p["gat2_W"],
      p["gat2_att_src"], p["gat2_att_dst"], p["res2_W"], p["res2_b"])


def _tc3_body(num, den, hw2, aa, res2, bg, g2, be2, Wc, bc,
              h2_o, part_o):
    aam = aa[...]
    hw2m = hw2[...]
    gat = jnp.zeros_like(res2[...])
    for h in range(4):
        esl = aam[h, :, 0:1] + aam[h, :, 1:2]
        wsl = jnp.exp(jnp.maximum(esl, 0.2 * esl))
        nh = num[...][h] + wsl * hw2m[:, h * 128:(h + 1) * 128]
        sh = den[...][h][:, 0:1] + wsl
        gat = gat + nh / (sh + 1e-16)
    gat = gat * 0.25 + bg[...]
    h2 = jax.nn.relu(_ln(gat + res2[...], g2[...], be2[...]))
    h2_o[...] = h2
    part_o[...] = jnp.dot(h2, Wc[...], preferred_element_type=F32) + bc[...]


def _tc3(num, den, hw2, aa, res2, p):
    return pl.pallas_call(
        _tc3_body,
        grid=(_GRID,),
        in_specs=[pl.BlockSpec((4, RB, 128), lambda i: (0, i, 0)),
                  pl.BlockSpec((4, RB, 16), lambda i: (0, i, 0)),
                  _rows(512), pl.BlockSpec((4, RB, 2), lambda i: (0, i, 0)),
                  _rows(128), _full(128), _full(128),
                  _full(128), _full(128, 64), _full(64)],
        out_specs=[_rows(128), _rows(64)],
        out_shape=[jax.ShapeDtypeStruct((NN, 128), F32),
                   jax.ShapeDtypeStruct((NN, 64), F32)],
    )(num, den, hw2, aa, res2, p["gat2_b"], p["ln2_g"], p["ln2_b"],
      p["sage3_Wr"] + p["res3_W"],
      p["res3_b"] + p["sage3_bl"])


def _tc4_body(accS, degp, part, Wl, g3, be3, W4, h4p_o):
    deg = degp[...]
    mean = accS[...] / jnp.maximum(deg, 1.0)
    h3 = jax.nn.relu(_ln(jnp.dot(mean, Wl[...], preferred_element_type=F32)
                         + part[...], g3[...], be3[...]))
    h4p = jnp.dot(h3, W4[...], preferred_element_type=F32) * lax.rsqrt(deg + 1.0)
    h4p_o[...] = jnp.concatenate([h4p, jnp.zeros((RB, 64), F32)], axis=1)


def _tc4(accS, degp, part, p):
    return pl.pallas_call(
        _tc4_body,
        grid=(_GRID,),
        in_specs=[_rows(128), _rows(1), _rows(64), _full(128, 64), _full(64),
                  _full(64), _full(64, 64)],
        out_specs=_rows(128),
        out_shape=jax.ShapeDtypeStruct((NN, 128), F32),
    )(accS, degp, part, p["sage3_Wl"], p["ln3_g"], p["ln3_b"], p["gc4_W"])


def _tc5_body(acc4, h4p, degp, batch3, b4, g4, be4, Wro, bro, pool_o):
    i = pl.program_id(0)
    dinv = lax.rsqrt(degp[...] + 1.0)
    h4 = jax.nn.relu(_ln(dinv * (acc4[...] + h4p[...]) + b4[...],
                         g4[...], be4[...]))
    gate = jax.nn.sigmoid(jnp.dot(h4, Wro[...], preferred_element_type=F32) + bro[...])
    gated = h4 * gate
    b = batch3[...][0, 0, :]
    P = (lax.broadcasted_iota(I32, (64, RB), 0) == b[None, :]).astype(F32)
    rhs = jnp.concatenate([gated, gate, jnp.zeros((RB, 63), F32)], axis=1)
    blk = jnp.dot(P, rhs, preferred_element_type=F32)

    @pl.when(i == 0)
    def _():
        pool_o[...] = jnp.zeros_like(pool_o)

    pool_o[...] += blk


def _tc5(acc4, h4p, degp, batch3, p):
    return pl.pallas_call(
        _tc5_body,
        grid=(_GRID,),
        in_specs=[_rows(64), _rows(64), _rows(1),
                  pl.BlockSpec((1, 1, RB), lambda i: (i, 0, 0)),
                  _full(64), _full(64), _full(64), _full(64, 1), _full(1)],
        out_specs=_full(64, 128),
        out_shape=jax.ShapeDtypeStruct((64, 128), F32),
    )(acc4, h4p, degp, batch3, p["gc4_b"], p["ln4_g"], p["ln4_b"],
      p["ro_W"], p["ro_b"])


def _tc6_body(pool, link3, W1, b1, W2, b2, W3, b3, out_o):
    pm = pool[...]
    emb = pm[:, :64] / (pm[:, 64:65] + 1e-8)
    lk = link3[...][:, 0, :]
    g_iota = lax.broadcasted_iota(I32, (4096, 64), 1)
    o1 = (lk[0][:, None] == g_iota).astype(F32)
    o2 = (lk[1][:, None] == g_iota).astype(F32)
    f = jnp.concatenate([
        jnp.dot(o1, emb, preferred_element_type=F32),
        jnp.dot(o2, emb, preferred_element_type=F32)], axis=1)
    f = jax.nn.relu(jnp.dot(f, W1[...], preferred_element_type=F32) + b1[...])
    f = jax.nn.relu(jnp.dot(f, W2[...], preferred_element_type=F32) + b2[...])
    out_o[...] = jax.nn.sigmoid(jnp.dot(f, W3[...], preferred_element_type=F32) + b3[...])


def _tc6(pool, link3, p):
    return pl.pallas_call(
        _tc6_body,
        in_specs=[pl.BlockSpec((64, 128), lambda: (0, 0)),
                  pl.BlockSpec((2, 1, 4096), lambda: (0, 0, 0)),
                  pl.BlockSpec((128, 64), lambda: (0, 0)),
                  pl.BlockSpec((64,), lambda: (0,)),
                  pl.BlockSpec((64, 64), lambda: (0, 0)),
                  pl.BlockSpec((64,), lambda: (0,)),
                  pl.BlockSpec((64, 1), lambda: (0, 0)),
                  pl.BlockSpec((1,), lambda: (0,)),
                  ],
        out_specs=pl.BlockSpec((4096, 1), lambda: (0, 0)),
        out_shape=jax.ShapeDtypeStruct((4096, 1), F32),
    )(pool, link3, p["mlp1_W"], p["mlp1_b"], p["mlp2_W"], p["mlp2_b"],
      p["mlp3_W"], p["mlp3_b"])


def kernel(x, edge_index, batch, link_indices, params):
    p = params

    clist, counts, deg = _sc_scan(edge_index)
    degp = deg[:NN, None]
    hw1p = _tc1(x, p["gcn1_W"], degp)
    acc1 = _sc_rowsum256(hw1p, clist, counts)[:NN]
    hw2, aa, res2 = _tc2(acc1, hw1p, degp, p)
    num, den = _sc_gat(hw2.reshape(NN * 4, 128),
                       aa.reshape(4, 1, 2 * NN), clist, counts)
    den = den.reshape(4, NT * RNG, 16)
    h2, part = _tc3(num[:, :NN], den[:, :NN], hw2, aa, res2, p)
    accS = _sc_rowsum128(h2, clist, counts)[:NN]
    h4p = _tc4(accS, degp, part, p)
    acc4 = _sc_rowsum128(h4p, clist, counts)[:NN, :64]
    pool = _tc5(acc4, h4p[:, :64], degp, batch.reshape(_GRID, 1, RB), p)
    out = _tc6(pool, link_indices.reshape(2, 1, 4096), p)
    return out[:, 0]


# speedup vs baseline: 1.3510x; 1.0015x over previous
import functools
import jax
import jax.numpy as jnp
from jax import lax
from jax.experimental import pallas as pl
from jax.experimental.pallas import tpu as pltpu
from jax.experimental.pallas import tpu_sc as plsc

F32 = jnp.float32
I32 = jnp.int32

NN = 10000
EE = 160000
NT = 32
RNG = 320
NROW = 328
CH = 128
SCAN_B = 3200
NSCAN = EE // SCAN_B
assert NSCAN * SCAN_B == EE
CLCAP = 10368
SPILL = 10240
CLW = EE + 128

_MESH = dict(core_axis_name="c", subcore_axis_name="s",
             num_cores=2, num_subcores=16)
_PARAMS = pltpu.CompilerParams(needs_layout_passes=False)


def _coli():
    io = lax.broadcasted_iota(I32, (16,), 0)
    return [io + c * 16 for c in range(16)]


class _ColI:
    def __getitem__(self, c):
        io = lax.broadcasted_iota(I32, (16,), 0)
        return io + c * 16


_COLI = _ColI()


def _worker(c, s):
    return c * 16 + s


@functools.partial(
    pl.kernel,
    out_type=(
        jax.ShapeDtypeStruct((NT, 1, CLW), I32),
        jax.ShapeDtypeStruct((NT, 1, 16), I32),
        jax.ShapeDtypeStruct((NT * RNG,), F32),
    ),
    mesh=plsc.VectorSubcoreMesh(**_MESH),
    compiler_params=_PARAMS,
    scratch_types=[
        pltpu.VMEM((2, SCAN_B), I32),
        pltpu.VMEM((CLCAP,), I32),
        pltpu.VMEM((NROW,), F32),
        pltpu.VMEM((16,), I32),
    ],
)
def _sc_scan(ei, clist, counts, deg, ebuf, clbuf, dacc, cstg):
    c = lax.axis_index("c")
    s = lax.axis_index("s")
    t = _worker(c, s)
    base = t * RNG
    iota = lax.broadcasted_iota(I32, (16,), 0)
    ones16 = jnp.ones((16,), F32)
    zero16 = jnp.zeros((16,), F32)

    def dz(r, _):
        dacc[pl.ds(r * 16, 16)] = zero16
        return 0

    lax.fori_loop(0, NROW // 16, dz, 0)

    def chunk(g, carry):
        pltpu.sync_copy(ei.at[:, pl.ds(g * SCAN_B, SCAN_B)], ebuf)

        def group(j, carry):
            off, hoff = carry
            s16 = ebuf[0, pl.ds(j * 16, 16)]
            d16 = ebuf[1, pl.ds(j * 16, 16)]
            ok = (d16 >= base) & (d16 < base + RNG)
            key = jnp.where(ok, iota, 16)
            _, sv = plsc.sort_key_val(key, s16 * 16384 + d16)
            clbuf[pl.ds(off, 16)] = sv
            cntv = plsc.all_reduce_population_count(ok)
            off = off + cntv[0]
            loc = jnp.clip(d16 - base, 0, RNG)
            plsc.addupdate_scatter(dacc, [loc], jnp.where(ok, ones16, zero16))

            @pl.when(off >= SPILL)
            def _():
                pltpu.sync_copy(
                    clbuf.at[pl.ds(0, SPILL)],
                    clist.at[t, 0, pl.ds(pl.multiple_of(hoff, 128), SPILL)])
                tail = clbuf[pl.ds(SPILL, 16)]
                clbuf[pl.ds(0, 16)] = tail

            spilled = off >= SPILL
            off = jnp.where(spilled, off - SPILL, off)
            hoff = jnp.where(spilled, hoff + SPILL, hoff)
            return (off, hoff)

        return lax.fori_loop(0, SCAN_B // 16, group, carry)

    off, hoff = lax.fori_loop(0, NSCAN, chunk, (0, 0))

    padv = jnp.zeros((16,), I32) + (base + RNG)
    for i in range(8):
        clbuf[pl.ds(off + i * 16, 16)] = padv
    nsp = (off + 127) // 128

    def spill_fin(g, _):
        o = pl.multiple_of(g * 128, 128)
        pltpu.sync_copy(
            clbuf.at[pl.ds(o, 128)],
            clist.at[t, 0, pl.ds(pl.multiple_of(hoff, 128) + o, 128)])
        return 0

    lax.fori_loop(0, nsp, spill_fin, 0)
    cstg[pl.ds(0, 16)] = jnp.zeros((16,), I32) + (hoff + nsp * 128)
    pltpu.sync_copy(cstg, counts.at[t, 0])
    pltpu.sync_copy(dacc.at[pl.ds(0, RNG)], deg.at[pl.ds(base, RNG)])


def _read_count(counts, cntb, t):
    pltpu.sync_copy(counts.at[t, 0], cntb)
    return cntb[pl.ds(0, 16)][0]


def _unpack_chunk(gibuf, dlbuf, base):
    for j in range(CH // 16):
        v16 = gibuf[pl.ds(j * 16, 16)]
        d16 = v16 & 16383
        dlbuf[pl.ds(j * 16, 16)] = jnp.clip(d16 - base, 0, RNG)
        gibuf[pl.ds(j * 16, 16)] = lax.shift_right_logical(v16, 14)


def _drain(acc, out_slice, base):
    for k in range(RNG // 64):
        pltpu.sync_copy(acc.at[pl.ds(k * 64, 64)],
                        out_slice.at[pl.ds(base + k * 64, 64)])


def _zero_acc(acc, w):
    z = jnp.zeros((16,), F32)

    def za(r, _):
        for j in range(w // 16):
            acc[r, pl.ds(j * 16, 16)] = z
        return 0

    lax.fori_loop(0, NROW, za, 0)


NQ = 4
QB = CH // NQ


def _make_rowsum(w, nbuf):
    @functools.partial(
        pl.kernel,
        out_type=jax.ShapeDtypeStruct((NT * RNG, w), F32),
        mesh=plsc.VectorSubcoreMesh(**_MESH),
        compiler_params=_PARAMS,
        scratch_types=[
            pltpu.VMEM((NROW, w), F32),
            pltpu.VMEM((nbuf, CH, w), F32),
            pltpu.VMEM((nbuf, CH), I32),
            pltpu.VMEM((nbuf, CH), I32),
            pltpu.VMEM((16,), I32),
        ] + [pltpu.SemaphoreType.DMA] * nbuf,
    )
    def rowsum(table, clist, counts, out, acc, rowbuf, gibuf, dlbuf, cntb,
               *sems):
        c = lax.axis_index("c")
        s = lax.axis_index("s")
        t = _worker(c, s)
        base = t * RNG
        _zero_acc(acc, w)
        nch = _read_count(counts, cntb, t) // CH

        def prep_fire(g, b):
            pltpu.sync_copy(clist.at[t, 0, pl.ds(g * CH, CH)], gibuf.at[b])
            for j in range(CH // 16):
                v16 = gibuf[b, pl.ds(j * 16, 16)]
                d16 = v16 & 16383
                dlbuf[b, pl.ds(j * 16, 16)] = jnp.clip(d16 - base, 0, RNG)
                gibuf[b, pl.ds(j * 16, 16)] = lax.shift_right_logical(v16, 14)
            pltpu.async_copy(table.at[gibuf.at[b]], rowbuf.at[b], sems[b])

        def wait_rmw(b):
            pltpu.make_async_copy(table.at[gibuf.at[b]], rowbuf.at[b],
                                  sems[b]).wait()

            def rmw(j, _):
                dv = dlbuf[b, pl.ds(j * 16, 16)]
                for l in range(16):
                    rows = lax.broadcast(dv[l], (16,))
                    r = j * 16 + l
                    for col in range(w // 16):
                        plsc.addupdate_scatter(
                            acc, [rows, _COLI[col]],
                            rowbuf[b, r, pl.ds(col * 16, 16)])
                return 0

            lax.fori_loop(0, CH // 16, rmw, 0)

        if nbuf == 1:
            def chunk(g, _):
                prep_fire(g, 0)
                wait_rmw(0)
                return 0

            lax.fori_loop(0, nch, chunk, 0)
        else:
            @pl.when(nch > 0)
            def _():
                prep_fire(0, 0)

            def pairbody(i, _):
                k1 = 2 * i + 1

                @pl.when(k1 < nch)
                def _():
                    prep_fire(k1, 1)

                wait_rmw(0)

                @pl.when(k1 < nch)
                def _():
                    @pl.when(k1 + 1 < nch)
                    def _():
                        prep_fire(k1 + 1, 0)

                    wait_rmw(1)

                return 0

            lax.fori_loop(0, (nch + 1) // 2, pairbody, 0)
        _drain(acc, out, base)

    return rowsum


_sc_rowsum256 = _make_rowsum(256, 1)
_sc_rowsum128 = _make_rowsum(128, 2)


@functools.partial(
    pl.kernel,
    out_type=(
        jax.ShapeDtypeStruct((4, NT * RNG, 128), F32),
        jax.ShapeDtypeStruct((4, NT * RNG * 16), F32),
    ),
    mesh=plsc.VectorSubcoreMesh(**_MESH),
    compiler_params=_PARAMS,
    scratch_types=[
        pltpu.VMEM((NROW, 128), F32),
        pltpu.VMEM((NROW * 16,), F32),
        pltpu.VMEM((2 * NN,), F32),
        pltpu.VMEM((2, CH, 128), F32),
        pltpu.VMEM((2, CH), F32),
        pltpu.VMEM((2, CH), I32),
        pltpu.VMEM((2, CH), I32),
        pltpu.VMEM((16,), I32),
    ] + [pltpu.SemaphoreType.DMA] * 2,
)
def _sc_gat(hw2, aa4, clist, counts, num_out, den_out,
            nacc, dacc, aav, rowbuf, wrow, gibuf, dlbuf, cntb, *sems):
    c = lax.axis_index("c")
    s = lax.axis_index("s")
    t = _worker(c, s)
    base = t * RNG
    lane0 = jnp.where(lax.broadcasted_iota(I32, (16,), 0) == 0, 1.0, 0.0)
    nch = _read_count(counts, cntb, t) // CH

    def head(h, _):
        pltpu.sync_copy(aa4.at[h, 0], aav)
        _zero_acc(nacc, 128)

        def dz(r, _):
            dacc[pl.ds(r * 16, 16)] = jnp.zeros((16,), F32)
            return 0

        lax.fori_loop(0, NROW, dz, 0)

        def prep_fire(g, b):
            pltpu.sync_copy(clist.at[t, 0, pl.ds(g * CH, CH)], gibuf.at[b])
            for j in range(CH // 16):
                v16 = gibuf[b, pl.ds(j * 16, 16)]
                d16 = v16 & 16383
                s16 = lax.shift_right_logical(v16, 14)
                a_s = plsc.load_gather(aav, [s16 * 2])
                a_d = plsc.load_gather(aav, [jnp.minimum(d16, NN - 1) * 2 + 1])
                e = a_s + a_d
                e = jnp.maximum(e, 0.2 * e)
                wrow[b, pl.ds(j * 16, 16)] = jnp.exp(e)
                dlbuf[b, pl.ds(j * 16, 16)] = jnp.clip(d16 - base, 0, RNG)
                gibuf[b, pl.ds(j * 16, 16)] = s16 * 4 + h
            pltpu.async_copy(hw2.at[gibuf.at[b]], rowbuf.at[b], sems[b])

        def wait_rmw(b):
            pltpu.make_async_copy(hw2.at[gibuf.at[b]], rowbuf.at[b],
                                  sems[b]).wait()

            def rmw(j, _):
                dv = dlbuf[b, pl.ds(j * 16, 16)]
                wv = wrow[b, pl.ds(j * 16, 16)]
                for l in range(16):
                    rows = lax.broadcast(dv[l], (16,))
                    r = j * 16 + l
                    wl = wv[l]
                    for col in range(8):
                        plsc.addupdate_scatter(
                            nacc, [rows, _COLI[col]],
                            rowbuf[b, r, pl.ds(col * 16, 16)] * wl)
                    plsc.addupdate_scatter(dacc, [rows * 16 + _COLI[0]],
                                           lane0 * wl)
                return 0

            lax.fori_loop(0, CH // 16, rmw, 0)

        @pl.when(nch > 0)
        def _():
            prep_fire(0, 0)

        def pairbody(i, _):
            k1 = 2 * i + 1

            @pl.when(k1 < nch)
            def _():
                prep_fire(k1, 1)

            wait_rmw(0)

            @pl.when(k1 < nch)
            def _():
                @pl.when(k1 + 1 < nch)
                def _():
                    prep_fire(k1 + 1, 0)

                wait_rmw(1)

            return 0

        lax.fori_loop(0, (nch + 1) // 2, pairbody, 0)
        _drain(nacc, num_out.at[h], base)
        pltpu.sync_copy(dacc.at[pl.ds(0, RNG * 16)],
                        den_out.at[h, pl.ds(base * 16, RNG * 16)])
        return 0

    lax.fori_loop(0, 4, head, 0)


RB = 1000
_GRID = NN // RB


def _ln(h, g, b):
    mu = jnp.mean(h, axis=-1, keepdims=True)
    var = jnp.mean((h - mu) ** 2, axis=-1, keepdims=True)
    return (h - mu) * lax.rsqrt(var + 1e-5) * g + b


def _rows(c):
    return pl.BlockSpec((RB, c), lambda i: (i, 0))


def _full(*shape):
    nd = len(shape)
    return pl.BlockSpec(shape, lambda i: (0,) * nd)


def _tc1_body(x, W, degp, out):
    dinv = lax.rsqrt(degp[...] + 1.0)
    out[...] = jnp.dot(x[...], W[...], preferred_element_type=F32) * dinv


def _tc1(x, W, degp):
    return pl.pallas_call(
        _tc1_body,
        grid=(_GRID,),
        in_specs=[_rows(256), _full(256, 256), _rows(1)],
        out_specs=_rows(256),
        out_shape=jax.ShapeDtypeStruct((NN, 256), F32),
    )(x, W, degp)


def _tc2_body(acc1, hw1p, degp, b1, g1, be1, Wg, asr, ads, Wr2, br2,
              hw2_o, aa_o, res2_o):
    dinv = lax.rsqrt(degp[...] + 1.0)
    h1 = jax.nn.relu(_ln(dinv * (acc1[...] + hw1p[...]) + b1[...],
                         g1[...], be1[...]))
    hw2 = jnp.dot(h1, Wg[...], preferred_element_type=F32)
    hw2_o[...] = hw2
    heads = []
    for h in range(4):
        blk = hw2[:, h * 128:(h + 1) * 128]
        a_s = jnp.sum(blk * asr[...][h][None, :], axis=1, keepdims=True)
        a_d = jnp.sum(blk * ads[...][h][None, :], axis=1, keepdims=True)
        heads.append(jnp.concatenate([a_s, a_d], axis=1)[None])
    aa_o[...] = jnp.concatenate(heads, axis=0)
    res2_o[...] = jnp.dot(h1, Wr2[...], preferred_element_type=F32) + br2[...]


def _tc2(acc1, hw1p, degp, p):
    return pl.pallas_call(
        _tc2_body,
        grid=(_GRID,),
        in_specs=[_rows(256), _rows(256), _rows(1), _full(256), _full(256),
                  _full(256), _full(256, 512), _full(4, 128), _full(4, 128),
                  _full(256, 128), _full(128)],
        out_specs=[_rows(512), pl.BlockSpec((4, RB, 2), lambda i: (0, i, 0)),
                   _rows(128)],
        out_shape=[jax.ShapeDtypeStruct((NN, 512), F32),
                   jax.ShapeDtypeStruct((4, NN, 2), F32),
                   jax.ShapeDtypeStruct((NN, 128), F32)],
    )(acc1, hw1p, degp, p["gcn1_b"], p["ln1_g"], p["ln1_b"], p["gat2_W"],
      p["gat2_att_src"], p["gat2_att_dst"], p["res2_W"], p["res2_b"])


def _tc3_body(num, den, hw2, aa, res2, bg, g2, be2, Wc, bc,
              h2_o, part_o):
    aam = aa[...]
    hw2m = hw2[...]
    gat = jnp.zeros_like(res2[...])
    for h in range(4):
        esl = aam[h, :, 0:1] + aam[h, :, 1:2]
        wsl = jnp.exp(jnp.maximum(esl, 0.2 * esl))
        nh = num[...][h] + wsl * hw2m[:, h * 128:(h + 1) * 128]
        sh = den[...][h][:, 0:1] + wsl
        gat = gat + nh / (sh + 1e-16)
    gat = gat * 0.25 + bg[...]
    h2 = jax.nn.relu(_ln(gat + res2[...], g2[...], be2[...]))
    h2_o[...] = h2
    part_o[...] = jnp.dot(h2, Wc[...], preferred_element_type=F32) + bc[...]


def _tc3(num, den, hw2, aa, res2, p):
    return pl.pallas_call(
        _tc3_body,
        grid=(_GRID,),
        in_specs=[pl.BlockSpec((4, RB, 128), lambda i: (0, i, 0)),
                  pl.BlockSpec((4, RB, 16), lambda i: (0, i, 0)),
                  _rows(512), pl.BlockSpec((4, RB, 2), lambda i: (0, i, 0)),
                  _rows(128), _full(128), _full(128),
                  _full(128), _full(128, 64), _full(64)],
        out_specs=[_rows(128), _rows(64)],
        out_shape=[jax.ShapeDtypeStruct((NN, 128), F32),
                   jax.ShapeDtypeStruct((NN, 64), F32)],
    )(num, den, hw2, aa, res2, p["gat2_b"], p["ln2_g"], p["ln2_b"],
      p["sage3_Wr"] + p["res3_W"],
      p["res3_b"] + p["sage3_bl"])


def _tc4_body(accS, degp, part, Wl, g3, be3, W4, h4p_o):
    deg = degp[...]
    mean = accS[...] / jnp.maximum(deg, 1.0)
    h3 = jax.nn.relu(_ln(jnp.dot(mean, Wl[...], preferred_element_type=F32)
                         + part[...], g3[...], be3[...]))
    h4p = jnp.dot(h3, W4[...], preferred_element_type=F32) * lax.rsqrt(deg + 1.0)
    h4p_o[...] = jnp.concatenate([h4p, jnp.zeros((RB, 64), F32)], axis=1)


def _tc4(accS, degp, part, p):
    return pl.pallas_call(
        _tc4_body,
        grid=(_GRID,),
        in_specs=[_rows(128), _rows(1), _rows(64), _full(128, 64), _full(64),
                  _full(64), _full(64, 64)],
        out_specs=_rows(128),
        out_shape=jax.ShapeDtypeStruct((NN, 128), F32),
    )(accS, degp, part, p["sage3_Wl"], p["ln3_g"], p["ln3_b"], p["gc4_W"])


def _tc5_body(acc4, h4p, degp, batch3, b4, g4, be4, Wro, bro, pool_o):
    i = pl.program_id(0)
    dinv = lax.rsqrt(degp[...] + 1.0)
    h4 = jax.nn.relu(_ln(dinv * (acc4[...] + h4p[...]) + b4[...],
                         g4[...], be4[...]))
    gate = jax.nn.sigmoid(jnp.dot(h4, Wro[...], preferred_element_type=F32) + bro[...])
    gated = h4 * gate
    b = batch3[...][0, 0, :]
    P = (lax.broadcasted_iota(I32, (64, RB), 0) == b[None, :]).astype(F32)
    rhs = jnp.concatenate([gated, gate, jnp.zeros((RB, 63), F32)], axis=1)
    blk = jnp.dot(P, rhs, preferred_element_type=F32)

    @pl.when(i == 0)
    def _():
        pool_o[...] = jnp.zeros_like(pool_o)

    pool_o[...] += blk


def _tc5(acc4, h4p, degp, batch3, p):
    return pl.pallas_call(
        _tc5_body,
        grid=(_GRID,),
        in_specs=[_rows(64), _rows(64), _rows(1),
                  pl.BlockSpec((1, 1, RB), lambda i: (i, 0, 0)),
                  _full(64), _full(64), _full(64), _full(64, 1), _full(1)],
        out_specs=_full(64, 128),
        out_shape=jax.ShapeDtypeStruct((64, 128), F32),
    )(acc4, h4p, degp, batch3, p["gc4_b"], p["ln4_g"], p["ln4_b"],
      p["ro_W"], p["ro_b"])


def _tc6_body(pool, link3, W1, b1, W2, b2, W3, b3, out_o):
    pm = pool[...]
    emb = pm[:, :64] / (pm[:, 64:65] + 1e-8)
    lk = link3[...][:, 0, :]
    g_iota = lax.broadcasted_iota(I32, (4096, 64), 1)
    o1 = (lk[0][:, None] == g_iota).astype(F32)
    o2 = (lk[1][:, None] == g_iota).astype(F32)
    f = jnp.concatenate([
        jnp.dot(o1, emb, preferred_element_type=F32),
        jnp.dot(o2, emb, preferred_element_type=F32)], axis=1)
    f = jax.nn.relu(jnp.dot(f, W1[...], preferred_element_type=F32) + b1[...])
    f = jax.nn.relu(jnp.dot(f, W2[...], preferred_element_type=F32) + b2[...])
    out_o[...] = jax.nn.sigmoid(jnp.dot(f, W3[...], preferred_element_type=F32) + b3[...])


def _tc6(pool, link3, p):
    return pl.pallas_call(
        _tc6_body,
        in_specs=[pl.BlockSpec((64, 128), lambda: (0, 0)),
                  pl.BlockSpec((2, 1, 4096), lambda: (0, 0, 0)),
                  pl.BlockSpec((128, 64), lambda: (0, 0)),
                  pl.BlockSpec((64,), lambda: (0,)),
                  pl.BlockSpec((64, 64), lambda: (0, 0)),
                  pl.BlockSpec((64,), lambda: (0,)),
                  pl.BlockSpec((64, 1), lambda: (0, 0)),
                  pl.BlockSpec((1,), lambda: (0,)),
                  ],
        out_specs=pl.BlockSpec((4096, 1), lambda: (0, 0)),
        out_shape=jax.ShapeDtypeStruct((4096, 1), F32),
    )(pool, link3, p["mlp1_W"], p["mlp1_b"], p["mlp2_W"], p["mlp2_b"],
      p["mlp3_W"], p["mlp3_b"])


def kernel(x, edge_index, batch, link_indices, params):
    p = params

    clist, counts, deg = _sc_scan(edge_index)
    degp = deg[:NN, None]
    hw1p = _tc1(x, p["gcn1_W"], degp)
    acc1 = _sc_rowsum256(hw1p, clist, counts)[:NN]
    hw2, aa, res2 = _tc2(acc1, hw1p, degp, p)
    num, den = _sc_gat(hw2.reshape(NN * 4, 128),
                       aa.reshape(4, 1, 2 * NN), clist, counts)
    den = den.reshape(4, NT * RNG, 16)
    h2, part = _tc3(num[:, :NN], den[:, :NN], hw2, aa, res2, p)
    accS = _sc_rowsum128(h2, clist, counts)[:NN]
    h4p = _tc4(accS, degp, part, p)
    acc4 = _sc_rowsum128(h4p, clist, counts)[:NN, :64]
    pool = _tc5(acc4, h4p[:, :64], degp, batch.reshape(_GRID, 1, RB), p)
    out = _tc6(pool, link_indices.reshape(2, 1, 4096), p)
    return out[:, 0]


# speedup vs baseline: 1.3887x; 1.0279x over previous
import functools
import jax
import jax.numpy as jnp
from jax import lax
from jax.experimental import pallas as pl
from jax.experimental.pallas import tpu as pltpu
from jax.experimental.pallas import tpu_sc as plsc

F32 = jnp.float32
I32 = jnp.int32

NN = 10000
EE = 160000
NT = 32
RNG = 320
NROW = 328
CH = 128
SCAN_B = 3200
NSCAN = EE // SCAN_B
assert NSCAN * SCAN_B == EE
CLCAP = 10368
SPILL = 10240
CLW = EE + 128

_MESH = dict(core_axis_name="c", subcore_axis_name="s",
             num_cores=2, num_subcores=16)
_PARAMS = pltpu.CompilerParams(needs_layout_passes=False)


def _coli():
    io = lax.broadcasted_iota(I32, (16,), 0)
    return [io + c * 16 for c in range(16)]


class _ColI:
    def __getitem__(self, c):
        io = lax.broadcasted_iota(I32, (16,), 0)
        return io + c * 16


_COLI = _ColI()


def _worker(c, s):
    return c * 16 + s


@functools.partial(
    pl.kernel,
    out_type=(
        jax.ShapeDtypeStruct((NT, 1, CLW), I32),
        jax.ShapeDtypeStruct((NT, 1, 16), I32),
        jax.ShapeDtypeStruct((NT * RNG,), F32),
    ),
    mesh=plsc.VectorSubcoreMesh(**_MESH),
    compiler_params=_PARAMS,
    scratch_types=[
        pltpu.VMEM((2, SCAN_B), I32),
        pltpu.VMEM((CLCAP,), I32),
        pltpu.VMEM((NROW,), F32),
        pltpu.VMEM((16,), I32),
    ],
)
def _sc_scan(ei, clist, counts, deg, ebuf, clbuf, dacc, cstg):
    c = lax.axis_index("c")
    s = lax.axis_index("s")
    t = _worker(c, s)
    base = t * RNG
    iota = lax.broadcasted_iota(I32, (16,), 0)
    ones16 = jnp.ones((16,), F32)
    zero16 = jnp.zeros((16,), F32)

    def dz(r, _):
        dacc[pl.ds(r * 16, 16)] = zero16
        return 0

    lax.fori_loop(0, NROW // 16, dz, 0)

    def chunk(g, carry):
        pltpu.sync_copy(ei.at[:, pl.ds(g * SCAN_B, SCAN_B)], ebuf)

        def group(j, carry):
            off, hoff = carry
            s16 = ebuf[0, pl.ds(j * 16, 16)]
            d16 = ebuf[1, pl.ds(j * 16, 16)]
            ok = (d16 >= base) & (d16 < base + RNG)
            key = jnp.where(ok, iota, 16)
            _, sv = plsc.sort_key_val(key, s16 * 16384 + d16)
            clbuf[pl.ds(off, 16)] = sv
            cntv = plsc.all_reduce_population_count(ok)
            off = off + cntv[0]
            loc = jnp.clip(d16 - base, 0, RNG)
            plsc.addupdate_scatter(dacc, [loc], jnp.where(ok, ones16, zero16))

            @pl.when(off >= SPILL)
            def _():
                pltpu.sync_copy(
                    clbuf.at[pl.ds(0, SPILL)],
                    clist.at[t, 0, pl.ds(pl.multiple_of(hoff, 128), SPILL)])
                tail = clbuf[pl.ds(SPILL, 16)]
                clbuf[pl.ds(0, 16)] = tail

            spilled = off >= SPILL
            off = jnp.where(spilled, off - SPILL, off)
            hoff = jnp.where(spilled, hoff + SPILL, hoff)
            return (off, hoff)

        return lax.fori_loop(0, SCAN_B // 16, group, carry)

    off, hoff = lax.fori_loop(0, NSCAN, chunk, (0, 0))

    padv = jnp.zeros((16,), I32) + (base + RNG)
    for i in range(8):
        clbuf[pl.ds(off + i * 16, 16)] = padv
    nsp = (off + 127) // 128

    def spill_fin(g, _):
        o = pl.multiple_of(g * 128, 128)
        pltpu.sync_copy(
            clbuf.at[pl.ds(o, 128)],
            clist.at[t, 0, pl.ds(pl.multiple_of(hoff, 128) + o, 128)])
        return 0

    lax.fori_loop(0, nsp, spill_fin, 0)
    cstg[pl.ds(0, 16)] = jnp.zeros((16,), I32) + (hoff + nsp * 128)
    pltpu.sync_copy(cstg, counts.at[t, 0])
    pltpu.sync_copy(dacc.at[pl.ds(0, RNG)], deg.at[pl.ds(base, RNG)])


def _read_count(counts, cntb, t):
    pltpu.sync_copy(counts.at[t, 0], cntb)
    return cntb[pl.ds(0, 16)][0]


def _unpack_chunk(gibuf, dlbuf, base):
    for j in range(CH // 16):
        v16 = gibuf[pl.ds(j * 16, 16)]
        d16 = v16 & 16383
        dlbuf[pl.ds(j * 16, 16)] = jnp.clip(d16 - base, 0, RNG)
        gibuf[pl.ds(j * 16, 16)] = lax.shift_right_logical(v16, 14)


def _drain(acc, out_slice, base):
    for k in range(RNG // 64):
        pltpu.sync_copy(acc.at[pl.ds(k * 64, 64)],
                        out_slice.at[pl.ds(base + k * 64, 64)])


def _zero_acc(acc, w):
    z = jnp.zeros((16,), F32)

    def za(r, _):
        for j in range(w // 16):
            acc[r, pl.ds(j * 16, 16)] = z
        return 0

    lax.fori_loop(0, NROW, za, 0)


NQ = 4
QB = CH // NQ


def _make_rowsum(w, nbuf, halfpipe=False):
    @functools.partial(
        pl.kernel,
        out_type=jax.ShapeDtypeStruct((NT * RNG, w), F32),
        mesh=plsc.VectorSubcoreMesh(**_MESH),
        compiler_params=_PARAMS,
        scratch_types=[
            pltpu.VMEM((NROW, w), F32),
            pltpu.VMEM((nbuf, 64 if halfpipe else CH, w), F32),
            pltpu.VMEM((nbuf, CH), I32),
            pltpu.VMEM((nbuf, CH), I32),
            pltpu.VMEM((16,), I32),
        ] + [pltpu.SemaphoreType.DMA] * nbuf,
    )
    def rowsum(table, clist, counts, out, acc, rowbuf, gibuf, dlbuf, cntb,
               *sems):
        c = lax.axis_index("c")
        s = lax.axis_index("s")
        t = _worker(c, s)
        base = t * RNG
        _zero_acc(acc, w)
        nch = _read_count(counts, cntb, t) // CH

        def prep_fire(g, b):
            pltpu.sync_copy(clist.at[t, 0, pl.ds(g * CH, CH)], gibuf.at[b])
            for j in range(CH // 16):
                v16 = gibuf[b, pl.ds(j * 16, 16)]
                d16 = v16 & 16383
                dlbuf[b, pl.ds(j * 16, 16)] = jnp.clip(d16 - base, 0, RNG)
                gibuf[b, pl.ds(j * 16, 16)] = lax.shift_right_logical(v16, 14)
            pltpu.async_copy(table.at[gibuf.at[b]], rowbuf.at[b], sems[b])

        def wait_rmw(b):
            pltpu.make_async_copy(table.at[gibuf.at[b]], rowbuf.at[b],
                                  sems[b]).wait()

            def rmw(j, _):
                dv = dlbuf[b, pl.ds(j * 16, 16)]
                for l in range(16):
                    rows = lax.broadcast(dv[l], (16,))
                    r = j * 16 + l
                    for col in range(w // 16):
                        plsc.addupdate_scatter(
                            acc, [rows, _COLI[col]],
                            rowbuf[b, r, pl.ds(col * 16, 16)])
                return 0

            lax.fori_loop(0, CH // 16, rmw, 0)

        if halfpipe:
            def load_unpack(g, cp):
                pltpu.sync_copy(clist.at[t, 0, pl.ds(g * CH, CH)],
                                gibuf.at[cp])
                for j in range(CH // 16):
                    v16 = gibuf[cp, pl.ds(j * 16, 16)]
                    d16 = v16 & 16383
                    dlbuf[cp, pl.ds(j * 16, 16)] = jnp.clip(d16 - base, 0, RNG)
                    gibuf[cp, pl.ds(j * 16, 16)] = lax.shift_right_logical(
                        v16, 14)

            def fire(cp, hf, b):
                pltpu.async_copy(
                    table.at[gibuf.at[cp, pl.ds(hf * 64, 64)]],
                    rowbuf.at[b], sems[b])

            def wait_rmw2(cp, hfoff, b):
                pltpu.make_async_copy(table.at[gibuf.at[0, pl.ds(0, 64)]],
                                      rowbuf.at[b], sems[b]).wait()

                def rmw(j, _):
                    dv = dlbuf[cp, pl.ds(hfoff + j * 16, 16)]
                    for l in range(16):
                        rows = lax.broadcast(dv[l], (16,))
                        r = j * 16 + l
                        for col in range(w // 16):
                            plsc.addupdate_scatter(
                                acc, [rows, _COLI[col]],
                                rowbuf[b, r, pl.ds(col * 16, 16)])
                    return 0

                lax.fori_loop(0, 4, rmw, 0)

            @pl.when(nch > 0)
            def _():
                load_unpack(0, 0)
                fire(0, 0, 0)

            def body(i, _):
                cp = i & 1
                fire(cp, 1, 1)

                @pl.when(i + 1 < nch)
                def _():
                    load_unpack(i + 1, (i + 1) & 1)

                wait_rmw2(cp, 0, 0)

                @pl.when(i + 1 < nch)
                def _():
                    fire((i + 1) & 1, 0, 0)

                wait_rmw2(cp, 64, 1)
                return 0

            lax.fori_loop(0, nch, body, 0)
        elif nbuf == 1:
            def chunk(g, _):
                prep_fire(g, 0)
                wait_rmw(0)
                return 0

            lax.fori_loop(0, nch, chunk, 0)
        else:
            @pl.when(nch > 0)
            def _():
                prep_fire(0, 0)

            def pairbody(i, _):
                k1 = 2 * i + 1

                @pl.when(k1 < nch)
                def _():
                    prep_fire(k1, 1)

                wait_rmw(0)

                @pl.when(k1 < nch)
                def _():
                    @pl.when(k1 + 1 < nch)
                    def _():
                        prep_fire(k1 + 1, 0)

                    wait_rmw(1)

                return 0

            lax.fori_loop(0, (nch + 1) // 2, pairbody, 0)
        _drain(acc, out, base)

    return rowsum


_sc_rowsum256 = _make_rowsum(256, 2, halfpipe=True)
_sc_rowsum128 = _make_rowsum(128, 2)


@functools.partial(
    pl.kernel,
    out_type=(
        jax.ShapeDtypeStruct((4, NT * RNG, 128), F32),
        jax.ShapeDtypeStruct((4, NT * RNG * 16), F32),
    ),
    mesh=plsc.VectorSubcoreMesh(**_MESH),
    compiler_params=_PARAMS,
    scratch_types=[
        pltpu.VMEM((NROW, 128), F32),
        pltpu.VMEM((NROW * 16,), F32),
        pltpu.VMEM((2 * NN,), F32),
        pltpu.VMEM((2, CH, 128), F32),
        pltpu.VMEM((2, CH), F32),
        pltpu.VMEM((2, CH), I32),
        pltpu.VMEM((2, CH), I32),
        pltpu.VMEM((16,), I32),
    ] + [pltpu.SemaphoreType.DMA] * 2,
)
def _sc_gat(hw2, aa4, clist, counts, num_out, den_out,
            nacc, dacc, aav, rowbuf, wrow, gibuf, dlbuf, cntb, *sems):
    c = lax.axis_index("c")
    s = lax.axis_index("s")
    t = _worker(c, s)
    base = t * RNG
    lane0 = jnp.where(lax.broadcasted_iota(I32, (16,), 0) == 0, 1.0, 0.0)
    nch = _read_count(counts, cntb, t) // CH

    def head(h, _):
        pltpu.sync_copy(aa4.at[h, 0], aav)
        _zero_acc(nacc, 128)

        def dz(r, _):
            dacc[pl.ds(r * 16, 16)] = jnp.zeros((16,), F32)
            return 0

        lax.fori_loop(0, NROW, dz, 0)

        def prep_fire(g, b):
            pltpu.sync_copy(clist.at[t, 0, pl.ds(g * CH, CH)], gibuf.at[b])
            for j in range(CH // 16):
                v16 = gibuf[b, pl.ds(j * 16, 16)]
                d16 = v16 & 16383
                s16 = lax.shift_right_logical(v16, 14)
                a_s = plsc.load_gather(aav, [s16 * 2])
                a_d = plsc.load_gather(aav, [jnp.minimum(d16, NN - 1) * 2 + 1])
                e = a_s + a_d
                e = jnp.maximum(e, 0.2 * e)
                wrow[b, pl.ds(j * 16, 16)] = jnp.exp(e)
                dlbuf[b, pl.ds(j * 16, 16)] = jnp.clip(d16 - base, 0, RNG)
                gibuf[b, pl.ds(j * 16, 16)] = s16 * 4 + h
            pltpu.async_copy(hw2.at[gibuf.at[b]], rowbuf.at[b], sems[b])

        def wait_rmw(b):
            pltpu.make_async_copy(hw2.at[gibuf.at[b]], rowbuf.at[b],
                                  sems[b]).wait()

            def rmw(j, _):
                dv = dlbuf[b, pl.ds(j * 16, 16)]
                wv = wrow[b, pl.ds(j * 16, 16)]
                for l in range(16):
                    rows = lax.broadcast(dv[l], (16,))
                    r = j * 16 + l
                    wl = wv[l]
                    for col in range(8):
                        plsc.addupdate_scatter(
                            nacc, [rows, _COLI[col]],
                            rowbuf[b, r, pl.ds(col * 16, 16)] * wl)
                    plsc.addupdate_scatter(dacc, [rows * 16 + _COLI[0]],
                                           lane0 * wl)
                return 0

            lax.fori_loop(0, CH // 16, rmw, 0)

        @pl.when(nch > 0)
        def _():
            prep_fire(0, 0)

        def pairbody(i, _):
            k1 = 2 * i + 1

            @pl.when(k1 < nch)
            def _():
                prep_fire(k1, 1)

            wait_rmw(0)

            @pl.when(k1 < nch)
            def _():
                @pl.when(k1 + 1 < nch)
                def _():
                    prep_fire(k1 + 1, 0)

                wait_rmw(1)

            return 0

        lax.fori_loop(0, (nch + 1) // 2, pairbody, 0)
        _drain(nacc, num_out.at[h], base)
        pltpu.sync_copy(dacc.at[pl.ds(0, RNG * 16)],
                        den_out.at[h, pl.ds(base * 16, RNG * 16)])
        return 0

    lax.fori_loop(0, 4, head, 0)


RB = 1000
_GRID = NN // RB


def _ln(h, g, b):
    mu = jnp.mean(h, axis=-1, keepdims=True)
    var = jnp.mean((h - mu) ** 2, axis=-1, keepdims=True)
    return (h - mu) * lax.rsqrt(var + 1e-5) * g + b


def _rows(c):
    return pl.BlockSpec((RB, c), lambda i: (i, 0))


def _full(*shape):
    nd = len(shape)
    return pl.BlockSpec(shape, lambda i: (0,) * nd)


def _tc1_body(x, W, degp, out):
    dinv = lax.rsqrt(degp[...] + 1.0)
    out[...] = jnp.dot(x[...], W[...], preferred_element_type=F32) * dinv


def _tc1(x, W, degp):
    return pl.pallas_call(
        _tc1_body,
        grid=(_GRID,),
        in_specs=[_rows(256), _full(256, 256), _rows(1)],
        out_specs=_rows(256),
        out_shape=jax.ShapeDtypeStruct((NN, 256), F32),
    )(x, W, degp)


def _tc2_body(acc1, hw1p, degp, b1, g1, be1, Wg, asr, ads, Wr2, br2,
              hw2_o, aa_o, res2_o):
    dinv = lax.rsqrt(degp[...] + 1.0)
    h1 = jax.nn.relu(_ln(dinv * (acc1[...] + hw1p[...]) + b1[...],
                         g1[...], be1[...]))
    hw2 = jnp.dot(h1, Wg[...], preferred_element_type=F32)
    hw2_o[...] = hw2
    heads = []
    for h in range(4):
        blk = hw2[:, h * 128:(h + 1) * 128]
        a_s = jnp.sum(blk * asr[...][h][None, :], axis=1, keepdims=True)
        a_d = jnp.sum(blk * ads[...][h][None, :], axis=1, keepdims=True)
        heads.append(jnp.concatenate([a_s, a_d], axis=1)[None])
    aa_o[...] = jnp.concatenate(heads, axis=0)
    res2_o[...] = jnp.dot(h1, Wr2[...], preferred_element_type=F32) + br2[...]


def _tc2(acc1, hw1p, degp, p):
    return pl.pallas_call(
        _tc2_body,
        grid=(_GRID,),
        in_specs=[_rows(256), _rows(256), _rows(1), _full(256), _full(256),
                  _full(256), _full(256, 512), _full(4, 128), _full(4, 128),
                  _full(256, 128), _full(128)],
        out_specs=[_rows(512), pl.BlockSpec((4, RB, 2), lambda i: (0, i, 0)),
                   _rows(128)],
        out_shape=[jax.ShapeDtypeStruct((NN, 512), F32),
                   jax.ShapeDtypeStruct((4, NN, 2), F32),
                   jax.ShapeDtypeStruct((NN, 128), F32)],
    )(acc1, hw1p, degp, p["gcn1_b"], p["ln1_g"], p["ln1_b"], p["gat2_W"],
      p["gat2_att_src"], p["gat2_att_dst"], p["res2_W"], p["res2_b"])


def _tc3_body(num, den, hw2, aa, res2, bg, g2, be2, Wc, bc,
              h2_o, part_o):
    aam = aa[...]
    hw2m = hw2[...]
    gat = jnp.zeros_like(res2[...])
    for h in range(4):
        esl = aam[h, :, 0:1] + aam[h, :, 1:2]
        wsl = jnp.exp(jnp.maximum(esl, 0.2 * esl))
        nh = num[...][h] + wsl * hw2m[:, h * 128:(h + 1) * 128]
        sh = den[...][h][:, 0:1] + wsl
        gat = gat + nh / (sh + 1e-16)
    gat = gat * 0.25 + bg[...]
    h2 = jax.nn.relu(_ln(gat + res2[...], g2[...], be2[...]))
    h2_o[...] = h2
    part_o[...] = jnp.dot(h2, Wc[...], preferred_element_type=F32) + bc[...]


def _tc3(num, den, hw2, aa, res2, p):
    return pl.pallas_call(
        _tc3_body,
        grid=(_GRID,),
        in_specs=[pl.BlockSpec((4, RB, 128), lambda i: (0, i, 0)),
                  pl.BlockSpec((4, RB, 16), lambda i: (0, i, 0)),
                  _rows(512), pl.BlockSpec((4, RB, 2), lambda i: (0, i, 0)),
                  _rows(128), _full(128), _full(128),
                  _full(128), _full(128, 64), _full(64)],
        out_specs=[_rows(128), _rows(64)],
        out_shape=[jax.ShapeDtypeStruct((NN, 128), F32),
                   jax.ShapeDtypeStruct((NN, 64), F32)],
    )(num, den, hw2, aa, res2, p["gat2_b"], p["ln2_g"], p["ln2_b"],
      p["sage3_Wr"] + p["res3_W"],
      p["res3_b"] + p["sage3_bl"])


def _tc4_body(accS, degp, part, Wl, g3, be3, W4, h4p_o):
    deg = degp[...]
    mean = accS[...] / jnp.maximum(deg, 1.0)
    h3 = jax.nn.relu(_ln(jnp.dot(mean, Wl[...], preferred_element_type=F32)
                         + part[...], g3[...], be3[...]))
    h4p = jnp.dot(h3, W4[...], preferred_element_type=F32) * lax.rsqrt(deg + 1.0)
    h4p_o[...] = jnp.concatenate([h4p, jnp.zeros((RB, 64), F32)], axis=1)


def _tc4(accS, degp, part, p):
    return pl.pallas_call(
        _tc4_body,
        grid=(_GRID,),
        in_specs=[_rows(128), _rows(1), _rows(64), _full(128, 64), _full(64),
                  _full(64), _full(64, 64)],
        out_specs=_rows(128),
        out_shape=jax.ShapeDtypeStruct((NN, 128), F32),
    )(accS, degp, part, p["sage3_Wl"], p["ln3_g"], p["ln3_b"], p["gc4_W"])


def _tc5_body(acc4, h4p, degp, batch3, b4, g4, be4, Wro, bro, pool_o):
    i = pl.program_id(0)
    dinv = lax.rsqrt(degp[...] + 1.0)
    h4 = jax.nn.relu(_ln(dinv * (acc4[...] + h4p[...]) + b4[...],
                         g4[...], be4[...]))
    gate = jax.nn.sigmoid(jnp.dot(h4, Wro[...], preferred_element_type=F32) + bro[...])
    gated = h4 * gate
    b = batch3[...][0, 0, :]
    P = (lax.broadcasted_iota(I32, (64, RB), 0) == b[None, :]).astype(F32)
    rhs = jnp.concatenate([gated, gate, jnp.zeros((RB, 63), F32)], axis=1)
    blk = jnp.dot(P, rhs, preferred_element_type=F32)

    @pl.when(i == 0)
    def _():
        pool_o[...] = jnp.zeros_like(pool_o)

    pool_o[...] += blk


def _tc5(acc4, h4p, degp, batch3, p):
    return pl.pallas_call(
        _tc5_body,
        grid=(_GRID,),
        in_specs=[_rows(64), _rows(64), _rows(1),
                  pl.BlockSpec((1, 1, RB), lambda i: (i, 0, 0)),
                  _full(64), _full(64), _full(64), _full(64, 1), _full(1)],
        out_specs=_full(64, 128),
        out_shape=jax.ShapeDtypeStruct((64, 128), F32),
    )(acc4, h4p, degp, batch3, p["gc4_b"], p["ln4_g"], p["ln4_b"],
      p["ro_W"], p["ro_b"])


def _tc6_body(pool, link3, W1, b1, W2, b2, W3, b3, out_o):
    pm = pool[...]
    emb = pm[:, :64] / (pm[:, 64:65] + 1e-8)
    lk = link3[...][:, 0, :]
    g_iota = lax.broadcasted_iota(I32, (4096, 64), 1)
    o1 = (lk[0][:, None] == g_iota).astype(F32)
    o2 = (lk[1][:, None] == g_iota).astype(F32)
    f = jnp.concatenate([
        jnp.dot(o1, emb, preferred_element_type=F32),
        jnp.dot(o2, emb, preferred_element_type=F32)], axis=1)
    f = jax.nn.relu(jnp.dot(f, W1[...], preferred_element_type=F32) + b1[...])
    f = jax.nn.relu(jnp.dot(f, W2[...], preferred_element_type=F32) + b2[...])
    out_o[...] = jax.nn.sigmoid(jnp.dot(f, W3[...], preferred_element_type=F32) + b3[...])


def _tc6(pool, link3, p):
    return pl.pallas_call(
        _tc6_body,
        in_specs=[pl.BlockSpec((64, 128), lambda: (0, 0)),
                  pl.BlockSpec((2, 1, 4096), lambda: (0, 0, 0)),
                  pl.BlockSpec((128, 64), lambda: (0, 0)),
                  pl.BlockSpec((64,), lambda: (0,)),
                  pl.BlockSpec((64, 64), lambda: (0, 0)),
                  pl.BlockSpec((64,), lambda: (0,)),
                  pl.BlockSpec((64, 1), lambda: (0, 0)),
                  pl.BlockSpec((1,), lambda: (0,)),
                  ],
        out_specs=pl.BlockSpec((4096, 1), lambda: (0, 0)),
        out_shape=jax.ShapeDtypeStruct((4096, 1), F32),
    )(pool, link3, p["mlp1_W"], p["mlp1_b"], p["mlp2_W"], p["mlp2_b"],
      p["mlp3_W"], p["mlp3_b"])


def kernel(x, edge_index, batch, link_indices, params):
    p = params

    clist, counts, deg = _sc_scan(edge_index)
    degp = deg[:NN, None]
    hw1p = _tc1(x, p["gcn1_W"], degp)
    acc1 = _sc_rowsum256(hw1p, clist, counts)[:NN]
    hw2, aa, res2 = _tc2(acc1, hw1p, degp, p)
    num, den = _sc_gat(hw2.reshape(NN * 4, 128),
                       aa.reshape(4, 1, 2 * NN), clist, counts)
    den = den.reshape(4, NT * RNG, 16)
    h2, part = _tc3(num[:, :NN], den[:, :NN], hw2, aa, res2, p)
    accS = _sc_rowsum128(h2, clist, counts)[:NN]
    h4p = _tc4(accS, degp, part, p)
    acc4 = _sc_rowsum128(h4p, clist, counts)[:NN, :64]
    pool = _tc5(acc4, h4p[:, :64], degp, batch.reshape(_GRID, 1, RB), p)
    out = _tc6(pool, link_indices.reshape(2, 1, 4096), p)
    return out[:, 0]


# speedup vs baseline: 1.4106x; 1.0158x over previous
import functools
import jax
import jax.numpy as jnp
from jax import lax
from jax.experimental import pallas as pl
from jax.experimental.pallas import tpu as pltpu
from jax.experimental.pallas import tpu_sc as plsc

F32 = jnp.float32
I32 = jnp.int32

NN = 10000
EE = 160000
NT = 32
RNG = 320
NROW = 328
CH = 128
SCAN_B = 3200
NSCAN = EE // SCAN_B
assert NSCAN * SCAN_B == EE
CLCAP = 10368
SPILL = 10240
CLW = EE + 128

_MESH = dict(core_axis_name="c", subcore_axis_name="s",
             num_cores=2, num_subcores=16)
_PARAMS = pltpu.CompilerParams(needs_layout_passes=False)


def _coli():
    io = lax.broadcasted_iota(I32, (16,), 0)
    return [io + c * 16 for c in range(16)]


class _ColI:
    def __getitem__(self, c):
        io = lax.broadcasted_iota(I32, (16,), 0)
        return io + c * 16


_COLI = _ColI()


def _worker(c, s):
    return c * 16 + s


@functools.partial(
    pl.kernel,
    out_type=(
        jax.ShapeDtypeStruct((NT, 1, CLW), I32),
        jax.ShapeDtypeStruct((NT, 1, 16), I32),
        jax.ShapeDtypeStruct((NT * RNG,), F32),
    ),
    mesh=plsc.VectorSubcoreMesh(**_MESH),
    compiler_params=_PARAMS,
    scratch_types=[
        pltpu.VMEM((2, 2, SCAN_B), I32),
        pltpu.VMEM((CLCAP,), I32),
        pltpu.VMEM((NROW,), F32),
        pltpu.VMEM((16,), I32),
        pltpu.SemaphoreType.DMA,
        pltpu.SemaphoreType.DMA,
    ],
)
def _sc_scan(ei, clist, counts, deg, ebuf, clbuf, dacc, cstg, esem0, esem1):
    c = lax.axis_index("c")
    s = lax.axis_index("s")
    t = _worker(c, s)
    base = t * RNG
    iota = lax.broadcasted_iota(I32, (16,), 0)
    ones16 = jnp.ones((16,), F32)
    zero16 = jnp.zeros((16,), F32)

    def dz(r, _):
        dacc[pl.ds(r * 16, 16)] = zero16
        return 0

    lax.fori_loop(0, NROW // 16, dz, 0)

    sems = (esem0, esem1)

    def fire_load(g, b):
        pltpu.async_copy(ei.at[:, pl.ds(g * SCAN_B, SCAN_B)], ebuf.at[b],
                         sems[b])

    def wait_load(b):
        pltpu.make_async_copy(ei.at[:, pl.ds(0, SCAN_B)], ebuf.at[b],
                              sems[b]).wait()

    def process(b, carry):
        def group(j, carry):
            off, hoff = carry
            s16 = ebuf[b, 0, pl.ds(j * 16, 16)]
            d16 = ebuf[b, 1, pl.ds(j * 16, 16)]
            ok = (d16 >= base) & (d16 < base + RNG)
            key = jnp.where(ok, iota, 16)
            _, sv = plsc.sort_key_val(key, s16 * 16384 + d16)
            clbuf[pl.ds(off, 16)] = sv
            cntv = plsc.all_reduce_population_count(ok)
            off = off + cntv[0]
            loc = jnp.clip(d16 - base, 0, RNG)
            plsc.addupdate_scatter(dacc, [loc], jnp.where(ok, ones16, zero16))

            @pl.when(off >= SPILL)
            def _():
                pltpu.sync_copy(
                    clbuf.at[pl.ds(0, SPILL)],
                    clist.at[t, 0, pl.ds(pl.multiple_of(hoff, 128), SPILL)])
                tail = clbuf[pl.ds(SPILL, 16)]
                clbuf[pl.ds(0, 16)] = tail

            spilled = off >= SPILL
            off = jnp.where(spilled, off - SPILL, off)
            hoff = jnp.where(spilled, hoff + SPILL, hoff)
            return (off, hoff)

        return lax.fori_loop(0, SCAN_B // 16, group, carry)

    fire_load(0, 0)

    def chunkpair(i, carry):
        fire_load(2 * i + 1, 1)
        wait_load(0)
        carry = process(0, carry)

        @pl.when(i + 1 < NSCAN // 2)
        def _():
            fire_load(2 * i + 2, 0)

        wait_load(1)
        return process(1, carry)

    off, hoff = lax.fori_loop(0, NSCAN // 2, chunkpair, (0, 0))

    padv = jnp.zeros((16,), I32) + (base + RNG)
    for i in range(8):
        clbuf[pl.ds(off + i * 16, 16)] = padv
    nsp = (off + 127) // 128

    def spill_fin(g, _):
        o = pl.multiple_of(g * 128, 128)
        pltpu.sync_copy(
            clbuf.at[pl.ds(o, 128)],
            clist.at[t, 0, pl.ds(pl.multiple_of(hoff, 128) + o, 128)])
        return 0

    lax.fori_loop(0, nsp, spill_fin, 0)
    cstg[pl.ds(0, 16)] = jnp.zeros((16,), I32) + (hoff + nsp * 128)
    pltpu.sync_copy(cstg, counts.at[t, 0])
    pltpu.sync_copy(dacc.at[pl.ds(0, RNG)], deg.at[pl.ds(base, RNG)])


def _read_count(counts, cntb, t):
    pltpu.sync_copy(counts.at[t, 0], cntb)
    return cntb[pl.ds(0, 16)][0]


def _unpack_chunk(gibuf, dlbuf, base):
    for j in range(CH // 16):
        v16 = gibuf[pl.ds(j * 16, 16)]
        d16 = v16 & 16383
        dlbuf[pl.ds(j * 16, 16)] = jnp.clip(d16 - base, 0, RNG)
        gibuf[pl.ds(j * 16, 16)] = lax.shift_right_logical(v16, 14)


def _drain(acc, out_slice, base):
    for k in range(RNG // 64):
        pltpu.sync_copy(acc.at[pl.ds(k * 64, 64)],
                        out_slice.at[pl.ds(base + k * 64, 64)])


def _zero_acc(acc, w):
    z = jnp.zeros((16,), F32)

    def za(r, _):
        for j in range(w // 16):
            acc[r, pl.ds(j * 16, 16)] = z
        return 0

    lax.fori_loop(0, NROW, za, 0)


NQ = 4
QB = CH // NQ


def _make_rowsum(w, nbuf, halfpipe=False):
    @functools.partial(
        pl.kernel,
        out_type=jax.ShapeDtypeStruct((NT * RNG, w), F32),
        mesh=plsc.VectorSubcoreMesh(**_MESH),
        compiler_params=_PARAMS,
        scratch_types=[
            pltpu.VMEM((NROW, w), F32),
            pltpu.VMEM((nbuf, 64 if halfpipe else CH, w), F32),
            pltpu.VMEM((nbuf, CH), I32),
            pltpu.VMEM((nbuf, CH), I32),
            pltpu.VMEM((16,), I32),
        ] + [pltpu.SemaphoreType.DMA] * nbuf,
    )
    def rowsum(table, clist, counts, out, acc, rowbuf, gibuf, dlbuf, cntb,
               *sems):
        c = lax.axis_index("c")
        s = lax.axis_index("s")
        t = _worker(c, s)
        base = t * RNG
        _zero_acc(acc, w)
        nch = _read_count(counts, cntb, t) // CH

        def prep_fire(g, b):
            pltpu.sync_copy(clist.at[t, 0, pl.ds(g * CH, CH)], gibuf.at[b])
            for j in range(CH // 16):
                v16 = gibuf[b, pl.ds(j * 16, 16)]
                d16 = v16 & 16383
                dlbuf[b, pl.ds(j * 16, 16)] = jnp.clip(d16 - base, 0, RNG)
                gibuf[b, pl.ds(j * 16, 16)] = lax.shift_right_logical(v16, 14)
            pltpu.async_copy(table.at[gibuf.at[b]], rowbuf.at[b], sems[b])

        def wait_rmw(b):
            pltpu.make_async_copy(table.at[gibuf.at[b]], rowbuf.at[b],
                                  sems[b]).wait()

            def rmw(j, _):
                dv = dlbuf[b, pl.ds(j * 16, 16)]
                for l in range(16):
                    rows = lax.broadcast(dv[l], (16,))
                    r = j * 16 + l
                    for col in range(w // 16):
                        plsc.addupdate_scatter(
                            acc, [rows, _COLI[col]],
                            rowbuf[b, r, pl.ds(col * 16, 16)])
                return 0

            lax.fori_loop(0, CH // 16, rmw, 0)

        if halfpipe:
            def load_unpack(g, cp):
                pltpu.sync_copy(clist.at[t, 0, pl.ds(g * CH, CH)],
                                gibuf.at[cp])
                for j in range(CH // 16):
                    v16 = gibuf[cp, pl.ds(j * 16, 16)]
                    d16 = v16 & 16383
                    dlbuf[cp, pl.ds(j * 16, 16)] = jnp.clip(d16 - base, 0, RNG)
                    gibuf[cp, pl.ds(j * 16, 16)] = lax.shift_right_logical(
                        v16, 14)

            def fire(cp, hf, b):
                pltpu.async_copy(
                    table.at[gibuf.at[cp, pl.ds(hf * 64, 64)]],
                    rowbuf.at[b], sems[b])

            def wait_rmw2(cp, hfoff, b):
                pltpu.make_async_copy(table.at[gibuf.at[0, pl.ds(0, 64)]],
                                      rowbuf.at[b], sems[b]).wait()

                def rmw(j, _):
                    dv = dlbuf[cp, pl.ds(hfoff + j * 16, 16)]
                    for l in range(16):
                        rows = lax.broadcast(dv[l], (16,))
                        r = j * 16 + l
                        for col in range(w // 16):
                            plsc.addupdate_scatter(
                                acc, [rows, _COLI[col]],
                                rowbuf[b, r, pl.ds(col * 16, 16)])
                    return 0

                lax.fori_loop(0, 4, rmw, 0)

            @pl.when(nch > 0)
            def _():
                load_unpack(0, 0)
                fire(0, 0, 0)

            def body(i, _):
                cp = i & 1
                fire(cp, 1, 1)

                @pl.when(i + 1 < nch)
                def _():
                    load_unpack(i + 1, (i + 1) & 1)

                wait_rmw2(cp, 0, 0)

                @pl.when(i + 1 < nch)
                def _():
                    fire((i + 1) & 1, 0, 0)

                wait_rmw2(cp, 64, 1)
                return 0

            lax.fori_loop(0, nch, body, 0)
        elif nbuf == 1:
            def chunk(g, _):
                prep_fire(g, 0)
                wait_rmw(0)
                return 0

            lax.fori_loop(0, nch, chunk, 0)
        else:
            @pl.when(nch > 0)
            def _():
                prep_fire(0, 0)

            def pairbody(i, _):
                k1 = 2 * i + 1

                @pl.when(k1 < nch)
                def _():
                    prep_fire(k1, 1)

                wait_rmw(0)

                @pl.when(k1 < nch)
                def _():
                    @pl.when(k1 + 1 < nch)
                    def _():
                        prep_fire(k1 + 1, 0)

                    wait_rmw(1)

                return 0

            lax.fori_loop(0, (nch + 1) // 2, pairbody, 0)
        _drain(acc, out, base)

    return rowsum


_sc_rowsum256 = _make_rowsum(256, 2, halfpipe=True)
_sc_rowsum128 = _make_rowsum(128, 2)


@functools.partial(
    pl.kernel,
    out_type=(
        jax.ShapeDtypeStruct((4, NT * RNG, 128), F32),
        jax.ShapeDtypeStruct((4, NT * RNG * 16), F32),
    ),
    mesh=plsc.VectorSubcoreMesh(**_MESH),
    compiler_params=_PARAMS,
    scratch_types=[
        pltpu.VMEM((NROW, 128), F32),
        pltpu.VMEM((NROW * 16,), F32),
        pltpu.VMEM((2 * NN,), F32),
        pltpu.VMEM((2, CH, 128), F32),
        pltpu.VMEM((2, CH), F32),
        pltpu.VMEM((2, CH), I32),
        pltpu.VMEM((2, CH), I32),
        pltpu.VMEM((16,), I32),
    ] + [pltpu.SemaphoreType.DMA] * 2,
)
def _sc_gat(hw2, aa4, clist, counts, num_out, den_out,
            nacc, dacc, aav, rowbuf, wrow, gibuf, dlbuf, cntb, *sems):
    c = lax.axis_index("c")
    s = lax.axis_index("s")
    t = _worker(c, s)
    base = t * RNG
    lane0 = jnp.where(lax.broadcasted_iota(I32, (16,), 0) == 0, 1.0, 0.0)
    nch = _read_count(counts, cntb, t) // CH

    def head(h, _):
        pltpu.sync_copy(aa4.at[h, 0], aav)
        _zero_acc(nacc, 128)

        def dz(r, _):
            dacc[pl.ds(r * 16, 16)] = jnp.zeros((16,), F32)
            return 0

        lax.fori_loop(0, NROW, dz, 0)

        def prep_fire(g, b):
            pltpu.sync_copy(clist.at[t, 0, pl.ds(g * CH, CH)], gibuf.at[b])
            for j in range(CH // 16):
                v16 = gibuf[b, pl.ds(j * 16, 16)]
                d16 = v16 & 16383
                s16 = lax.shift_right_logical(v16, 14)
                a_s = plsc.load_gather(aav, [s16 * 2])
                a_d = plsc.load_gather(aav, [jnp.minimum(d16, NN - 1) * 2 + 1])
                e = a_s + a_d
                e = jnp.maximum(e, 0.2 * e)
                wrow[b, pl.ds(j * 16, 16)] = jnp.exp(e)
                dlbuf[b, pl.ds(j * 16, 16)] = jnp.clip(d16 - base, 0, RNG)
                gibuf[b, pl.ds(j * 16, 16)] = s16 * 4 + h
            pltpu.async_copy(hw2.at[gibuf.at[b]], rowbuf.at[b], sems[b])

        def wait_rmw(b):
            pltpu.make_async_copy(hw2.at[gibuf.at[b]], rowbuf.at[b],
                                  sems[b]).wait()

            def rmw(j, _):
                dv = dlbuf[b, pl.ds(j * 16, 16)]
                wv = wrow[b, pl.ds(j * 16, 16)]
                for l in range(16):
                    rows = lax.broadcast(dv[l], (16,))
                    r = j * 16 + l
                    wl = wv[l]
                    for col in range(8):
                        plsc.addupdate_scatter(
                            nacc, [rows, _COLI[col]],
                            rowbuf[b, r, pl.ds(col * 16, 16)] * wl)
                    plsc.addupdate_scatter(dacc, [rows * 16 + _COLI[0]],
                                           lane0 * wl)
                return 0

            lax.fori_loop(0, CH // 16, rmw, 0)

        @pl.when(nch > 0)
        def _():
            prep_fire(0, 0)

        def pairbody(i, _):
            k1 = 2 * i + 1

            @pl.when(k1 < nch)
            def _():
                prep_fire(k1, 1)

            wait_rmw(0)

            @pl.when(k1 < nch)
            def _():
                @pl.when(k1 + 1 < nch)
                def _():
                    prep_fire(k1 + 1, 0)

                wait_rmw(1)

            return 0

        lax.fori_loop(0, (nch + 1) // 2, pairbody, 0)
        _drain(nacc, num_out.at[h], base)
        pltpu.sync_copy(dacc.at[pl.ds(0, RNG * 16)],
                        den_out.at[h, pl.ds(base * 16, RNG * 16)])
        return 0

    lax.fori_loop(0, 4, head, 0)


RB = 1000
_GRID = NN // RB


def _ln(h, g, b):
    mu = jnp.mean(h, axis=-1, keepdims=True)
    var = jnp.mean((h - mu) ** 2, axis=-1, keepdims=True)
    return (h - mu) * lax.rsqrt(var + 1e-5) * g + b


def _rows(c):
    return pl.BlockSpec((RB, c), lambda i: (i, 0))


def _full(*shape):
    nd = len(shape)
    return pl.BlockSpec(shape, lambda i: (0,) * nd)


def _tc1_body(x, W, degp, out):
    dinv = lax.rsqrt(degp[...] + 1.0)
    out[...] = jnp.dot(x[...], W[...], preferred_element_type=F32) * dinv


def _tc1(x, W, degp):
    return pl.pallas_call(
        _tc1_body,
        grid=(_GRID,),
        in_specs=[_rows(256), _full(256, 256), _rows(1)],
        out_specs=_rows(256),
        out_shape=jax.ShapeDtypeStruct((NN, 256), F32),
    )(x, W, degp)


def _tc2_body(acc1, hw1p, degp, b1, g1, be1, Wg, asr, ads, Wr2, br2,
              hw2_o, aa_o, res2_o):
    dinv = lax.rsqrt(degp[...] + 1.0)
    h1 = jax.nn.relu(_ln(dinv * (acc1[...] + hw1p[...]) + b1[...],
                         g1[...], be1[...]))
    hw2 = jnp.dot(h1, Wg[...], preferred_element_type=F32)
    hw2_o[...] = hw2
    heads = []
    for h in range(4):
        blk = hw2[:, h * 128:(h + 1) * 128]
        a_s = jnp.sum(blk * asr[...][h][None, :], axis=1, keepdims=True)
        a_d = jnp.sum(blk * ads[...][h][None, :], axis=1, keepdims=True)
        heads.append(jnp.concatenate([a_s, a_d], axis=1)[None])
    aa_o[...] = jnp.concatenate(heads, axis=0)
    res2_o[...] = jnp.dot(h1, Wr2[...], preferred_element_type=F32) + br2[...]


def _tc2(acc1, hw1p, degp, p):
    return pl.pallas_call(
        _tc2_body,
        grid=(_GRID,),
        in_specs=[_rows(256), _rows(256), _rows(1), _full(256), _full(256),
                  _full(256), _full(256, 512), _full(4, 128), _full(4, 128),
                  _full(256, 128), _full(128)],
        out_specs=[_rows(512), pl.BlockSpec((4, RB, 2), lambda i: (0, i, 0)),
                   _rows(128)],
        out_shape=[jax.ShapeDtypeStruct((NN, 512), F32),
                   jax.ShapeDtypeStruct((4, NN, 2), F32),
                   jax.ShapeDtypeStruct((NN, 128), F32)],
    )(acc1, hw1p, degp, p["gcn1_b"], p["ln1_g"], p["ln1_b"], p["gat2_W"],
      p["gat2_att_src"], p["gat2_att_dst"], p["res2_W"], p["res2_b"])


def _tc3_body(num, den, hw2, aa, res2, bg, g2, be2, Wc, bc,
              h2_o, part_o):
    aam = aa[...]
    hw2m = hw2[...]
    gat = jnp.zeros_like(res2[...])
    for h in range(4):
        esl = aam[h, :, 0:1] + aam[h, :, 1:2]
        wsl = jnp.exp(jnp.maximum(esl, 0.2 * esl))
        nh = num[...][h] + wsl * hw2m[:, h * 128:(h + 1) * 128]
        sh = den[...][h][:, 0:1] + wsl
        gat = gat + nh / (sh + 1e-16)
    gat = gat * 0.25 + bg[...]
    h2 = jax.nn.relu(_ln(gat + res2[...], g2[...], be2[...]))
    h2_o[...] = h2
    part_o[...] = jnp.dot(h2, Wc[...], preferred_element_type=F32) + bc[...]


def _tc3(num, den, hw2, aa, res2, p):
    return pl.pallas_call(
        _tc3_body,
        grid=(_GRID,),
        in_specs=[pl.BlockSpec((4, RB, 128), lambda i: (0, i, 0)),
                  pl.BlockSpec((4, RB, 16), lambda i: (0, i, 0)),
                  _rows(512), pl.BlockSpec((4, RB, 2), lambda i: (0, i, 0)),
                  _rows(128), _full(128), _full(128),
                  _full(128), _full(128, 64), _full(64)],
        out_specs=[_rows(128), _rows(64)],
        out_shape=[jax.ShapeDtypeStruct((NN, 128), F32),
                   jax.ShapeDtypeStruct((NN, 64), F32)],
    )(num, den, hw2, aa, res2, p["gat2_b"], p["ln2_g"], p["ln2_b"],
      p["sage3_Wr"] + p["res3_W"],
      p["res3_b"] + p["sage3_bl"])


def _tc4_body(accS, degp, part, Wl, g3, be3, W4, h4p_o):
    deg = degp[...]
    mean = accS[...] / jnp.maximum(deg, 1.0)
    h3 = jax.nn.relu(_ln(jnp.dot(mean, Wl[...], preferred_element_type=F32)
                         + part[...], g3[...], be3[...]))
    h4p = jnp.dot(h3, W4[...], preferred_element_type=F32) * lax.rsqrt(deg + 1.0)
    h4p_o[...] = jnp.concatenate([h4p, jnp.zeros((RB, 64), F32)], axis=1)


def _tc4(accS, degp, part, p):
    return pl.pallas_call(
        _tc4_body,
        grid=(_GRID,),
        in_specs=[_rows(128), _rows(1), _rows(64), _full(128, 64), _full(64),
                  _full(64), _full(64, 64)],
        out_specs=_rows(128),
        out_shape=jax.ShapeDtypeStruct((NN, 128), F32),
    )(accS, degp, part, p["sage3_Wl"], p["ln3_g"], p["ln3_b"], p["gc4_W"])


def _tc5_body(acc4, h4p, degp, batch3, b4, g4, be4, Wro, bro, pool_o):
    i = pl.program_id(0)
    dinv = lax.rsqrt(degp[...] + 1.0)
    h4 = jax.nn.relu(_ln(dinv * (acc4[...] + h4p[...]) + b4[...],
                         g4[...], be4[...]))
    gate = jax.nn.sigmoid(jnp.dot(h4, Wro[...], preferred_element_type=F32) + bro[...])
    gated = h4 * gate
    b = batch3[...][0, 0, :]
    P = (lax.broadcasted_iota(I32, (64, RB), 0) == b[None, :]).astype(F32)
    rhs = jnp.concatenate([gated, gate, jnp.zeros((RB, 63), F32)], axis=1)
    blk = jnp.dot(P, rhs, preferred_element_type=F32)

    @pl.when(i == 0)
    def _():
        pool_o[...] = jnp.zeros_like(pool_o)

    pool_o[...] += blk


def _tc5(acc4, h4p, degp, batch3, p):
    return pl.pallas_call(
        _tc5_body,
        grid=(_GRID,),
        in_specs=[_rows(64), _rows(64), _rows(1),
                  pl.BlockSpec((1, 1, RB), lambda i: (i, 0, 0)),
                  _full(64), _full(64), _full(64), _full(64, 1), _full(1)],
        out_specs=_full(64, 128),
        out_shape=jax.ShapeDtypeStruct((64, 128), F32),
    )(acc4, h4p, degp, batch3, p["gc4_b"], p["ln4_g"], p["ln4_b"],
      p["ro_W"], p["ro_b"])


def _tc6_body(pool, link3, W1, b1, W2, b2, W3, b3, out_o):
    pm = pool[...]
    emb = pm[:, :64] / (pm[:, 64:65] + 1e-8)
    lk = link3[...][:, 0, :]
    g_iota = lax.broadcasted_iota(I32, (4096, 64), 1)
    o1 = (lk[0][:, None] == g_iota).astype(F32)
    o2 = (lk[1][:, None] == g_iota).astype(F32)
    f = jnp.concatenate([
        jnp.dot(o1, emb, preferred_element_type=F32),
        jnp.dot(o2, emb, preferred_element_type=F32)], axis=1)
    f = jax.nn.relu(jnp.dot(f, W1[...], preferred_element_type=F32) + b1[...])
    f = jax.nn.relu(jnp.dot(f, W2[...], preferred_element_type=F32) + b2[...])
    out_o[...] = jax.nn.sigmoid(jnp.dot(f, W3[...], preferred_element_type=F32) + b3[...])


def _tc6(pool, link3, p):
    return pl.pallas_call(
        _tc6_body,
        in_specs=[pl.BlockSpec((64, 128), lambda: (0, 0)),
                  pl.BlockSpec((2, 1, 4096), lambda: (0, 0, 0)),
                  pl.BlockSpec((128, 64), lambda: (0, 0)),
                  pl.BlockSpec((64,), lambda: (0,)),
                  pl.BlockSpec((64, 64), lambda: (0, 0)),
                  pl.BlockSpec((64,), lambda: (0,)),
                  pl.BlockSpec((64, 1), lambda: (0, 0)),
                  pl.BlockSpec((1,), lambda: (0,)),
                  ],
        out_specs=pl.BlockSpec((4096, 1), lambda: (0, 0)),
        out_shape=jax.ShapeDtypeStruct((4096, 1), F32),
    )(pool, link3, p["mlp1_W"], p["mlp1_b"], p["mlp2_W"], p["mlp2_b"],
      p["mlp3_W"], p["mlp3_b"])


def kernel(x, edge_index, batch, link_indices, params):
    p = params

    clist, counts, deg = _sc_scan(edge_index)
    degp = deg[:NN, None]
    hw1p = _tc1(x, p["gcn1_W"], degp)
    acc1 = _sc_rowsum256(hw1p, clist, counts)[:NN]
    hw2, aa, res2 = _tc2(acc1, hw1p, degp, p)
    num, den = _sc_gat(hw2.reshape(NN * 4, 128),
                       aa.reshape(4, 1, 2 * NN), clist, counts)
    den = den.reshape(4, NT * RNG, 16)
    h2, part = _tc3(num[:, :NN], den[:, :NN], hw2, aa, res2, p)
    accS = _sc_rowsum128(h2, clist, counts)[:NN]
    h4p = _tc4(accS, degp, part, p)
    acc4 = _sc_rowsum128(h4p, clist, counts)[:NN, :64]
    pool = _tc5(acc4, h4p[:, :64], degp, batch.reshape(_GRID, 1, RB), p)
    out = _tc6(pool, link_indices.reshape(2, 1, 4096), p)
    return out[:, 0]
